# baseline (speedup 1.0000x reference)
import functools

import jax
import jax.numpy as jnp
from jax import lax
from jax.experimental import pallas as pl
from jax.experimental.pallas import tpu as pltpu
from jax.experimental.pallas import tpu_sc as plsc

_N = 10000
_E = 320000
_H1 = 8
_HID = 64
_OUT2 = 32
_NC = 16

_L = 16
_NCORE = 2
_NSUB = 16

_SC_PARAMS = dict(
    compiler_params=pltpu.CompilerParams(
        use_tc_tiling_on_sc=False, needs_layout_passes=False),
)


def _sc_mesh():
    return plsc.VectorSubcoreMesh(core_axis_name="c", subcore_axis_name="s")


def _zero_rows(ref, nrows):
    z = jnp.zeros((_L,), jnp.float32)

    def body(i, _):
        ref[i] = z
        return 0

    lax.fori_loop(0, nrows, body, 0)


def _zero_i32(ref, n):
    z = jnp.zeros((_L,), jnp.int32)

    def body(i, _):
        ref[pl.ds(i * _L, _L)] = z
        return 0

    lax.fori_loop(0, n // _L, body, 0)


def _fill_iota(ref, n, base):
    nfull = n // _L

    def body(i, _):
        ref[pl.ds(i * _L, _L)] = lax.iota(jnp.int32, _L) + (base + i * _L)
        return 0

    lax.fori_loop(0, nfull, body, 0)
    if n % _L:
        off = n - _L
        ref[pl.ds(off, _L)] = lax.iota(jnp.int32, _L) + (base + off)


def _elu(v):
    return jnp.where(v > 0, v, jnp.exp(v) - 1.0)


def _vlrelu(v):
    return jnp.where(v > 0, v, 0.2 * v)


def _att_den_kernel(asrc_hbm, adst_hbm, src_hbm, dst_hbm, den_hbm,
                    bs, bd, exb, srcv, dstv, spden, sem):
    c = lax.axis_index("c")
    s = lax.axis_index("s")
    B = 1000
    ESH = _E // (_NCORE * _NSUB)
    ebase = (c * _NSUB + s) * ESH

    _zero_rows(bs, B)

    @pl.when(s < 10)
    def _():
        pltpu.sync_copy(bs.at[pl.ds(0, B)], spden.at[pl.ds(s * B, B)])

    plsc.subcore_barrier()

    def chunk(k, _):
        base = ebase + k * B
        pltpu.sync_copy(src_hbm.at[pl.ds(base, B)], srcv)
        pltpu.sync_copy(dst_hbm.at[pl.ds(base, B)], dstv)
        d1 = pltpu.async_copy(asrc_hbm.at[srcv], bs, sem)
        d1.wait()
        pltpu.async_copy(adst_hbm.at[dstv], bd, sem).wait()

        def edge(e, _):
            exb[e] = jnp.exp(_vlrelu(bs[e] + bd[e]))
            return 0

        lax.fori_loop(0, B, edge, 0)
        pltpu.sync_copy(exb, spden.at[dstv], add=True)
        return 0

    lax.fori_loop(0, ESH // B, chunk, 0)
    plsc.subcore_barrier()

    @pl.when(s < 10)
    def _():
        pltpu.sync_copy(spden.at[pl.ds(s * 1000, 1000)],
                        den_hbm.at[c, pl.ds(s * 1000, 1000)])


def _att_den(asrc16, adst16, src, dst):
    B = 1000
    f = pl.kernel(
        _att_den_kernel,
        mesh=_sc_mesh(),
        out_type=jax.ShapeDtypeStruct((_NCORE, _N, 16), jnp.float32),
        scratch_types=[
            pltpu.VMEM((B, 16), jnp.float32),
            pltpu.VMEM((B, 16), jnp.float32),
            pltpu.VMEM((B, 16), jnp.float32),
            pltpu.VMEM((B,), jnp.int32),
            pltpu.VMEM((B,), jnp.int32),
            pltpu.VMEM_SHARED((_N, 16), jnp.float32),
            pltpu.SemaphoreType.DMA,
        ],
        **_SC_PARAMS,
    )
    return f(asrc16, adst16, src, dst)


def _coef_kernel(asrc_hbm, adst_hbm, dent_hbm, src_hbm, dst_hbm, coef_hbm,
                 bs, bd, dn, cfb, srcv, dstv, sem):
    c = lax.axis_index("c")
    s = lax.axis_index("s")
    B = 1000
    ESH = _E // (_NCORE * _NSUB)
    ebase = (c * _NSUB + s) * ESH

    def chunk(k, _):
        base = ebase + k * B
        pltpu.sync_copy(src_hbm.at[pl.ds(base, B)], srcv)
        pltpu.sync_copy(dst_hbm.at[pl.ds(base, B)], dstv)
        d1 = pltpu.async_copy(asrc_hbm.at[srcv], bs, sem)
        d2 = pltpu.async_copy(adst_hbm.at[dstv], bd, sem)
        d1.wait()
        d2.wait()
        pltpu.async_copy(dent_hbm.at[dstv], dn, sem).wait()

        def edge(e, _):
            ex = jnp.exp(_vlrelu(bs[e] + bd[e]))
            cfb[e] = ex / (dn[e] + 1e-16)
            return 0

        lax.fori_loop(0, B, edge, 0)
        pltpu.sync_copy(cfb, coef_hbm.at[pl.ds(base, B)])
        return 0

    lax.fori_loop(0, ESH // B, chunk, 0)


def _coef(asrc16, adst16, dent16, src, dst):
    B = 1000
    f = pl.kernel(
        _coef_kernel,
        mesh=_sc_mesh(),
        out_type=jax.ShapeDtypeStruct((_E, 16), jnp.float32),
        scratch_types=[
            pltpu.VMEM((B, 16), jnp.float32),
            pltpu.VMEM((B, 16), jnp.float32),
            pltpu.VMEM((B, 16), jnp.float32),
            pltpu.VMEM((B, 16), jnp.float32),
            pltpu.VMEM((B,), jnp.int32),
            pltpu.VMEM((B,), jnp.int32),
            pltpu.SemaphoreType.DMA,
        ],
        **_SC_PARAMS,
    )
    return f(asrc16, adst16, dent16, src, dst)


def _gat1_agg_kernel(h1t_hbm, coef_hbm, src_hbm, dst_hbm, out_hbm,
                     table, pend_src, pend_ld, pend_eid, srcv, dstv,
                     rows, cfb, sem):
    c = lax.axis_index("c")
    s = lax.axis_index("s")
    w = c * _NSUB + s
    B = 800
    G = 128
    RSZ = 624

    _zero_i32(pend_src, B + _L)
    _zero_i32(pend_eid, B + _L)

    def combo(q, _):
        cid = w * 2 + q
        r = cid % 16
        g = cid // 16
        lo = r * RSZ
        hi = jnp.where(r == 15, _N, lo + RSZ)
        gbase = g * _N

        zv = jnp.zeros((_L,), jnp.float32)

        def zrow(i, _):
            for t in range(8):
                table[i, pl.ds(t * _L, _L)] = zv
            return 0

        lax.fori_loop(0, 640, zrow, 0)

        def chunk(k, _):
            base = k * B
            pltpu.sync_copy(src_hbm.at[pl.ds(base, B)], srcv)
            pltpu.sync_copy(dst_hbm.at[pl.ds(base, B)], dstv)

            def scan(v, cnt):
                d = dstv[pl.ds(v * _L, _L)]
                m = (d >= lo) & (d < hi)
                csum = plsc.cumsum(m.astype(jnp.int32))
                pos = cnt + csum - 1
                plsc.store_scatter(pend_ld, [pos], d - lo, mask=m)
                plsc.store_scatter(pend_src, [pos],
                                   srcv[pl.ds(v * _L, _L)] + gbase, mask=m)
                plsc.store_scatter(pend_eid, [pos],
                                   lax.iota(jnp.int32, _L) + (base + v * _L),
                                   mask=m)
                return cnt + jnp.max(csum)

            cnt = lax.fori_loop(0, B // _L, scan, jnp.int32(0))

            def flush(b, _):
                off = b * G
                d1 = pltpu.async_copy(h1t_hbm.at[pend_src.at[pl.ds(off, G)]],
                                      rows, sem)
                d2 = pltpu.async_copy(coef_hbm.at[pend_eid.at[pl.ds(off, G)]],
                                      cfb, sem)
                d1.wait()
                d2.wait()
                nin = jnp.minimum(cnt - off, G)
                col0 = jnp.full((_L,), 2 * g, jnp.int32)
                col1 = col0 + 1

                def acc(i, _):
                    ld = pend_ld[pl.ds(off + i, _L)][0]
                    iv = jnp.full((_L,), i, jnp.int32)
                    cf0 = plsc.load_gather(cfb, [iv, col0])
                    cf1 = plsc.load_gather(cfb, [iv, col1])
                    for t in range(8):
                        cf = cf0 if t < 4 else cf1
                        plsc.addupdate(table.at[ld, pl.ds(t * _L, _L)],
                                       cf * rows[i, pl.ds(t * _L, _L)])
                    return 0

                lax.fori_loop(0, nin, acc, 0)
                return 0

            lax.fori_loop(0, (cnt + G - 1) // G, flush, 0)
            return 0

        lax.fori_loop(0, _E // B, chunk, 0)

        pltpu.sync_copy(table.at[pl.ds(0, RSZ)],
                        out_hbm.at[pl.ds(gbase + lo, RSZ)])

        @pl.when(r == 15)
        def _():
            pltpu.sync_copy(table.at[pl.ds(RSZ, 16)],
                            out_hbm.at[pl.ds(gbase + lo + RSZ, 16)])

        return 0

    lax.fori_loop(0, 2, combo, 0)


def _gat1_agg(h1t, coef16, src, dst):
    B = 800
    G = 128
    f = pl.kernel(
        _gat1_agg_kernel,
        mesh=_sc_mesh(),
        out_type=jax.ShapeDtypeStruct((4 * _N, 128), jnp.float32),
        scratch_types=[
            pltpu.VMEM((640, 128), jnp.float32),
            pltpu.VMEM((B + _L,), jnp.int32),
            pltpu.VMEM((B + _L,), jnp.int32),
            pltpu.VMEM((B + _L,), jnp.int32),
            pltpu.VMEM((B,), jnp.int32),
            pltpu.VMEM((B,), jnp.int32),
            pltpu.VMEM((G, 128), jnp.float32),
            pltpu.VMEM((G, 16), jnp.float32),
            pltpu.SemaphoreType.DMA,
        ],
        **_SC_PARAMS,
    )
    return f(h1t, coef16, src, dst)


def _att_den2_kernel(asrc_hbm, adst_hbm, src_hbm, dst_hbm, den_hbm, ex_hbm,
                     bs, bd, exb, ex2v, srcv, dstv, spden, sem):
    c = lax.axis_index("c")
    s = lax.axis_index("s")
    B = 1000
    ESH = _E // (_NCORE * _NSUB)
    ebase = (c * _NSUB + s) * ESH

    _zero_rows(bs, B)

    @pl.when(s < 10)
    def _():
        pltpu.sync_copy(bs.at[pl.ds(0, B)], spden.at[pl.ds(s * B, B)])

    plsc.subcore_barrier()

    def chunk(k, _):
        base = ebase + k * B
        pltpu.sync_copy(src_hbm.at[pl.ds(base, B)], srcv)
        pltpu.sync_copy(dst_hbm.at[pl.ds(base, B)], dstv)
        d1 = pltpu.async_copy(asrc_hbm.at[srcv], bs, sem)
        d1.wait()
        pltpu.async_copy(adst_hbm.at[dstv], bd, sem).wait()

        def edge(e, _):
            exb[e] = jnp.exp(_vlrelu(bs[e] + bd[e]))
            return 0

        lax.fori_loop(0, B, edge, 0)
        pltpu.sync_copy(exb, spden.at[dstv], add=True)
        for jj in range(63):
            off = jj * _L if jj < 62 else B - _L
            rowv = lax.iota(jnp.int32, _L) + off
            ex2v[pl.ds(off, _L)] = plsc.load_gather(
                exb, [rowv, jnp.zeros((_L,), jnp.int32)])
        pltpu.sync_copy(ex2v, ex_hbm.at[pl.ds(base, B)])
        return 0

    lax.fori_loop(0, ESH // B, chunk, 0)
    plsc.subcore_barrier()

    @pl.when(s < 10)
    def _():
        pltpu.sync_copy(spden.at[pl.ds(s * 1000, 1000)],
                        den_hbm.at[c, pl.ds(s * 1000, 1000)])


def _att_den2(asrc16, adst16, src, dst):
    B = 1000
    f = pl.kernel(
        _att_den2_kernel,
        mesh=_sc_mesh(),
        out_type=[
            jax.ShapeDtypeStruct((_NCORE, _N, 16), jnp.float32),
            jax.ShapeDtypeStruct((_E,), jnp.float32),
        ],
        scratch_types=[
            pltpu.VMEM((B, 16), jnp.float32),
            pltpu.VMEM((B, 16), jnp.float32),
            pltpu.VMEM((B, 16), jnp.float32),
            pltpu.VMEM((B,), jnp.float32),
            pltpu.VMEM((B,), jnp.int32),
            pltpu.VMEM((B,), jnp.int32),
            pltpu.VMEM_SHARED((_N, 16), jnp.float32),
            pltpu.SemaphoreType.DMA,
        ],
        **_SC_PARAMS,
    )
    return f(asrc16, adst16, src, dst)


def _gat2_agg_kernel(h2_hbm, ex_hbm, dent_hbm, src_hbm, dst_hbm, out_hbm,
                     table, pend_src, pend_ld, pend_cf, srcv, dstv, exv,
                     dn, rows, idxv, spacc, sem):
    c = lax.axis_index("c")
    s = lax.axis_index("s")
    r = s // 4
    j = s % 4
    B = 800
    G = 128
    RNG = 2500
    lo = r * RNG
    ESH = _E // 8
    ebase = c * (_E // 2) + j * ESH

    zv = jnp.zeros((_L,), jnp.float32)

    def zrow(i, _):
        table[i, pl.ds(0, _L)] = zv
        table[i, pl.ds(_L, _L)] = zv
        return 0

    lax.fori_loop(0, RNG, zrow, 0)
    _zero_i32(pend_src, B + _L)

    @pl.when(s < 10)
    def _():
        pltpu.sync_copy(table.at[pl.ds(0, 1000)],
                        spacc.at[pl.ds(s * 1000, 1000)])

    plsc.subcore_barrier()

    def chunk(k, _):
        base = ebase + k * B
        pltpu.sync_copy(src_hbm.at[pl.ds(base, B)], srcv)
        pltpu.sync_copy(dst_hbm.at[pl.ds(base, B)], dstv)
        pltpu.sync_copy(ex_hbm.at[pl.ds(base, B)], exv)
        pltpu.async_copy(dent_hbm.at[dstv], dn, sem).wait()

        def scan(v, cnt):
            d = dstv[pl.ds(v * _L, _L)]
            m = (d >= lo) & (d < lo + RNG)
            rowv = lax.iota(jnp.int32, _L) + v * _L
            dnv = plsc.load_gather(dn, [rowv, jnp.zeros((_L,), jnp.int32)])
            cf = exv[pl.ds(v * _L, _L)] / (dnv + 1e-16)
            csum = plsc.cumsum(m.astype(jnp.int32))
            pos = cnt + csum - 1
            plsc.store_scatter(pend_ld, [pos], d - lo, mask=m)
            plsc.store_scatter(pend_src, [pos], srcv[pl.ds(v * _L, _L)],
                               mask=m)
            plsc.store_scatter(pend_cf, [pos], cf, mask=m)
            return cnt + jnp.max(csum)

        cnt = lax.fori_loop(0, B // _L, scan, jnp.int32(0))

        def flush(b, _):
            off = b * G
            pltpu.async_copy(h2_hbm.at[pend_src.at[pl.ds(off, G)]], rows,
                             sem).wait()
            nin = jnp.minimum(cnt - off, G)

            def acc(i, _):
                ld = pend_ld[pl.ds(off + i, _L)][0]
                cf = jnp.full((_L,), pend_cf[pl.ds(off + i, _L)][0])
                plsc.addupdate(table.at[ld, pl.ds(0, _L)],
                               cf * rows[i, pl.ds(0, _L)])
                plsc.addupdate(table.at[ld, pl.ds(_L, _L)],
                               cf * rows[i, pl.ds(_L, _L)])
                return 0

            lax.fori_loop(0, nin, acc, 0)
            return 0

        lax.fori_loop(0, (cnt + G - 1) // G, flush, 0)
        return 0

    lax.fori_loop(0, ESH // B, chunk, 0)

    _fill_iota(idxv, RNG, lo)
    pltpu.sync_copy(table, spacc.at[idxv], add=True)
    plsc.subcore_barrier()

    @pl.when(s < 10)
    def _():
        pltpu.sync_copy(spacc.at[pl.ds(s * 1000, 1000)],
                        out_hbm.at[c, pl.ds(s * 1000, 1000)])


def _gat2_agg(h2, ex2, dent2, src, dst):
    B = 800
    G = 128
    RNG = 2500
    f = pl.kernel(
        _gat2_agg_kernel,
        mesh=_sc_mesh(),
        out_type=jax.ShapeDtypeStruct((_NCORE, _N, 32), jnp.float32),
        scratch_types=[
            pltpu.VMEM((RNG, 32), jnp.float32),
            pltpu.VMEM((B + _L,), jnp.int32),
            pltpu.VMEM((B + _L,), jnp.int32),
            pltpu.VMEM((B + _L,), jnp.float32),
            pltpu.VMEM((B,), jnp.int32),
            pltpu.VMEM((B,), jnp.int32),
            pltpu.VMEM((B,), jnp.float32),
            pltpu.VMEM((B, 16), jnp.float32),
            pltpu.VMEM((G, 32), jnp.float32),
            pltpu.VMEM((RNG,), jnp.int32),
            pltpu.VMEM_SHARED((_N, 32), jnp.float32),
            pltpu.SemaphoreType.DMA,
        ],
        **_SC_PARAMS,
    )
    return f(h2, ex2, dent2, src, dst)


def _gcn_gather_kernel(g_hbm, src_hbm, dst_hbm, out_hbm,
                       table, pend_src, pend_ld, srcv, dstv, rows, idxv,
                       spmem, sem):
    c = lax.axis_index("c")
    s = lax.axis_index("s")
    r = s // 4
    j = s % 4
    RNG = 2500
    ESH = _E // 8
    B = 800
    G = 128
    lo = r * RNG
    ebase = c * (_E // 2) + j * ESH

    _zero_rows(table, RNG)
    _zero_i32(pend_src, B + _L)
    zrows = 1000

    @pl.when(s < 10)
    def _():
        pltpu.sync_copy(table.at[pl.ds(0, zrows)],
                        spmem.at[pl.ds(s * zrows, zrows)])

    plsc.subcore_barrier()

    def chunk_body(k, _):
        pltpu.sync_copy(src_hbm.at[pl.ds(ebase + k * B, B)], srcv)
        pltpu.sync_copy(dst_hbm.at[pl.ds(ebase + k * B, B)], dstv)

        def scan_body(v, cnt):
            d = dstv[pl.ds(v * _L, _L)]
            m = (d >= lo) & (d < lo + RNG)
            csum = plsc.cumsum(m.astype(jnp.int32))
            pos = cnt + csum - 1
            plsc.store_scatter(pend_ld, [pos], d - lo, mask=m)
            plsc.store_scatter(pend_src, [pos], srcv[pl.ds(v * _L, _L)],
                               mask=m)
            return cnt + jnp.max(csum)

        cnt = lax.fori_loop(0, B // _L, scan_body, jnp.int32(0))

        def flush_body(b, _):
            off = b * G
            pltpu.async_copy(g_hbm.at[pend_src.at[pl.ds(off, G)]], rows,
                             sem).wait()
            nin = jnp.minimum(cnt - off, G)

            def acc_body(i, _):
                ld = pend_ld[pl.ds(off + i, _L)][0]
                plsc.addupdate(table.at[ld], rows[i])
                return 0

            lax.fori_loop(0, nin, acc_body, 0)
            return 0

        lax.fori_loop(0, (cnt + G - 1) // G, flush_body, 0)
        return 0

    lax.fori_loop(0, ESH // B, chunk_body, 0)

    _fill_iota(idxv, RNG, lo)
    pltpu.sync_copy(table, spmem.at[idxv], add=True)
    plsc.subcore_barrier()

    @pl.when(s < 10)
    def _():
        pltpu.sync_copy(spmem.at[pl.ds(s * zrows, zrows)],
                        out_hbm.at[c, pl.ds(s * zrows, zrows)])


def _gcn_gather(g, src, dst):
    B = 800
    G = 128
    RNG = 2500
    f = pl.kernel(
        _gcn_gather_kernel,
        mesh=_sc_mesh(),
        out_type=jax.ShapeDtypeStruct((_NCORE, _N, 16), jnp.float32),
        scratch_types=[
            pltpu.VMEM((RNG, 16), jnp.float32),
            pltpu.VMEM((B + _L,), jnp.int32),
            pltpu.VMEM((B + _L,), jnp.int32),
            pltpu.VMEM((B,), jnp.int32),
            pltpu.VMEM((B,), jnp.int32),
            pltpu.VMEM((G, 16), jnp.float32),
            pltpu.VMEM((RNG,), jnp.int32),
            pltpu.VMEM_SHARED((_N, 16), jnp.float32),
            pltpu.SemaphoreType.DMA,
        ],
        **_SC_PARAMS,
    )
    return f(g, src, dst)


def _dense1_body(x_ref, w_ref, aws_ref, awd_ref, h_ref, as_ref, ad_ref):
    g = pl.program_id(1)
    h = jnp.dot(x_ref[...], w_ref[...], preferred_element_type=jnp.float32)
    h_ref[...] = h
    das = jnp.dot(h, aws_ref[...], preferred_element_type=jnp.float32)
    dad = jnp.dot(h, awd_ref[...], preferred_element_type=jnp.float32)

    @pl.when(g == 0)
    def _():
        as_ref[...] = das
        ad_ref[...] = dad

    @pl.when(g > 0)
    def _():
        as_ref[...] += das
        ad_ref[...] += dad


def _dense1(x, W1, aws, awd):
    bn = 1000
    return pl.pallas_call(
        _dense1_body,
        grid=(_N // bn, 4),
        in_specs=[
            pl.BlockSpec((bn, 128), lambda i, g: (i, 0)),
            pl.BlockSpec((128, 128), lambda i, g: (0, g)),
            pl.BlockSpec((128, 16), lambda i, g: (g, 0)),
            pl.BlockSpec((128, 16), lambda i, g: (g, 0)),
        ],
        out_specs=[
            pl.BlockSpec((bn, 128), lambda i, g: (g * (_N // bn) + i, 0)),
            pl.BlockSpec((bn, 16), lambda i, g: (i, 0)),
            pl.BlockSpec((bn, 16), lambda i, g: (i, 0)),
        ],
        out_shape=[
            jax.ShapeDtypeStruct((4 * _N, 128), jnp.float32),
            jax.ShapeDtypeStruct((_N, 16), jnp.float32),
            jax.ShapeDtypeStruct((_N, 16), jnp.float32),
        ],
    )(x, W1, aws, awd)


def _combine1_body(dp_ref, as_ref, ad_ref, dent_ref):
    ex_self = jnp.exp(_vlrelu(as_ref[...] + ad_ref[...]))
    dent_ref[...] = dp_ref[0] + dp_ref[1] + ex_self


def _combine1(denp, asrc16, adst16):
    bn = 1000
    return pl.pallas_call(
        _combine1_body,
        grid=(_N // bn,),
        in_specs=[
            pl.BlockSpec((2, bn, 16), lambda i: (0, i, 0)),
            pl.BlockSpec((bn, 16), lambda i: (i, 0)),
            pl.BlockSpec((bn, 16), lambda i: (i, 0)),
        ],
        out_specs=pl.BlockSpec((bn, 16), lambda i: (i, 0)),
        out_shape=jax.ShapeDtypeStruct((_N, 16), jnp.float32),
    )(denp, asrc16, adst16)


def _dense2_body(o1_ref, h1_ref, as_ref, ad_ref, dent_ref, w2_ref, b1_ref,
                 aws2_ref, awd2_ref, h2_ref, as2_ref, ad2_ref):
    g = pl.program_id(1)
    selfc = jnp.exp(_vlrelu(as_ref[...] + ad_ref[...])) / (dent_ref[...] + 1e-16)
    col = lax.broadcasted_iota(jnp.int32, selfc.shape, 1)
    s0 = jnp.sum(jnp.where(col == 2 * g, selfc, 0.0), axis=1, keepdims=True)
    s1 = jnp.sum(jnp.where(col == 2 * g + 1, selfc, 0.0), axis=1,
                 keepdims=True)
    h1b = h1_ref[...]
    b1full = b1_ref[...]
    row = lax.broadcasted_iota(jnp.int32, b1full.shape, 0)
    b1g = jnp.sum(jnp.where(row == g, b1full, 0.0), axis=0, keepdims=True)
    slab = o1_ref[...] + jnp.concatenate(
        [s0 * h1b[:, :64], s1 * h1b[:, 64:]], axis=1) + b1g
    g1 = _elu(slab)
    dh2 = jnp.dot(g1, w2_ref[...], preferred_element_type=jnp.float32)

    @pl.when(g == 0)
    def _():
        h2_ref[...] = dh2

    @pl.when(g > 0)
    def _():
        h2_ref[...] += dh2

    @pl.when(g == 3)
    def _():
        h2f = h2_ref[...]
        as2_ref[...] = jnp.dot(h2f, aws2_ref[...],
                               preferred_element_type=jnp.float32)
        ad2_ref[...] = jnp.dot(h2f, awd2_ref[...],
                               preferred_element_type=jnp.float32)


def _dense2(out1t, h1t, asrc16, adst16, dent16, W2, b1, aws2, awd2):
    bn = 1000
    nb = _N // bn
    b1r = b1.reshape(4, 128)
    return pl.pallas_call(
        _dense2_body,
        grid=(nb, 4),
        in_specs=[
            pl.BlockSpec((bn, 128), lambda i, g: (g * nb + i, 0)),
            pl.BlockSpec((bn, 128), lambda i, g: (g * nb + i, 0)),
            pl.BlockSpec((bn, 16), lambda i, g: (i, 0)),
            pl.BlockSpec((bn, 16), lambda i, g: (i, 0)),
            pl.BlockSpec((bn, 16), lambda i, g: (i, 0)),
            pl.BlockSpec((128, 32), lambda i, g: (g, 0)),
            pl.BlockSpec((4, 128), lambda i, g: (0, 0)),
            pl.BlockSpec((32, 16), lambda i, g: (0, 0)),
            pl.BlockSpec((32, 16), lambda i, g: (0, 0)),
        ],
        out_specs=[
            pl.BlockSpec((bn, 32), lambda i, g: (i, 0)),
            pl.BlockSpec((bn, 16), lambda i, g: (i, 0)),
            pl.BlockSpec((bn, 16), lambda i, g: (i, 0)),
        ],
        out_shape=[
            jax.ShapeDtypeStruct((_N, 32), jnp.float32),
            jax.ShapeDtypeStruct((_N, 16), jnp.float32),
            jax.ShapeDtypeStruct((_N, 16), jnp.float32),
        ],
    )(out1t, h1t, asrc16, adst16, dent16, W2, b1r, aws2, awd2)


def _combine2_body(dp_ref, as_ref, ad_ref, dent_ref):
    ex_self = jnp.exp(_vlrelu(as_ref[...] + ad_ref[...]))
    dent_ref[...] = dp_ref[0] + dp_ref[1] + ex_self


def _combine2(denp2, as2_16, ad2_16):
    bn = 1000
    return pl.pallas_call(
        _combine2_body,
        grid=(_N // bn,),
        in_specs=[
            pl.BlockSpec((2, bn, 16), lambda i: (0, i, 0)),
            pl.BlockSpec((bn, 16), lambda i: (i, 0)),
            pl.BlockSpec((bn, 16), lambda i: (i, 0)),
        ],
        out_specs=pl.BlockSpec((bn, 16), lambda i: (i, 0)),
        out_shape=jax.ShapeDtypeStruct((_N, 16), jnp.float32),
    )(denp2, as2_16, ad2_16)


def _dense3_body(op_ref, h2_ref, as2_ref, ad2_ref, dent2_ref,
                 dent1_ref, w3_ref, b2_ref, g_ref, h3_ref, dinv_ref):
    selfc = jnp.exp(_vlrelu(as2_ref[...] + ad2_ref[...])) / (dent2_ref[...]
                                                             + 1e-16)
    out2 = (op_ref[0] + op_ref[1] + selfc[:, 0:1] * h2_ref[...]
            + b2_ref[...])
    g2 = _elu(out2)
    h3 = jnp.dot(g2, w3_ref[...], preferred_element_type=jnp.float32)
    deg = dent1_ref[:, 8:9]
    dinv = lax.rsqrt(deg)
    h3_ref[...] = h3
    g_ref[...] = dinv * h3
    dinv_ref[...] = jnp.broadcast_to(dinv, h3.shape)


def _dense3(out2p, h2, as2_16, ad2_16, dent2, dent1, W3, b2):
    bn = 1000
    b2c = b2.reshape(1, 32)
    return pl.pallas_call(
        _dense3_body,
        grid=(_N // bn,),
        in_specs=[
            pl.BlockSpec((2, bn, 32), lambda i: (0, i, 0)),
            pl.BlockSpec((bn, 32), lambda i: (i, 0)),
            pl.BlockSpec((bn, 16), lambda i: (i, 0)),
            pl.BlockSpec((bn, 16), lambda i: (i, 0)),
            pl.BlockSpec((bn, 16), lambda i: (i, 0)),
            pl.BlockSpec((bn, 16), lambda i: (i, 0)),
            pl.BlockSpec((32, 16), lambda i: (0, 0)),
            pl.BlockSpec((1, 32), lambda i: (0, 0)),
        ],
        out_specs=[
            pl.BlockSpec((bn, 16), lambda i: (i, 0)),
            pl.BlockSpec((bn, 16), lambda i: (i, 0)),
            pl.BlockSpec((bn, 16), lambda i: (i, 0)),
        ],
        out_shape=[
            jax.ShapeDtypeStruct((_N, 16), jnp.float32),
            jax.ShapeDtypeStruct((_N, 16), jnp.float32),
            jax.ShapeDtypeStruct((_N, 16), jnp.float32),
        ],
    )(out2p, h2, as2_16, ad2_16, dent2, dent1, W3, b2c)


def _dense4_body(ap_ref, h3_ref, dinv_ref, b3_ref, out_ref):
    dinv = dinv_ref[...]
    out_ref[...] = (dinv * (ap_ref[0] + ap_ref[1])
                    + dinv * dinv * h3_ref[...] + b3_ref[...])


def _dense4(accp, h3, dinv, b3):
    bn = 1000
    b3c = b3.reshape(1, 16)
    return pl.pallas_call(
        _dense4_body,
        grid=(_N // bn,),
        in_specs=[
            pl.BlockSpec((2, bn, 16), lambda i: (0, i, 0)),
            pl.BlockSpec((bn, 16), lambda i: (i, 0)),
            pl.BlockSpec((bn, 16), lambda i: (i, 0)),
            pl.BlockSpec((1, 16), lambda i: (0, 0)),
        ],
        out_specs=pl.BlockSpec((bn, 16), lambda i: (i, 0)),
        out_shape=jax.ShapeDtypeStruct((_N, 16), jnp.float32),
    )(accp, h3, dinv, b3c)


def _logit_weights16(a):
    heads, ch = a.shape
    eye = jnp.eye(16, dtype=a.dtype)[:heads]
    return (a[:, :, None] * eye[:, None, :]).reshape(heads * ch, 16)


def kernel(x, edge_index, W1, a_src1, a_dst1, b1, W2, a_src2, a_dst2, b2, W3, b3):
    src = edge_index[0]
    dst = edge_index[1]

    aws1 = _logit_weights16(a_src1)
    awd1 = _logit_weights16(a_dst1)
    h1t, asrc16, adst16 = _dense1(x, W1, aws1, awd1)
    den1p = _att_den(asrc16, adst16, src, dst)
    dent1 = _combine1(den1p, asrc16, adst16)
    coef16 = _coef(asrc16, adst16, dent1, src, dst)
    out1t = _gat1_agg(h1t, coef16, src, dst)

    aws2 = _logit_weights16(a_src2)
    awd2 = _logit_weights16(a_dst2)
    h2, as2_16, ad2_16 = _dense2(out1t, h1t, asrc16, adst16, dent1,
                                 W2, b1, aws2, awd2)
    den2p, ex2 = _att_den2(as2_16, ad2_16, src, dst)
    dent2 = _combine2(den2p, as2_16, ad2_16)
    out2p = _gat2_agg(h2, ex2, dent2, src, dst)

    g, h3, dinv = _dense3(out2p, h2, as2_16, ad2_16, dent2, dent1, W3, b2)
    accp = _gcn_gather(g, src, dst)
    out = _dense4(accp, h3, dinv, b3)
    return out


# speedup vs baseline: 10.2369x; 10.2369x over previous
import functools

import jax
import jax.numpy as jnp
from jax import lax
from jax.experimental import pallas as pl
from jax.experimental.pallas import tpu as pltpu
from jax.experimental.pallas import tpu_sc as plsc

_N = 10000
_E = 320000
_H1 = 8
_HID = 64
_OUT2 = 32
_NC = 16

_L = 16
_NCORE = 2
_NSUB = 16

_SC_PARAMS = dict(
    compiler_params=pltpu.CompilerParams(
        use_tc_tiling_on_sc=False, needs_layout_passes=False),
)


def _sc_mesh():
    return plsc.VectorSubcoreMesh(core_axis_name="c", subcore_axis_name="s")


def _zero_rows(ref, nrows):
    z = jnp.zeros((_L,), jnp.float32)

    def body(i, _):
        ref[i] = z
        return 0

    lax.fori_loop(0, nrows, body, 0)


def _zero_i32(ref, n):
    z = jnp.zeros((_L,), jnp.int32)

    def body(i, _):
        ref[pl.ds(i * _L, _L)] = z
        return 0

    lax.fori_loop(0, n // _L, body, 0)


def _fill_iota(ref, n, base):
    nfull = n // _L

    def body(i, _):
        ref[pl.ds(i * _L, _L)] = lax.iota(jnp.int32, _L) + (base + i * _L)
        return 0

    lax.fori_loop(0, nfull, body, 0)
    if n % _L:
        off = n - _L
        ref[pl.ds(off, _L)] = lax.iota(jnp.int32, _L) + (base + off)


def _elu(v):
    return jnp.where(v > 0, v, jnp.exp(v) - 1.0)


def _vlrelu(v):
    return jnp.where(v > 0, v, 0.2 * v)


def _att_den_kernel(asrc_hbm, adst_hbm, src_hbm, dst_hbm, den_hbm,
                    bs, bd, exb, srcv, dstv, spden, sem):
    c = lax.axis_index("c")
    s = lax.axis_index("s")
    B = 1000
    ESH = _E // (_NCORE * _NSUB)
    ebase = (c * _NSUB + s) * ESH

    _zero_rows(bs, B)

    @pl.when(s < 10)
    def _():
        pltpu.sync_copy(bs.at[pl.ds(0, B)], spden.at[pl.ds(s * B, B)])

    plsc.subcore_barrier()

    def chunk(k, _):
        base = ebase + k * B
        pltpu.sync_copy(src_hbm.at[pl.ds(base, B)], srcv)
        pltpu.sync_copy(dst_hbm.at[pl.ds(base, B)], dstv)
        d1 = pltpu.async_copy(asrc_hbm.at[srcv], bs, sem)
        d1.wait()
        pltpu.async_copy(adst_hbm.at[dstv], bd, sem).wait()

        def edge(e, _):
            exb[e] = jnp.exp(_vlrelu(bs[e] + bd[e]))
            return 0

        lax.fori_loop(0, B, edge, 0)
        pltpu.sync_copy(exb, spden.at[dstv], add=True)
        return 0

    lax.fori_loop(0, ESH // B, chunk, 0)
    plsc.subcore_barrier()

    @pl.when(s < 10)
    def _():
        pltpu.sync_copy(spden.at[pl.ds(s * 1000, 1000)],
                        den_hbm.at[c, pl.ds(s * 1000, 1000)])


def _att_den(asrc16, adst16, src, dst):
    B = 1000
    f = pl.kernel(
        _att_den_kernel,
        mesh=_sc_mesh(),
        out_type=jax.ShapeDtypeStruct((_NCORE, _N, 16), jnp.float32),
        scratch_types=[
            pltpu.VMEM((B, 16), jnp.float32),
            pltpu.VMEM((B, 16), jnp.float32),
            pltpu.VMEM((B, 16), jnp.float32),
            pltpu.VMEM((B,), jnp.int32),
            pltpu.VMEM((B,), jnp.int32),
            pltpu.VMEM_SHARED((_N, 16), jnp.float32),
            pltpu.SemaphoreType.DMA,
        ],
        **_SC_PARAMS,
    )
    return f(asrc16, adst16, src, dst)


def _coef_kernel(asrc_hbm, adst_hbm, dent_hbm, src_hbm, dst_hbm, coef_hbm,
                 bs, bd, dn, cfb, srcv, dstv, sem):
    c = lax.axis_index("c")
    s = lax.axis_index("s")
    B = 1000
    ESH = _E // (_NCORE * _NSUB)
    ebase = (c * _NSUB + s) * ESH

    def chunk(k, _):
        base = ebase + k * B
        pltpu.sync_copy(src_hbm.at[pl.ds(base, B)], srcv)
        pltpu.sync_copy(dst_hbm.at[pl.ds(base, B)], dstv)
        d1 = pltpu.async_copy(asrc_hbm.at[srcv], bs, sem)
        d2 = pltpu.async_copy(adst_hbm.at[dstv], bd, sem)
        d1.wait()
        d2.wait()
        pltpu.async_copy(dent_hbm.at[dstv], dn, sem).wait()

        def edge(e, _):
            ex = jnp.exp(_vlrelu(bs[e] + bd[e]))
            cfb[e] = ex / (dn[e] + 1e-16)
            return 0

        lax.fori_loop(0, B, edge, 0)
        pltpu.sync_copy(cfb, coef_hbm.at[pl.ds(base, B)])
        return 0

    lax.fori_loop(0, ESH // B, chunk, 0)


def _coef(asrc16, adst16, dent16, src, dst):
    B = 1000
    f = pl.kernel(
        _coef_kernel,
        mesh=_sc_mesh(),
        out_type=jax.ShapeDtypeStruct((_E, 16), jnp.float32),
        scratch_types=[
            pltpu.VMEM((B, 16), jnp.float32),
            pltpu.VMEM((B, 16), jnp.float32),
            pltpu.VMEM((B, 16), jnp.float32),
            pltpu.VMEM((B, 16), jnp.float32),
            pltpu.VMEM((B,), jnp.int32),
            pltpu.VMEM((B,), jnp.int32),
            pltpu.SemaphoreType.DMA,
        ],
        **_SC_PARAMS,
    )
    return f(asrc16, adst16, dent16, src, dst)


def _gat1_agg_kernel(h1t_hbm, coef_hbm, src_hbm, dst_hbm, out_hbm,
                     table, pend_src, pend_ld, pend_eid, srcv, dstv,
                     rows, cfb, sem):
    c = lax.axis_index("c")
    s = lax.axis_index("s")
    w = c * _NSUB + s
    B = 800
    G = 128
    NP = B + 2 * G
    KCH = _E // B
    RSZ = 624

    _zero_i32(pend_src, NP)
    _zero_i32(pend_eid, NP)
    _zero_i32(pend_ld, NP)

    def combo(q, _):
        cid = w * 2 + q
        r = cid % 16
        g = cid // 16
        lo = r * RSZ
        hi = jnp.where(r == 15, _N, lo + RSZ)
        gbase = g * _N

        zv = jnp.zeros((_L,), jnp.float32)

        def zrow(i, _):
            for t in range(8):
                table[i, pl.ds(t * _L, _L)] = zv
            return 0

        lax.fori_loop(0, 640, zrow, 0)

        col0 = jnp.full((_L,), 2 * g, jnp.int32)
        col1 = col0 + 1

        def chunk(k, np_):
            base = k * B
            pltpu.sync_copy(src_hbm.at[pl.ds(base, B)], srcv)
            pltpu.sync_copy(dst_hbm.at[pl.ds(base, B)], dstv)

            def scan(v, cnt):
                d = dstv[pl.ds(v * _L, _L)]
                m = (d >= lo) & (d < hi)
                csum = plsc.cumsum(m.astype(jnp.int32))
                pos = cnt + csum - 1
                plsc.store_scatter(pend_ld, [pos], d - lo, mask=m)
                plsc.store_scatter(pend_src, [pos],
                                   srcv[pl.ds(v * _L, _L)] + gbase, mask=m)
                plsc.store_scatter(pend_eid, [pos],
                                   lax.iota(jnp.int32, _L) + (base + v * _L),
                                   mask=m)
                return cnt + jnp.max(csum)

            cnt = lax.fori_loop(0, B // _L, scan, np_)
            nf = jnp.where(k == KCH - 1, (cnt + G - 1) // G, cnt // G)

            def flush(b, _):
                off = b * G
                d1 = pltpu.async_copy(h1t_hbm.at[pend_src.at[pl.ds(off, G)]],
                                      rows, sem)
                d2 = pltpu.async_copy(coef_hbm.at[pend_eid.at[pl.ds(off, G)]],
                                      cfb, sem)
                d1.wait()
                d2.wait()
                nin = jnp.minimum(cnt - off, G)

                def acc(i, _):
                    ld = pend_ld[pl.ds(off + i, _L)][0]
                    iv = jnp.full((_L,), i, jnp.int32)
                    cf0 = plsc.load_gather(cfb, [iv, col0])
                    cf1 = plsc.load_gather(cfb, [iv, col1])
                    for t in range(8):
                        cf = cf0 if t < 4 else cf1
                        plsc.addupdate(table.at[ld, pl.ds(t * _L, _L)],
                                       cf * rows[i, pl.ds(t * _L, _L)])
                    return 0

                lax.fori_loop(0, nin, acc, 0)
                return 0

            lax.fori_loop(0, nf, flush, 0)
            rem = jnp.maximum(cnt - nf * G, 0)

            @pl.when(nf > 0)
            def _():
                fb = nf * G
                for jj in range(8):
                    sl = pl.ds(jj * _L, _L)
                    sr = pl.ds(fb + jj * _L, _L)
                    pend_ld[sl] = pend_ld[sr]
                    pend_src[sl] = pend_src[sr]
                    pend_eid[sl] = pend_eid[sr]

            return rem

        lax.fori_loop(0, KCH, chunk, jnp.int32(0))

        pltpu.sync_copy(table.at[pl.ds(0, RSZ)],
                        out_hbm.at[pl.ds(gbase + lo, RSZ)])

        @pl.when(r == 15)
        def _():
            pltpu.sync_copy(table.at[pl.ds(RSZ, 16)],
                            out_hbm.at[pl.ds(gbase + lo + RSZ, 16)])

        return 0

    lax.fori_loop(0, 2, combo, 0)


def _gat1_agg(h1t, coef16, src, dst):
    B = 800
    G = 128
    f = pl.kernel(
        _gat1_agg_kernel,
        mesh=_sc_mesh(),
        out_type=jax.ShapeDtypeStruct((4 * _N, 128), jnp.float32),
        scratch_types=[
            pltpu.VMEM((640, 128), jnp.float32),
            pltpu.VMEM((B + 2 * G,), jnp.int32),
            pltpu.VMEM((B + 2 * G,), jnp.int32),
            pltpu.VMEM((B + 2 * G,), jnp.int32),
            pltpu.VMEM((B,), jnp.int32),
            pltpu.VMEM((B,), jnp.int32),
            pltpu.VMEM((G, 128), jnp.float32),
            pltpu.VMEM((G, 16), jnp.float32),
            pltpu.SemaphoreType.DMA,
        ],
        **_SC_PARAMS,
    )
    return f(h1t, coef16, src, dst)


def _att_den2_kernel(asrc_hbm, adst_hbm, src_hbm, dst_hbm, den_hbm, ex_hbm,
                     bs, bd, exb, ex2v, srcv, dstv, spden, sem):
    c = lax.axis_index("c")
    s = lax.axis_index("s")
    B = 1000
    ESH = _E // (_NCORE * _NSUB)
    ebase = (c * _NSUB + s) * ESH

    _zero_rows(bs, B)

    @pl.when(s < 10)
    def _():
        pltpu.sync_copy(bs.at[pl.ds(0, B)], spden.at[pl.ds(s * B, B)])

    plsc.subcore_barrier()

    def chunk(k, _):
        base = ebase + k * B
        pltpu.sync_copy(src_hbm.at[pl.ds(base, B)], srcv)
        pltpu.sync_copy(dst_hbm.at[pl.ds(base, B)], dstv)
        d1 = pltpu.async_copy(asrc_hbm.at[srcv], bs, sem)
        d1.wait()
        pltpu.async_copy(adst_hbm.at[dstv], bd, sem).wait()

        def edge(e, _):
            exb[e] = jnp.exp(_vlrelu(bs[e] + bd[e]))
            return 0

        lax.fori_loop(0, B, edge, 0)
        pltpu.sync_copy(exb, spden.at[dstv], add=True)
        for jj in range(63):
            off = jj * _L if jj < 62 else B - _L
            rowv = lax.iota(jnp.int32, _L) + off
            ex2v[pl.ds(off, _L)] = plsc.load_gather(
                exb, [rowv, jnp.zeros((_L,), jnp.int32)])
        pltpu.sync_copy(ex2v, ex_hbm.at[pl.ds(base, B)])
        return 0

    lax.fori_loop(0, ESH // B, chunk, 0)
    plsc.subcore_barrier()

    @pl.when(s < 10)
    def _():
        pltpu.sync_copy(spden.at[pl.ds(s * 1000, 1000)],
                        den_hbm.at[c, pl.ds(s * 1000, 1000)])


def _att_den2(asrc16, adst16, src, dst):
    B = 1000
    f = pl.kernel(
        _att_den2_kernel,
        mesh=_sc_mesh(),
        out_type=[
            jax.ShapeDtypeStruct((_NCORE, _N, 16), jnp.float32),
            jax.ShapeDtypeStruct((_E,), jnp.float32),
        ],
        scratch_types=[
            pltpu.VMEM((B, 16), jnp.float32),
            pltpu.VMEM((B, 16), jnp.float32),
            pltpu.VMEM((B, 16), jnp.float32),
            pltpu.VMEM((B,), jnp.float32),
            pltpu.VMEM((B,), jnp.int32),
            pltpu.VMEM((B,), jnp.int32),
            pltpu.VMEM_SHARED((_N, 16), jnp.float32),
            pltpu.SemaphoreType.DMA,
        ],
        **_SC_PARAMS,
    )
    return f(asrc16, adst16, src, dst)


def _gat2_agg_kernel(h2_hbm, ex_hbm, dent_hbm, src_hbm, dst_hbm, out_hbm,
                     table, pend_src, pend_ld, pend_cf, srcv, dstv, exv,
                     dn, rows, idxv, spacc, sem):
    c = lax.axis_index("c")
    s = lax.axis_index("s")
    r = s // 4
    j = s % 4
    B = 800
    G = 128
    RNG = 2500
    lo = r * RNG
    ESH = _E // 8
    ebase = c * (_E // 2) + j * ESH

    zv = jnp.zeros((_L,), jnp.float32)

    def zrow(i, _):
        table[i, pl.ds(0, _L)] = zv
        table[i, pl.ds(_L, _L)] = zv
        return 0

    lax.fori_loop(0, RNG, zrow, 0)
    _zero_i32(pend_src, B + _L)

    @pl.when(s < 10)
    def _():
        pltpu.sync_copy(table.at[pl.ds(0, 1000)],
                        spacc.at[pl.ds(s * 1000, 1000)])

    plsc.subcore_barrier()

    def chunk(k, _):
        base = ebase + k * B
        pltpu.sync_copy(src_hbm.at[pl.ds(base, B)], srcv)
        pltpu.sync_copy(dst_hbm.at[pl.ds(base, B)], dstv)
        pltpu.sync_copy(ex_hbm.at[pl.ds(base, B)], exv)
        pltpu.async_copy(dent_hbm.at[dstv], dn, sem).wait()

        def scan(v, cnt):
            d = dstv[pl.ds(v * _L, _L)]
            m = (d >= lo) & (d < lo + RNG)
            rowv = lax.iota(jnp.int32, _L) + v * _L
            dnv = plsc.load_gather(dn, [rowv, jnp.zeros((_L,), jnp.int32)])
            cf = exv[pl.ds(v * _L, _L)] / (dnv + 1e-16)
            csum = plsc.cumsum(m.astype(jnp.int32))
            pos = cnt + csum - 1
            plsc.store_scatter(pend_ld, [pos], d - lo, mask=m)
            plsc.store_scatter(pend_src, [pos], srcv[pl.ds(v * _L, _L)],
                               mask=m)
            plsc.store_scatter(pend_cf, [pos], cf, mask=m)
            return cnt + jnp.max(csum)

        cnt = lax.fori_loop(0, B // _L, scan, jnp.int32(0))

        def flush(b, _):
            off = b * G
            pltpu.async_copy(h2_hbm.at[pend_src.at[pl.ds(off, G)]], rows,
                             sem).wait()
            nin = jnp.minimum(cnt - off, G)

            def acc(i, _):
                ld = pend_ld[pl.ds(off + i, _L)][0]
                cf = jnp.full((_L,), pend_cf[pl.ds(off + i, _L)][0])
                plsc.addupdate(table.at[ld, pl.ds(0, _L)],
                               cf * rows[i, pl.ds(0, _L)])
                plsc.addupdate(table.at[ld, pl.ds(_L, _L)],
                               cf * rows[i, pl.ds(_L, _L)])
                return 0

            lax.fori_loop(0, nin, acc, 0)
            return 0

        lax.fori_loop(0, (cnt + G - 1) // G, flush, 0)
        return 0

    lax.fori_loop(0, ESH // B, chunk, 0)

    _fill_iota(idxv, RNG, lo)
    pltpu.sync_copy(table, spacc.at[idxv], add=True)
    plsc.subcore_barrier()

    @pl.when(s < 10)
    def _():
        pltpu.sync_copy(spacc.at[pl.ds(s * 1000, 1000)],
                        out_hbm.at[c, pl.ds(s * 1000, 1000)])


def _gat2_agg(h2, ex2, dent2, src, dst):
    B = 800
    G = 128
    RNG = 2500
    f = pl.kernel(
        _gat2_agg_kernel,
        mesh=_sc_mesh(),
        out_type=jax.ShapeDtypeStruct((_NCORE, _N, 32), jnp.float32),
        scratch_types=[
            pltpu.VMEM((RNG, 32), jnp.float32),
            pltpu.VMEM((B + _L,), jnp.int32),
            pltpu.VMEM((B + _L,), jnp.int32),
            pltpu.VMEM((B + _L,), jnp.float32),
            pltpu.VMEM((B,), jnp.int32),
            pltpu.VMEM((B,), jnp.int32),
            pltpu.VMEM((B,), jnp.float32),
            pltpu.VMEM((B, 16), jnp.float32),
            pltpu.VMEM((G, 32), jnp.float32),
            pltpu.VMEM((RNG,), jnp.int32),
            pltpu.VMEM_SHARED((_N, 32), jnp.float32),
            pltpu.SemaphoreType.DMA,
        ],
        **_SC_PARAMS,
    )
    return f(h2, ex2, dent2, src, dst)


def _gcn_gather_kernel(g_hbm, src_hbm, dst_hbm, out_hbm,
                       table, pend_src, pend_ld, srcv, dstv, rows, idxv,
                       spmem, sem):
    c = lax.axis_index("c")
    s = lax.axis_index("s")
    r = s // 4
    j = s % 4
    RNG = 2500
    ESH = _E // 8
    B = 800
    G = 128
    lo = r * RNG
    ebase = c * (_E // 2) + j * ESH

    _zero_rows(table, RNG)
    _zero_i32(pend_src, B + _L)
    zrows = 1000

    @pl.when(s < 10)
    def _():
        pltpu.sync_copy(table.at[pl.ds(0, zrows)],
                        spmem.at[pl.ds(s * zrows, zrows)])

    plsc.subcore_barrier()

    def chunk_body(k, _):
        pltpu.sync_copy(src_hbm.at[pl.ds(ebase + k * B, B)], srcv)
        pltpu.sync_copy(dst_hbm.at[pl.ds(ebase + k * B, B)], dstv)

        def scan_body(v, cnt):
            d = dstv[pl.ds(v * _L, _L)]
            m = (d >= lo) & (d < lo + RNG)
            csum = plsc.cumsum(m.astype(jnp.int32))
            pos = cnt + csum - 1
            plsc.store_scatter(pend_ld, [pos], d - lo, mask=m)
            plsc.store_scatter(pend_src, [pos], srcv[pl.ds(v * _L, _L)],
                               mask=m)
            return cnt + jnp.max(csum)

        cnt = lax.fori_loop(0, B // _L, scan_body, jnp.int32(0))

        def flush_body(b, _):
            off = b * G
            pltpu.async_copy(g_hbm.at[pend_src.at[pl.ds(off, G)]], rows,
                             sem).wait()
            nin = jnp.minimum(cnt - off, G)

            def acc_body(i, _):
                ld = pend_ld[pl.ds(off + i, _L)][0]
                table[ld] = table[ld] + rows[i]
                return 0

            lax.fori_loop(0, nin, acc_body, 0)
            return 0

        lax.fori_loop(0, (cnt + G - 1) // G, flush_body, 0)
        return 0

    lax.fori_loop(0, ESH // B, chunk_body, 0)

    _fill_iota(idxv, RNG, lo)
    pltpu.sync_copy(table, spmem.at[idxv], add=True)
    plsc.subcore_barrier()

    @pl.when(s < 10)
    def _():
        pltpu.sync_copy(spmem.at[pl.ds(s * zrows, zrows)],
                        out_hbm.at[c, pl.ds(s * zrows, zrows)])


def _gcn_gather(g, src, dst):
    B = 800
    G = 128
    RNG = 2500
    f = pl.kernel(
        _gcn_gather_kernel,
        mesh=_sc_mesh(),
        out_type=jax.ShapeDtypeStruct((_NCORE, _N, 16), jnp.float32),
        scratch_types=[
            pltpu.VMEM((RNG, 16), jnp.float32),
            pltpu.VMEM((B + _L,), jnp.int32),
            pltpu.VMEM((B + _L,), jnp.int32),
            pltpu.VMEM((B,), jnp.int32),
            pltpu.VMEM((B,), jnp.int32),
            pltpu.VMEM((G, 16), jnp.float32),
            pltpu.VMEM((RNG,), jnp.int32),
            pltpu.VMEM_SHARED((_N, 16), jnp.float32),
            pltpu.SemaphoreType.DMA,
        ],
        **_SC_PARAMS,
    )
    return f(g, src, dst)


def _dense1_body(x_ref, w_ref, aws_ref, awd_ref, h_ref, as_ref, ad_ref):
    g = pl.program_id(1)
    h = jnp.dot(x_ref[...], w_ref[...], preferred_element_type=jnp.float32)
    h_ref[...] = h
    das = jnp.dot(h, aws_ref[...], preferred_element_type=jnp.float32)
    dad = jnp.dot(h, awd_ref[...], preferred_element_type=jnp.float32)

    @pl.when(g == 0)
    def _():
        as_ref[...] = das
        ad_ref[...] = dad

    @pl.when(g > 0)
    def _():
        as_ref[...] += das
        ad_ref[...] += dad


def _dense1(x, W1, aws, awd):
    bn = 1000
    return pl.pallas_call(
        _dense1_body,
        grid=(_N // bn, 4),
        in_specs=[
            pl.BlockSpec((bn, 128), lambda i, g: (i, 0)),
            pl.BlockSpec((128, 128), lambda i, g: (0, g)),
            pl.BlockSpec((128, 16), lambda i, g: (g, 0)),
            pl.BlockSpec((128, 16), lambda i, g: (g, 0)),
        ],
        out_specs=[
            pl.BlockSpec((bn, 128), lambda i, g: (g * (_N // bn) + i, 0)),
            pl.BlockSpec((bn, 16), lambda i, g: (i, 0)),
            pl.BlockSpec((bn, 16), lambda i, g: (i, 0)),
        ],
        out_shape=[
            jax.ShapeDtypeStruct((4 * _N, 128), jnp.float32),
            jax.ShapeDtypeStruct((_N, 16), jnp.float32),
            jax.ShapeDtypeStruct((_N, 16), jnp.float32),
        ],
    )(x, W1, aws, awd)


def _combine1_body(dp_ref, as_ref, ad_ref, dent_ref):
    ex_self = jnp.exp(_vlrelu(as_ref[...] + ad_ref[...]))
    dent_ref[...] = dp_ref[0] + dp_ref[1] + ex_self


def _combine1(denp, asrc16, adst16):
    bn = 1000
    return pl.pallas_call(
        _combine1_body,
        grid=(_N // bn,),
        in_specs=[
            pl.BlockSpec((2, bn, 16), lambda i: (0, i, 0)),
            pl.BlockSpec((bn, 16), lambda i: (i, 0)),
            pl.BlockSpec((bn, 16), lambda i: (i, 0)),
        ],
        out_specs=pl.BlockSpec((bn, 16), lambda i: (i, 0)),
        out_shape=jax.ShapeDtypeStruct((_N, 16), jnp.float32),
    )(denp, asrc16, adst16)


def _dense2_body(o1_ref, h1_ref, as_ref, ad_ref, dent_ref, w2_ref, b1_ref,
                 aws2_ref, awd2_ref, h2_ref, as2_ref, ad2_ref):
    g = pl.program_id(1)
    selfc = jnp.exp(_vlrelu(as_ref[...] + ad_ref[...])) / (dent_ref[...] + 1e-16)
    col = lax.broadcasted_iota(jnp.int32, selfc.shape, 1)
    s0 = jnp.sum(jnp.where(col == 2 * g, selfc, 0.0), axis=1, keepdims=True)
    s1 = jnp.sum(jnp.where(col == 2 * g + 1, selfc, 0.0), axis=1,
                 keepdims=True)
    h1b = h1_ref[...]
    b1full = b1_ref[...]
    row = lax.broadcasted_iota(jnp.int32, b1full.shape, 0)
    b1g = jnp.sum(jnp.where(row == g, b1full, 0.0), axis=0, keepdims=True)
    slab = o1_ref[...] + jnp.concatenate(
        [s0 * h1b[:, :64], s1 * h1b[:, 64:]], axis=1) + b1g
    g1 = _elu(slab)
    dh2 = jnp.dot(g1, w2_ref[...], preferred_element_type=jnp.float32)

    @pl.when(g == 0)
    def _():
        h2_ref[...] = dh2

    @pl.when(g > 0)
    def _():
        h2_ref[...] += dh2

    @pl.when(g == 3)
    def _():
        h2f = h2_ref[...]
        as2_ref[...] = jnp.dot(h2f, aws2_ref[...],
                               preferred_element_type=jnp.float32)
        ad2_ref[...] = jnp.dot(h2f, awd2_ref[...],
                               preferred_element_type=jnp.float32)


def _dense2(out1t, h1t, asrc16, adst16, dent16, W2, b1, aws2, awd2):
    bn = 1000
    nb = _N // bn
    b1r = b1.reshape(4, 128)
    return pl.pallas_call(
        _dense2_body,
        grid=(nb, 4),
        in_specs=[
            pl.BlockSpec((bn, 128), lambda i, g: (g * nb + i, 0)),
            pl.BlockSpec((bn, 128), lambda i, g: (g * nb + i, 0)),
            pl.BlockSpec((bn, 16), lambda i, g: (i, 0)),
            pl.BlockSpec((bn, 16), lambda i, g: (i, 0)),
            pl.BlockSpec((bn, 16), lambda i, g: (i, 0)),
            pl.BlockSpec((128, 32), lambda i, g: (g, 0)),
            pl.BlockSpec((4, 128), lambda i, g: (0, 0)),
            pl.BlockSpec((32, 16), lambda i, g: (0, 0)),
            pl.BlockSpec((32, 16), lambda i, g: (0, 0)),
        ],
        out_specs=[
            pl.BlockSpec((bn, 32), lambda i, g: (i, 0)),
            pl.BlockSpec((bn, 16), lambda i, g: (i, 0)),
            pl.BlockSpec((bn, 16), lambda i, g: (i, 0)),
        ],
        out_shape=[
            jax.ShapeDtypeStruct((_N, 32), jnp.float32),
            jax.ShapeDtypeStruct((_N, 16), jnp.float32),
            jax.ShapeDtypeStruct((_N, 16), jnp.float32),
        ],
    )(out1t, h1t, asrc16, adst16, dent16, W2, b1r, aws2, awd2)


def _combine2_body(dp_ref, as_ref, ad_ref, dent_ref):
    ex_self = jnp.exp(_vlrelu(as_ref[...] + ad_ref[...]))
    dent_ref[...] = dp_ref[0] + dp_ref[1] + ex_self


def _combine2(denp2, as2_16, ad2_16):
    bn = 1000
    return pl.pallas_call(
        _combine2_body,
        grid=(_N // bn,),
        in_specs=[
            pl.BlockSpec((2, bn, 16), lambda i: (0, i, 0)),
            pl.BlockSpec((bn, 16), lambda i: (i, 0)),
            pl.BlockSpec((bn, 16), lambda i: (i, 0)),
        ],
        out_specs=pl.BlockSpec((bn, 16), lambda i: (i, 0)),
        out_shape=jax.ShapeDtypeStruct((_N, 16), jnp.float32),
    )(denp2, as2_16, ad2_16)


def _dense3_body(op_ref, h2_ref, as2_ref, ad2_ref, dent2_ref,
                 dent1_ref, w3_ref, b2_ref, g_ref, h3_ref, dinv_ref):
    selfc = jnp.exp(_vlrelu(as2_ref[...] + ad2_ref[...])) / (dent2_ref[...]
                                                             + 1e-16)
    out2 = (op_ref[0] + op_ref[1] + selfc[:, 0:1] * h2_ref[...]
            + b2_ref[...])
    g2 = _elu(out2)
    h3 = jnp.dot(g2, w3_ref[...], preferred_element_type=jnp.float32)
    deg = dent1_ref[:, 8:9]
    dinv = lax.rsqrt(deg)
    h3_ref[...] = h3
    g_ref[...] = dinv * h3
    dinv_ref[...] = jnp.broadcast_to(dinv, h3.shape)


def _dense3(out2p, h2, as2_16, ad2_16, dent2, dent1, W3, b2):
    bn = 1000
    b2c = b2.reshape(1, 32)
    return pl.pallas_call(
        _dense3_body,
        grid=(_N // bn,),
        in_specs=[
            pl.BlockSpec((2, bn, 32), lambda i: (0, i, 0)),
            pl.BlockSpec((bn, 32), lambda i: (i, 0)),
            pl.BlockSpec((bn, 16), lambda i: (i, 0)),
            pl.BlockSpec((bn, 16), lambda i: (i, 0)),
            pl.BlockSpec((bn, 16), lambda i: (i, 0)),
            pl.BlockSpec((bn, 16), lambda i: (i, 0)),
            pl.BlockSpec((32, 16), lambda i: (0, 0)),
            pl.BlockSpec((1, 32), lambda i: (0, 0)),
        ],
        out_specs=[
            pl.BlockSpec((bn, 16), lambda i: (i, 0)),
            pl.BlockSpec((bn, 16), lambda i: (i, 0)),
            pl.BlockSpec((bn, 16), lambda i: (i, 0)),
        ],
        out_shape=[
            jax.ShapeDtypeStruct((_N, 16), jnp.float32),
            jax.ShapeDtypeStruct((_N, 16), jnp.float32),
            jax.ShapeDtypeStruct((_N, 16), jnp.float32),
        ],
    )(out2p, h2, as2_16, ad2_16, dent2, dent1, W3, b2c)


def _dense4_body(ap_ref, h3_ref, dinv_ref, b3_ref, out_ref):
    dinv = dinv_ref[...]
    out_ref[...] = (dinv * (ap_ref[0] + ap_ref[1])
                    + dinv * dinv * h3_ref[...] + b3_ref[...])


def _dense4(accp, h3, dinv, b3):
    bn = 1000
    b3c = b3.reshape(1, 16)
    return pl.pallas_call(
        _dense4_body,
        grid=(_N // bn,),
        in_specs=[
            pl.BlockSpec((2, bn, 16), lambda i: (0, i, 0)),
            pl.BlockSpec((bn, 16), lambda i: (i, 0)),
            pl.BlockSpec((bn, 16), lambda i: (i, 0)),
            pl.BlockSpec((1, 16), lambda i: (0, 0)),
        ],
        out_specs=pl.BlockSpec((bn, 16), lambda i: (i, 0)),
        out_shape=jax.ShapeDtypeStruct((_N, 16), jnp.float32),
    )(accp, h3, dinv, b3c)


def _logit_weights16(a):
    heads, ch = a.shape
    eye = jnp.eye(16, dtype=a.dtype)[:heads]
    return (a[:, :, None] * eye[:, None, :]).reshape(heads * ch, 16)


def kernel(x, edge_index, W1, a_src1, a_dst1, b1, W2, a_src2, a_dst2, b2, W3, b3):
    src = edge_index[0]
    dst = edge_index[1]

    aws1 = _logit_weights16(a_src1)
    awd1 = _logit_weights16(a_dst1)
    h1t, asrc16, adst16 = _dense1(x, W1, aws1, awd1)
    den1p = _att_den(asrc16, adst16, src, dst)
    dent1 = _combine1(den1p, asrc16, adst16)
    coef16 = _coef(asrc16, adst16, dent1, src, dst)
    out1t = _gat1_agg(h1t, coef16, src, dst)

    aws2 = _logit_weights16(a_src2)
    awd2 = _logit_weights16(a_dst2)
    h2, as2_16, ad2_16 = _dense2(out1t, h1t, asrc16, adst16, dent1,
                                 W2, b1, aws2, awd2)
    den2p, ex2 = _att_den2(as2_16, ad2_16, src, dst)
    dent2 = _combine2(den2p, as2_16, ad2_16)
    out2p = _gat2_agg(h2, ex2, dent2, src, dst)

    g, h3, dinv = _dense3(out2p, h2, as2_16, ad2_16, dent2, dent1, W3, b2)
    accp = _gcn_gather(g, src, dst)
    out = _dense4(accp, h3, dinv, b3)
    return out


# speedup vs baseline: 12.0665x; 1.1787x over previous
import functools

import jax
import jax.numpy as jnp
from jax import lax
from jax.experimental import pallas as pl
from jax.experimental.pallas import tpu as pltpu
from jax.experimental.pallas import tpu_sc as plsc

_N = 10000
_E = 320000
_H1 = 8
_HID = 64
_OUT2 = 32
_NC = 16

_L = 16
_NCORE = 2
_NSUB = 16

_SC_PARAMS = dict(
    compiler_params=pltpu.CompilerParams(
        use_tc_tiling_on_sc=False, needs_layout_passes=False),
)


def _sc_mesh():
    return plsc.VectorSubcoreMesh(core_axis_name="c", subcore_axis_name="s")


def _zero_rows(ref, nrows):
    z = jnp.zeros((_L,), jnp.float32)

    def body(i, _):
        ref[i] = z
        return 0

    lax.fori_loop(0, nrows, body, 0)


def _zero_i32(ref, n):
    z = jnp.zeros((_L,), jnp.int32)

    def body(i, _):
        ref[pl.ds(i * _L, _L)] = z
        return 0

    lax.fori_loop(0, n // _L, body, 0)


def _fill_iota(ref, n, base):
    nfull = n // _L

    def body(i, _):
        ref[pl.ds(i * _L, _L)] = lax.iota(jnp.int32, _L) + (base + i * _L)
        return 0

    lax.fori_loop(0, nfull, body, 0)
    if n % _L:
        off = n - _L
        ref[pl.ds(off, _L)] = lax.iota(jnp.int32, _L) + (base + off)


def _elu(v):
    return jnp.where(v > 0, v, jnp.exp(v) - 1.0)


def _vlrelu(v):
    return jnp.where(v > 0, v, 0.2 * v)


def _att_den_kernel(asrc_hbm, adst_hbm, src_hbm, dst_hbm, den_hbm,
                    bs, bd, exb, srcv, dstv, spden, sem):
    c = lax.axis_index("c")
    s = lax.axis_index("s")
    B = 1000
    ESH = _E // (_NCORE * _NSUB)
    ebase = (c * _NSUB + s) * ESH

    _zero_rows(bs, B)

    @pl.when(s < 10)
    def _():
        pltpu.sync_copy(bs.at[pl.ds(0, B)], spden.at[pl.ds(s * B, B)])

    plsc.subcore_barrier()

    def chunk(k, _):
        base = ebase + k * B
        pltpu.sync_copy(src_hbm.at[pl.ds(base, B)], srcv)
        pltpu.sync_copy(dst_hbm.at[pl.ds(base, B)], dstv)
        d1 = pltpu.async_copy(asrc_hbm.at[srcv], bs, sem)
        d1.wait()
        pltpu.async_copy(adst_hbm.at[dstv], bd, sem).wait()

        def edge(e, _):
            exb[e] = jnp.exp(_vlrelu(bs[e] + bd[e]))
            return 0

        lax.fori_loop(0, B, edge, 0)
        pltpu.sync_copy(exb, spden.at[dstv], add=True)
        return 0

    lax.fori_loop(0, ESH // B, chunk, 0)
    plsc.subcore_barrier()

    @pl.when(s < 10)
    def _():
        pltpu.sync_copy(spden.at[pl.ds(s * 1000, 1000)],
                        den_hbm.at[c, pl.ds(s * 1000, 1000)])


def _att_den(asrc16, adst16, src, dst):
    B = 1000
    f = pl.kernel(
        _att_den_kernel,
        mesh=_sc_mesh(),
        out_type=jax.ShapeDtypeStruct((_NCORE, _N, 16), jnp.float32),
        scratch_types=[
            pltpu.VMEM((B, 16), jnp.float32),
            pltpu.VMEM((B, 16), jnp.float32),
            pltpu.VMEM((B, 16), jnp.float32),
            pltpu.VMEM((B,), jnp.int32),
            pltpu.VMEM((B,), jnp.int32),
            pltpu.VMEM_SHARED((_N, 16), jnp.float32),
            pltpu.SemaphoreType.DMA,
        ],
        **_SC_PARAMS,
    )
    return f(asrc16, adst16, src, dst)


def _coef_kernel(asrc_hbm, adst_hbm, dent_hbm, src_hbm, dst_hbm, coef_hbm,
                 bs, bd, dn, cfb, srcv, dstv, sem):
    c = lax.axis_index("c")
    s = lax.axis_index("s")
    B = 1000
    ESH = _E // (_NCORE * _NSUB)
    ebase = (c * _NSUB + s) * ESH

    def chunk(k, _):
        base = ebase + k * B
        pltpu.sync_copy(src_hbm.at[pl.ds(base, B)], srcv)
        pltpu.sync_copy(dst_hbm.at[pl.ds(base, B)], dstv)
        d1 = pltpu.async_copy(asrc_hbm.at[srcv], bs, sem)
        d2 = pltpu.async_copy(adst_hbm.at[dstv], bd, sem)
        d1.wait()
        d2.wait()
        pltpu.async_copy(dent_hbm.at[dstv], dn, sem).wait()

        def edge(e, _):
            ex = jnp.exp(_vlrelu(bs[e] + bd[e]))
            cfb[e] = ex / (dn[e] + 1e-16)
            return 0

        lax.fori_loop(0, B, edge, 0)
        pltpu.sync_copy(cfb, coef_hbm.at[pl.ds(base, B)])
        return 0

    lax.fori_loop(0, ESH // B, chunk, 0)


def _coef(asrc16, adst16, dent16, src, dst):
    B = 1000
    f = pl.kernel(
        _coef_kernel,
        mesh=_sc_mesh(),
        out_type=jax.ShapeDtypeStruct((_E, 16), jnp.float32),
        scratch_types=[
            pltpu.VMEM((B, 16), jnp.float32),
            pltpu.VMEM((B, 16), jnp.float32),
            pltpu.VMEM((B, 16), jnp.float32),
            pltpu.VMEM((B, 16), jnp.float32),
            pltpu.VMEM((B,), jnp.int32),
            pltpu.VMEM((B,), jnp.int32),
            pltpu.SemaphoreType.DMA,
        ],
        **_SC_PARAMS,
    )
    return f(asrc16, adst16, dent16, src, dst)


def _gat1_agg_kernel(h1t_hbm, coef_hbm, src_hbm, dst_hbm, out_hbm,
                     table, pend_src, pend_ld, pend_eid, srcv, dstv,
                     rows, cfb, sem):
    c = lax.axis_index("c")
    s = lax.axis_index("s")
    w = c * _NSUB + s
    B = 1600
    G = 128
    NP = B + 2 * G
    KCH = _E // B
    RSZ = 624

    _zero_i32(pend_src, NP)
    _zero_i32(pend_eid, NP)
    _zero_i32(pend_ld, NP)

    def combo(q, _):
        cid = w * 2 + q
        r = cid % 16
        g = cid // 16
        lo = r * RSZ
        hi = jnp.where(r == 15, _N, lo + RSZ)
        gbase = g * _N

        zv = jnp.zeros((_L,), jnp.float32)

        def zrow(i, _):
            for t in range(8):
                table[i, pl.ds(t * _L, _L)] = zv
            return 0

        lax.fori_loop(0, 640, zrow, 0)

        col0 = jnp.full((_L,), 2 * g, jnp.int32)
        col1 = col0 + 1

        def chunk(k, np_):
            base = k * B
            dc1 = pltpu.async_copy(src_hbm.at[pl.ds(base, B)], srcv, sem)
            dc2 = pltpu.async_copy(dst_hbm.at[pl.ds(base, B)], dstv, sem)
            dc1.wait()
            dc2.wait()

            def scan(v, cnt):
                d = dstv[pl.ds(v * _L, _L)]
                m = (d >= lo) & (d < hi)
                csum = plsc.cumsum(m.astype(jnp.int32))
                pos = cnt + csum - 1
                plsc.store_scatter(pend_ld, [pos], d - lo, mask=m)
                plsc.store_scatter(pend_src, [pos],
                                   srcv[pl.ds(v * _L, _L)] + gbase, mask=m)
                plsc.store_scatter(pend_eid, [pos],
                                   lax.iota(jnp.int32, _L) + (base + v * _L),
                                   mask=m)
                return cnt + jnp.max(csum)

            cnt = lax.fori_loop(0, B // _L, scan, np_)
            nf = jnp.where(k == KCH - 1, (cnt + G - 1) // G, cnt // G)

            def flush(b, _):
                off = b * G
                d1 = pltpu.async_copy(h1t_hbm.at[pend_src.at[pl.ds(off, G)]],
                                      rows, sem)
                d2 = pltpu.async_copy(coef_hbm.at[pend_eid.at[pl.ds(off, G)]],
                                      cfb, sem)
                d1.wait()
                d2.wait()
                nin = jnp.minimum(cnt - off, G)

                def acc(i, _):
                    ld = pend_ld[pl.ds(off + i, _L)][0]
                    iv = jnp.full((_L,), i, jnp.int32)
                    cf0 = plsc.load_gather(cfb, [iv, col0])
                    cf1 = plsc.load_gather(cfb, [iv, col1])
                    for t in range(8):
                        cf = cf0 if t < 4 else cf1
                        plsc.addupdate(table.at[ld, pl.ds(t * _L, _L)],
                                       cf * rows[i, pl.ds(t * _L, _L)])
                    return 0

                lax.fori_loop(0, nin, acc, 0)
                return 0

            lax.fori_loop(0, nf, flush, 0)
            rem = jnp.maximum(cnt - nf * G, 0)

            @pl.when(nf > 0)
            def _():
                fb = nf * G
                for jj in range(8):
                    sl = pl.ds(jj * _L, _L)
                    sr = pl.ds(fb + jj * _L, _L)
                    pend_ld[sl] = pend_ld[sr]
                    pend_src[sl] = pend_src[sr]
                    pend_eid[sl] = pend_eid[sr]

            return rem

        lax.fori_loop(0, KCH, chunk, jnp.int32(0))

        pltpu.sync_copy(table.at[pl.ds(0, RSZ)],
                        out_hbm.at[pl.ds(gbase + lo, RSZ)])

        @pl.when(r == 15)
        def _():
            pltpu.sync_copy(table.at[pl.ds(RSZ, 16)],
                            out_hbm.at[pl.ds(gbase + lo + RSZ, 16)])

        return 0

    lax.fori_loop(0, 2, combo, 0)


def _gat1_agg(h1t, coef16, src, dst):
    B = 1600
    G = 128
    f = pl.kernel(
        _gat1_agg_kernel,
        mesh=_sc_mesh(),
        out_type=jax.ShapeDtypeStruct((4 * _N, 128), jnp.float32),
        scratch_types=[
            pltpu.VMEM((640, 128), jnp.float32),
            pltpu.VMEM((B + 2 * G,), jnp.int32),
            pltpu.VMEM((B + 2 * G,), jnp.int32),
            pltpu.VMEM((B + 2 * G,), jnp.int32),
            pltpu.VMEM((B,), jnp.int32),
            pltpu.VMEM((B,), jnp.int32),
            pltpu.VMEM((G, 128), jnp.float32),
            pltpu.VMEM((G, 16), jnp.float32),
            pltpu.SemaphoreType.DMA,
        ],
        **_SC_PARAMS,
    )
    return f(h1t, coef16, src, dst)


def _att_den2_kernel(asrc_hbm, adst_hbm, src_hbm, dst_hbm, den_hbm, ex_hbm,
                     bs, bd, exb, ex2v, srcv, dstv, spden, sem):
    c = lax.axis_index("c")
    s = lax.axis_index("s")
    B = 1000
    ESH = _E // (_NCORE * _NSUB)
    ebase = (c * _NSUB + s) * ESH

    _zero_rows(bs, B)

    @pl.when(s < 10)
    def _():
        pltpu.sync_copy(bs.at[pl.ds(0, B)], spden.at[pl.ds(s * B, B)])

    plsc.subcore_barrier()

    def chunk(k, _):
        base = ebase + k * B
        pltpu.sync_copy(src_hbm.at[pl.ds(base, B)], srcv)
        pltpu.sync_copy(dst_hbm.at[pl.ds(base, B)], dstv)
        d1 = pltpu.async_copy(asrc_hbm.at[srcv], bs, sem)
        d1.wait()
        pltpu.async_copy(adst_hbm.at[dstv], bd, sem).wait()

        def edge(e, _):
            exb[e] = jnp.exp(_vlrelu(bs[e] + bd[e]))
            return 0

        lax.fori_loop(0, B, edge, 0)
        pltpu.sync_copy(exb, spden.at[dstv], add=True)
        for jj in range(63):
            off = jj * _L if jj < 62 else B - _L
            rowv = lax.iota(jnp.int32, _L) + off
            ex2v[pl.ds(off, _L)] = plsc.load_gather(
                exb, [rowv, jnp.zeros((_L,), jnp.int32)])
        pltpu.sync_copy(ex2v, ex_hbm.at[pl.ds(base, B)])
        return 0

    lax.fori_loop(0, ESH // B, chunk, 0)
    plsc.subcore_barrier()

    @pl.when(s < 10)
    def _():
        pltpu.sync_copy(spden.at[pl.ds(s * 1000, 1000)],
                        den_hbm.at[c, pl.ds(s * 1000, 1000)])


def _att_den2(asrc16, adst16, src, dst):
    B = 1000
    f = pl.kernel(
        _att_den2_kernel,
        mesh=_sc_mesh(),
        out_type=[
            jax.ShapeDtypeStruct((_NCORE, _N, 16), jnp.float32),
            jax.ShapeDtypeStruct((_E,), jnp.float32),
        ],
        scratch_types=[
            pltpu.VMEM((B, 16), jnp.float32),
            pltpu.VMEM((B, 16), jnp.float32),
            pltpu.VMEM((B, 16), jnp.float32),
            pltpu.VMEM((B,), jnp.float32),
            pltpu.VMEM((B,), jnp.int32),
            pltpu.VMEM((B,), jnp.int32),
            pltpu.VMEM_SHARED((_N, 16), jnp.float32),
            pltpu.SemaphoreType.DMA,
        ],
        **_SC_PARAMS,
    )
    return f(asrc16, adst16, src, dst)


def _gat2_agg_kernel(h2_hbm, ex_hbm, dent_hbm, src_hbm, dst_hbm, out_hbm,
                     table, pend_src, pend_ld, pend_cf, srcv, dstv, exv,
                     dn, rows, idxv, spacc, sem):
    c = lax.axis_index("c")
    s = lax.axis_index("s")
    r = s // 4
    j = s % 4
    B = 800
    G = 128
    RNG = 2500
    lo = r * RNG
    ESH = _E // 8
    ebase = c * (_E // 2) + j * ESH

    zv = jnp.zeros((_L,), jnp.float32)

    def zrow(i, _):
        table[i, pl.ds(0, _L)] = zv
        table[i, pl.ds(_L, _L)] = zv
        return 0

    lax.fori_loop(0, RNG, zrow, 0)
    _zero_i32(pend_src, B + 2 * G)

    @pl.when(s < 10)
    def _():
        pltpu.sync_copy(table.at[pl.ds(0, 1000)],
                        spacc.at[pl.ds(s * 1000, 1000)])

    plsc.subcore_barrier()

    KCH = ESH // B

    def chunk(k, np_):
        base = ebase + k * B
        pltpu.sync_copy(src_hbm.at[pl.ds(base, B)], srcv)
        pltpu.sync_copy(dst_hbm.at[pl.ds(base, B)], dstv)
        pltpu.sync_copy(ex_hbm.at[pl.ds(base, B)], exv)
        pltpu.async_copy(dent_hbm.at[dstv], dn, sem).wait()

        def scan(v, cnt):
            d = dstv[pl.ds(v * _L, _L)]
            m = (d >= lo) & (d < lo + RNG)
            rowv = lax.iota(jnp.int32, _L) + v * _L
            dnv = plsc.load_gather(dn, [rowv, jnp.zeros((_L,), jnp.int32)])
            cf = exv[pl.ds(v * _L, _L)] / (dnv + 1e-16)
            csum = plsc.cumsum(m.astype(jnp.int32))
            pos = cnt + csum - 1
            plsc.store_scatter(pend_ld, [pos], d - lo, mask=m)
            plsc.store_scatter(pend_src, [pos], srcv[pl.ds(v * _L, _L)],
                               mask=m)
            plsc.store_scatter(pend_cf, [pos], cf, mask=m)
            return cnt + jnp.max(csum)

        cnt = lax.fori_loop(0, B // _L, scan, np_)
        nf = jnp.where(k == KCH - 1, (cnt + G - 1) // G, cnt // G)

        def flush(b, _):
            off = b * G
            pltpu.async_copy(h2_hbm.at[pend_src.at[pl.ds(off, G)]], rows,
                             sem).wait()
            nin = jnp.minimum(cnt - off, G)

            def acc(i, _):
                ld = pend_ld[pl.ds(off + i, _L)][0]
                cf = jnp.full((_L,), pend_cf[pl.ds(off + i, _L)][0])
                plsc.addupdate(table.at[ld, pl.ds(0, _L)],
                               cf * rows[i, pl.ds(0, _L)])
                plsc.addupdate(table.at[ld, pl.ds(_L, _L)],
                               cf * rows[i, pl.ds(_L, _L)])
                return 0

            lax.fori_loop(0, nin, acc, 0)
            return 0

        lax.fori_loop(0, nf, flush, 0)
        rem = jnp.maximum(cnt - nf * G, 0)

        @pl.when(nf > 0)
        def _():
            fb = nf * G
            for jj in range(8):
                sl = pl.ds(jj * _L, _L)
                sr = pl.ds(fb + jj * _L, _L)
                pend_ld[sl] = pend_ld[sr]
                pend_src[sl] = pend_src[sr]
                pend_cf[sl] = pend_cf[sr]

        return rem

    lax.fori_loop(0, ESH // B, chunk, jnp.int32(0))

    _fill_iota(idxv, RNG, lo)
    pltpu.sync_copy(table, spacc.at[idxv], add=True)
    plsc.subcore_barrier()

    @pl.when(s < 10)
    def _():
        pltpu.sync_copy(spacc.at[pl.ds(s * 1000, 1000)],
                        out_hbm.at[c, pl.ds(s * 1000, 1000)])


def _gat2_agg(h2, ex2, dent2, src, dst):
    B = 800
    G = 128
    RNG = 2500
    f = pl.kernel(
        _gat2_agg_kernel,
        mesh=_sc_mesh(),
        out_type=jax.ShapeDtypeStruct((_NCORE, _N, 32), jnp.float32),
        scratch_types=[
            pltpu.VMEM((RNG, 32), jnp.float32),
            pltpu.VMEM((B + 2 * G,), jnp.int32),
            pltpu.VMEM((B + 2 * G,), jnp.int32),
            pltpu.VMEM((B + 2 * G,), jnp.float32),
            pltpu.VMEM((B,), jnp.int32),
            pltpu.VMEM((B,), jnp.int32),
            pltpu.VMEM((B,), jnp.float32),
            pltpu.VMEM((B, 16), jnp.float32),
            pltpu.VMEM((G, 32), jnp.float32),
            pltpu.VMEM((RNG,), jnp.int32),
            pltpu.VMEM_SHARED((_N, 32), jnp.float32),
            pltpu.SemaphoreType.DMA,
        ],
        **_SC_PARAMS,
    )
    return f(h2, ex2, dent2, src, dst)


def _gcn_gather_kernel(g_hbm, src_hbm, dst_hbm, out_hbm,
                       table, pend_src, pend_ld, srcv, dstv, rows, idxv,
                       spmem, sem):
    c = lax.axis_index("c")
    s = lax.axis_index("s")
    r = s // 4
    j = s % 4
    RNG = 2500
    ESH = _E // 8
    B = 1600
    G = 128
    KCH = ESH // B
    lo = r * RNG
    ebase = c * (_E // 2) + j * ESH

    _zero_rows(table, RNG)
    _zero_i32(pend_src, B + 2 * G)
    zrows = 1000

    @pl.when(s < 10)
    def _():
        pltpu.sync_copy(table.at[pl.ds(0, zrows)],
                        spmem.at[pl.ds(s * zrows, zrows)])

    plsc.subcore_barrier()

    def chunk_body(k, np_):
        pltpu.sync_copy(src_hbm.at[pl.ds(ebase + k * B, B)], srcv)
        pltpu.sync_copy(dst_hbm.at[pl.ds(ebase + k * B, B)], dstv)

        def scan_body(v, cnt):
            d = dstv[pl.ds(v * _L, _L)]
            m = (d >= lo) & (d < lo + RNG)
            csum = plsc.cumsum(m.astype(jnp.int32))
            pos = cnt + csum - 1
            plsc.store_scatter(pend_ld, [pos], d - lo, mask=m)
            plsc.store_scatter(pend_src, [pos], srcv[pl.ds(v * _L, _L)],
                               mask=m)
            return cnt + jnp.max(csum)

        cnt = lax.fori_loop(0, B // _L, scan_body, np_)
        nf = jnp.where(k == KCH - 1, (cnt + G - 1) // G, cnt // G)

        def flush_body(b, _):
            off = b * G
            pltpu.async_copy(g_hbm.at[pend_src.at[pl.ds(off, G)]], rows,
                             sem).wait()
            nin = jnp.minimum(cnt - off, G)

            def acc_body(i, _):
                ld = pend_ld[pl.ds(off + i, _L)][0]
                table[ld] = table[ld] + rows[i]
                return 0

            lax.fori_loop(0, nin, acc_body, 0)
            return 0

        lax.fori_loop(0, nf, flush_body, 0)
        rem = jnp.maximum(cnt - nf * G, 0)

        @pl.when(nf > 0)
        def _():
            fb = nf * G
            for jj in range(8):
                sl = pl.ds(jj * _L, _L)
                sr = pl.ds(fb + jj * _L, _L)
                pend_ld[sl] = pend_ld[sr]
                pend_src[sl] = pend_src[sr]

        return rem

    lax.fori_loop(0, ESH // B, chunk_body, jnp.int32(0))

    _fill_iota(idxv, RNG, lo)
    pltpu.sync_copy(table, spmem.at[idxv], add=True)
    plsc.subcore_barrier()

    @pl.when(s < 10)
    def _():
        pltpu.sync_copy(spmem.at[pl.ds(s * zrows, zrows)],
                        out_hbm.at[c, pl.ds(s * zrows, zrows)])


def _gcn_gather(g, src, dst):
    B = 1600
    G = 128
    RNG = 2500
    f = pl.kernel(
        _gcn_gather_kernel,
        mesh=_sc_mesh(),
        out_type=jax.ShapeDtypeStruct((_NCORE, _N, 16), jnp.float32),
        scratch_types=[
            pltpu.VMEM((RNG, 16), jnp.float32),
            pltpu.VMEM((B + 2 * G,), jnp.int32),
            pltpu.VMEM((B + 2 * G,), jnp.int32),
            pltpu.VMEM((B,), jnp.int32),
            pltpu.VMEM((B,), jnp.int32),
            pltpu.VMEM((G, 16), jnp.float32),
            pltpu.VMEM((RNG,), jnp.int32),
            pltpu.VMEM_SHARED((_N, 16), jnp.float32),
            pltpu.SemaphoreType.DMA,
        ],
        **_SC_PARAMS,
    )
    return f(g, src, dst)


def _dense1_body(x_ref, w_ref, aws_ref, awd_ref, h_ref, as_ref, ad_ref):
    g = pl.program_id(1)
    h = jnp.dot(x_ref[...], w_ref[...], preferred_element_type=jnp.float32)
    h_ref[...] = h
    das = jnp.dot(h, aws_ref[...], preferred_element_type=jnp.float32)
    dad = jnp.dot(h, awd_ref[...], preferred_element_type=jnp.float32)

    @pl.when(g == 0)
    def _():
        as_ref[...] = das
        ad_ref[...] = dad

    @pl.when(g > 0)
    def _():
        as_ref[...] += das
        ad_ref[...] += dad


def _dense1(x, W1, aws, awd):
    bn = 1000
    return pl.pallas_call(
        _dense1_body,
        grid=(_N // bn, 4),
        in_specs=[
            pl.BlockSpec((bn, 128), lambda i, g: (i, 0)),
            pl.BlockSpec((128, 128), lambda i, g: (0, g)),
            pl.BlockSpec((128, 16), lambda i, g: (g, 0)),
            pl.BlockSpec((128, 16), lambda i, g: (g, 0)),
        ],
        out_specs=[
            pl.BlockSpec((bn, 128), lambda i, g: (g * (_N // bn) + i, 0)),
            pl.BlockSpec((bn, 16), lambda i, g: (i, 0)),
            pl.BlockSpec((bn, 16), lambda i, g: (i, 0)),
        ],
        out_shape=[
            jax.ShapeDtypeStruct((4 * _N, 128), jnp.float32),
            jax.ShapeDtypeStruct((_N, 16), jnp.float32),
            jax.ShapeDtypeStruct((_N, 16), jnp.float32),
        ],
    )(x, W1, aws, awd)


def _combine1_body(dp_ref, as_ref, ad_ref, dent_ref):
    ex_self = jnp.exp(_vlrelu(as_ref[...] + ad_ref[...]))
    dent_ref[...] = dp_ref[0] + dp_ref[1] + ex_self


def _combine1(denp, asrc16, adst16):
    bn = 1000
    return pl.pallas_call(
        _combine1_body,
        grid=(_N // bn,),
        in_specs=[
            pl.BlockSpec((2, bn, 16), lambda i: (0, i, 0)),
            pl.BlockSpec((bn, 16), lambda i: (i, 0)),
            pl.BlockSpec((bn, 16), lambda i: (i, 0)),
        ],
        out_specs=pl.BlockSpec((bn, 16), lambda i: (i, 0)),
        out_shape=jax.ShapeDtypeStruct((_N, 16), jnp.float32),
    )(denp, asrc16, adst16)


def _dense2_body(o1_ref, h1_ref, as_ref, ad_ref, dent_ref, w2_ref, b1_ref,
                 aws2_ref, awd2_ref, h2_ref, as2_ref, ad2_ref):
    g = pl.program_id(1)
    selfc = jnp.exp(_vlrelu(as_ref[...] + ad_ref[...])) / (dent_ref[...] + 1e-16)
    col = lax.broadcasted_iota(jnp.int32, selfc.shape, 1)
    s0 = jnp.sum(jnp.where(col == 2 * g, selfc, 0.0), axis=1, keepdims=True)
    s1 = jnp.sum(jnp.where(col == 2 * g + 1, selfc, 0.0), axis=1,
                 keepdims=True)
    h1b = h1_ref[...]
    b1full = b1_ref[...]
    row = lax.broadcasted_iota(jnp.int32, b1full.shape, 0)
    b1g = jnp.sum(jnp.where(row == g, b1full, 0.0), axis=0, keepdims=True)
    slab = o1_ref[...] + jnp.concatenate(
        [s0 * h1b[:, :64], s1 * h1b[:, 64:]], axis=1) + b1g
    g1 = _elu(slab)
    dh2 = jnp.dot(g1, w2_ref[...], preferred_element_type=jnp.float32)

    @pl.when(g == 0)
    def _():
        h2_ref[...] = dh2

    @pl.when(g > 0)
    def _():
        h2_ref[...] += dh2

    @pl.when(g == 3)
    def _():
        h2f = h2_ref[...]
        as2_ref[...] = jnp.dot(h2f, aws2_ref[...],
                               preferred_element_type=jnp.float32)
        ad2_ref[...] = jnp.dot(h2f, awd2_ref[...],
                               preferred_element_type=jnp.float32)


def _dense2(out1t, h1t, asrc16, adst16, dent16, W2, b1, aws2, awd2):
    bn = 1000
    nb = _N // bn
    b1r = b1.reshape(4, 128)
    return pl.pallas_call(
        _dense2_body,
        grid=(nb, 4),
        in_specs=[
            pl.BlockSpec((bn, 128), lambda i, g: (g * nb + i, 0)),
            pl.BlockSpec((bn, 128), lambda i, g: (g * nb + i, 0)),
            pl.BlockSpec((bn, 16), lambda i, g: (i, 0)),
            pl.BlockSpec((bn, 16), lambda i, g: (i, 0)),
            pl.BlockSpec((bn, 16), lambda i, g: (i, 0)),
            pl.BlockSpec((128, 32), lambda i, g: (g, 0)),
            pl.BlockSpec((4, 128), lambda i, g: (0, 0)),
            pl.BlockSpec((32, 16), lambda i, g: (0, 0)),
            pl.BlockSpec((32, 16), lambda i, g: (0, 0)),
        ],
        out_specs=[
            pl.BlockSpec((bn, 32), lambda i, g: (i, 0)),
            pl.BlockSpec((bn, 16), lambda i, g: (i, 0)),
            pl.BlockSpec((bn, 16), lambda i, g: (i, 0)),
        ],
        out_shape=[
            jax.ShapeDtypeStruct((_N, 32), jnp.float32),
            jax.ShapeDtypeStruct((_N, 16), jnp.float32),
            jax.ShapeDtypeStruct((_N, 16), jnp.float32),
        ],
    )(out1t, h1t, asrc16, adst16, dent16, W2, b1r, aws2, awd2)


def _combine2_body(dp_ref, as_ref, ad_ref, dent_ref):
    ex_self = jnp.exp(_vlrelu(as_ref[...] + ad_ref[...]))
    dent_ref[...] = dp_ref[0] + dp_ref[1] + ex_self


def _combine2(denp2, as2_16, ad2_16):
    bn = 1000
    return pl.pallas_call(
        _combine2_body,
        grid=(_N // bn,),
        in_specs=[
            pl.BlockSpec((2, bn, 16), lambda i: (0, i, 0)),
            pl.BlockSpec((bn, 16), lambda i: (i, 0)),
            pl.BlockSpec((bn, 16), lambda i: (i, 0)),
        ],
        out_specs=pl.BlockSpec((bn, 16), lambda i: (i, 0)),
        out_shape=jax.ShapeDtypeStruct((_N, 16), jnp.float32),
    )(denp2, as2_16, ad2_16)


def _dense3_body(op_ref, h2_ref, as2_ref, ad2_ref, dent2_ref,
                 dent1_ref, w3_ref, b2_ref, g_ref, h3_ref, dinv_ref):
    selfc = jnp.exp(_vlrelu(as2_ref[...] + ad2_ref[...])) / (dent2_ref[...]
                                                             + 1e-16)
    out2 = (op_ref[0] + op_ref[1] + selfc[:, 0:1] * h2_ref[...]
            + b2_ref[...])
    g2 = _elu(out2)
    h3 = jnp.dot(g2, w3_ref[...], preferred_element_type=jnp.float32)
    deg = dent1_ref[:, 8:9]
    dinv = lax.rsqrt(deg)
    h3_ref[...] = h3
    g_ref[...] = dinv * h3
    dinv_ref[...] = jnp.broadcast_to(dinv, h3.shape)


def _dense3(out2p, h2, as2_16, ad2_16, dent2, dent1, W3, b2):
    bn = 1000
    b2c = b2.reshape(1, 32)
    return pl.pallas_call(
        _dense3_body,
        grid=(_N // bn,),
        in_specs=[
            pl.BlockSpec((2, bn, 32), lambda i: (0, i, 0)),
            pl.BlockSpec((bn, 32), lambda i: (i, 0)),
            pl.BlockSpec((bn, 16), lambda i: (i, 0)),
            pl.BlockSpec((bn, 16), lambda i: (i, 0)),
            pl.BlockSpec((bn, 16), lambda i: (i, 0)),
            pl.BlockSpec((bn, 16), lambda i: (i, 0)),
            pl.BlockSpec((32, 16), lambda i: (0, 0)),
            pl.BlockSpec((1, 32), lambda i: (0, 0)),
        ],
        out_specs=[
            pl.BlockSpec((bn, 16), lambda i: (i, 0)),
            pl.BlockSpec((bn, 16), lambda i: (i, 0)),
            pl.BlockSpec((bn, 16), lambda i: (i, 0)),
        ],
        out_shape=[
            jax.ShapeDtypeStruct((_N, 16), jnp.float32),
            jax.ShapeDtypeStruct((_N, 16), jnp.float32),
            jax.ShapeDtypeStruct((_N, 16), jnp.float32),
        ],
    )(out2p, h2, as2_16, ad2_16, dent2, dent1, W3, b2c)


def _dense4_body(ap_ref, h3_ref, dinv_ref, b3_ref, out_ref):
    dinv = dinv_ref[...]
    out_ref[...] = (dinv * (ap_ref[0] + ap_ref[1])
                    + dinv * dinv * h3_ref[...] + b3_ref[...])


def _dense4(accp, h3, dinv, b3):
    bn = 1000
    b3c = b3.reshape(1, 16)
    return pl.pallas_call(
        _dense4_body,
        grid=(_N // bn,),
        in_specs=[
            pl.BlockSpec((2, bn, 16), lambda i: (0, i, 0)),
            pl.BlockSpec((bn, 16), lambda i: (i, 0)),
            pl.BlockSpec((bn, 16), lambda i: (i, 0)),
            pl.BlockSpec((1, 16), lambda i: (0, 0)),
        ],
        out_specs=pl.BlockSpec((bn, 16), lambda i: (i, 0)),
        out_shape=jax.ShapeDtypeStruct((_N, 16), jnp.float32),
    )(accp, h3, dinv, b3c)


def _logit_weights16(a):
    heads, ch = a.shape
    eye = jnp.eye(16, dtype=a.dtype)[:heads]
    return (a[:, :, None] * eye[:, None, :]).reshape(heads * ch, 16)


def kernel(x, edge_index, W1, a_src1, a_dst1, b1, W2, a_src2, a_dst2, b2, W3, b3):
    src = edge_index[0]
    dst = edge_index[1]

    aws1 = _logit_weights16(a_src1)
    awd1 = _logit_weights16(a_dst1)
    h1t, asrc16, adst16 = _dense1(x, W1, aws1, awd1)
    den1p = _att_den(asrc16, adst16, src, dst)
    dent1 = _combine1(den1p, asrc16, adst16)
    coef16 = _coef(asrc16, adst16, dent1, src, dst)
    out1t = _gat1_agg(h1t, coef16, src, dst)

    aws2 = _logit_weights16(a_src2)
    awd2 = _logit_weights16(a_dst2)
    h2, as2_16, ad2_16 = _dense2(out1t, h1t, asrc16, adst16, dent1,
                                 W2, b1, aws2, awd2)
    den2p, ex2 = _att_den2(as2_16, ad2_16, src, dst)
    dent2 = _combine2(den2p, as2_16, ad2_16)
    out2p = _gat2_agg(h2, ex2, dent2, src, dst)

    g, h3, dinv = _dense3(out2p, h2, as2_16, ad2_16, dent2, dent1, W3, b2)
    accp = _gcn_gather(g, src, dst)
    out = _dense4(accp, h3, dinv, b3)
    return out


# speedup vs baseline: 12.0669x; 1.0000x over previous
import functools

import jax
import jax.numpy as jnp
from jax import lax
from jax.experimental import pallas as pl
from jax.experimental.pallas import tpu as pltpu
from jax.experimental.pallas import tpu_sc as plsc

_N = 10000
_E = 320000
_H1 = 8
_HID = 64
_OUT2 = 32
_NC = 16

_L = 16
_NCORE = 2
_NSUB = 16

_SC_PARAMS = dict(
    compiler_params=pltpu.CompilerParams(
        use_tc_tiling_on_sc=False, needs_layout_passes=False),
)


def _sc_mesh():
    return plsc.VectorSubcoreMesh(core_axis_name="c", subcore_axis_name="s")


def _zero_rows(ref, nrows):
    z = jnp.zeros((_L,), jnp.float32)

    def body(i, _):
        ref[i] = z
        return 0

    lax.fori_loop(0, nrows, body, 0)


def _zero_i32(ref, n):
    z = jnp.zeros((_L,), jnp.int32)

    def body(i, _):
        ref[pl.ds(i * _L, _L)] = z
        return 0

    lax.fori_loop(0, n // _L, body, 0)


def _fill_iota(ref, n, base):
    nfull = n // _L

    def body(i, _):
        ref[pl.ds(i * _L, _L)] = lax.iota(jnp.int32, _L) + (base + i * _L)
        return 0

    lax.fori_loop(0, nfull, body, 0)
    if n % _L:
        off = n - _L
        ref[pl.ds(off, _L)] = lax.iota(jnp.int32, _L) + (base + off)


def _elu(v):
    return jnp.where(v > 0, v, jnp.exp(v) - 1.0)


def _vlrelu(v):
    return jnp.where(v > 0, v, 0.2 * v)


def _att_den_kernel(asrc_hbm, adst_hbm, src_hbm, dst_hbm, den_hbm,
                    bs, bd, exb, srcv, dstv, spden, sem):
    c = lax.axis_index("c")
    s = lax.axis_index("s")
    B = 1000
    ESH = _E // (_NCORE * _NSUB)
    ebase = (c * _NSUB + s) * ESH

    _zero_rows(bs, B)

    @pl.when(s < 10)
    def _():
        pltpu.sync_copy(bs.at[pl.ds(0, B)], spden.at[pl.ds(s * B, B)])

    plsc.subcore_barrier()

    def chunk(k, _):
        base = ebase + k * B
        pltpu.sync_copy(src_hbm.at[pl.ds(base, B)], srcv)
        pltpu.sync_copy(dst_hbm.at[pl.ds(base, B)], dstv)
        d1 = pltpu.async_copy(asrc_hbm.at[srcv], bs, sem)
        d1.wait()
        pltpu.async_copy(adst_hbm.at[dstv], bd, sem).wait()

        def edge(e, _):
            exb[e] = jnp.exp(_vlrelu(bs[e] + bd[e]))
            return 0

        lax.fori_loop(0, B, edge, 0)
        pltpu.sync_copy(exb, spden.at[dstv], add=True)
        return 0

    lax.fori_loop(0, ESH // B, chunk, 0)
    plsc.subcore_barrier()

    @pl.when(s < 10)
    def _():
        pltpu.sync_copy(spden.at[pl.ds(s * 1000, 1000)],
                        den_hbm.at[c, pl.ds(s * 1000, 1000)])


def _att_den(asrc16, adst16, src, dst):
    B = 1000
    f = pl.kernel(
        _att_den_kernel,
        mesh=_sc_mesh(),
        out_type=jax.ShapeDtypeStruct((_NCORE, _N, 16), jnp.float32),
        scratch_types=[
            pltpu.VMEM((B, 16), jnp.float32),
            pltpu.VMEM((B, 16), jnp.float32),
            pltpu.VMEM((B, 16), jnp.float32),
            pltpu.VMEM((B,), jnp.int32),
            pltpu.VMEM((B,), jnp.int32),
            pltpu.VMEM_SHARED((_N, 16), jnp.float32),
            pltpu.SemaphoreType.DMA,
        ],
        **_SC_PARAMS,
    )
    return f(asrc16, adst16, src, dst)


def _coef_kernel(asrc_hbm, adst_hbm, dent_hbm, src_hbm, dst_hbm, coef_hbm,
                 bs, bd, dn, cfb, srcv, dstv, sem):
    c = lax.axis_index("c")
    s = lax.axis_index("s")
    B = 1000
    ESH = _E // (_NCORE * _NSUB)
    ebase = (c * _NSUB + s) * ESH

    def chunk(k, _):
        base = ebase + k * B
        pltpu.sync_copy(src_hbm.at[pl.ds(base, B)], srcv)
        pltpu.sync_copy(dst_hbm.at[pl.ds(base, B)], dstv)
        d1 = pltpu.async_copy(asrc_hbm.at[srcv], bs, sem)
        d2 = pltpu.async_copy(adst_hbm.at[dstv], bd, sem)
        d1.wait()
        d2.wait()
        pltpu.async_copy(dent_hbm.at[dstv], dn, sem).wait()

        def edge(e, _):
            ex = jnp.exp(_vlrelu(bs[e] + bd[e]))
            cfb[e] = ex / (dn[e] + 1e-16)
            return 0

        lax.fori_loop(0, B, edge, 0)
        pltpu.sync_copy(cfb, coef_hbm.at[pl.ds(base, B)])
        return 0

    lax.fori_loop(0, ESH // B, chunk, 0)


def _coef(asrc16, adst16, dent16, src, dst):
    B = 1000
    f = pl.kernel(
        _coef_kernel,
        mesh=_sc_mesh(),
        out_type=jax.ShapeDtypeStruct((_E, 16), jnp.float32),
        scratch_types=[
            pltpu.VMEM((B, 16), jnp.float32),
            pltpu.VMEM((B, 16), jnp.float32),
            pltpu.VMEM((B, 16), jnp.float32),
            pltpu.VMEM((B, 16), jnp.float32),
            pltpu.VMEM((B,), jnp.int32),
            pltpu.VMEM((B,), jnp.int32),
            pltpu.SemaphoreType.DMA,
        ],
        **_SC_PARAMS,
    )
    return f(asrc16, adst16, dent16, src, dst)


def _gat1_agg_kernel(h1t_hbm, coef_hbm, src_hbm, dst_hbm, out_hbm,
                     table, pend_src, pend_ld, pend_eid, srcv, dstv,
                     rows, cfb, sem):
    c = lax.axis_index("c")
    s = lax.axis_index("s")
    w = c * _NSUB + s
    B = 1600
    G = 128
    NP = B + 2 * G
    KCH = _E // B
    RSZ = 624

    _zero_i32(pend_src, NP)
    _zero_i32(pend_eid, NP)
    _zero_i32(pend_ld, NP)

    def combo(q, _):
        cid = w * 2 + q
        r = cid % 16
        g = cid // 16
        lo = r * RSZ
        hi = jnp.where(r == 15, _N, lo + RSZ)
        gbase = g * _N

        zv = jnp.zeros((_L,), jnp.float32)

        def zrow(i, _):
            for t in range(8):
                table[i, pl.ds(t * _L, _L)] = zv
            return 0

        lax.fori_loop(0, 640, zrow, 0)

        col0 = jnp.full((_L,), 2 * g, jnp.int32)
        col1 = col0 + 1

        def chunk(k, np_):
            base = k * B
            dc1 = pltpu.async_copy(src_hbm.at[pl.ds(base, B)], srcv, sem)
            dc2 = pltpu.async_copy(dst_hbm.at[pl.ds(base, B)], dstv, sem)
            dc1.wait()
            dc2.wait()

            def scan(v, cnt):
                d = dstv[pl.ds(v * _L, _L)]
                m = (d >= lo) & (d < hi)
                csum = plsc.cumsum(m.astype(jnp.int32))
                pos = cnt + csum - 1
                plsc.store_scatter(pend_ld, [pos], d - lo, mask=m)
                plsc.store_scatter(pend_src, [pos],
                                   srcv[pl.ds(v * _L, _L)] + gbase, mask=m)
                plsc.store_scatter(pend_eid, [pos],
                                   lax.iota(jnp.int32, _L) + (base + v * _L),
                                   mask=m)
                return cnt + jnp.max(csum)

            cnt = lax.fori_loop(0, B // _L, scan, np_)
            nf = jnp.where(k == KCH - 1, (cnt + G - 1) // G, cnt // G)

            def flush(b, _):
                off = b * G
                ds_ = []
                for p in range(4):
                    ds_.append(pltpu.async_copy(
                        h1t_hbm.at[pend_src.at[pl.ds(off + p * 32, 32)]],
                        rows.at[pl.ds(p * 32, 32)], sem))
                d2 = pltpu.async_copy(coef_hbm.at[pend_eid.at[pl.ds(off, G)]],
                                      cfb, sem)
                for d in ds_:
                    d.wait()
                d2.wait()
                nin = jnp.minimum(cnt - off, G)

                def acc(i, _):
                    ld = pend_ld[pl.ds(off + i, _L)][0]
                    iv = jnp.full((_L,), i, jnp.int32)
                    cf0 = plsc.load_gather(cfb, [iv, col0])
                    cf1 = plsc.load_gather(cfb, [iv, col1])
                    for t in range(8):
                        cf = cf0 if t < 4 else cf1
                        plsc.addupdate(table.at[ld, pl.ds(t * _L, _L)],
                                       cf * rows[i, pl.ds(t * _L, _L)])
                    return 0

                lax.fori_loop(0, nin, acc, 0)
                return 0

            lax.fori_loop(0, nf, flush, 0)
            rem = jnp.maximum(cnt - nf * G, 0)

            @pl.when(nf > 0)
            def _():
                fb = nf * G
                for jj in range(8):
                    sl = pl.ds(jj * _L, _L)
                    sr = pl.ds(fb + jj * _L, _L)
                    pend_ld[sl] = pend_ld[sr]
                    pend_src[sl] = pend_src[sr]
                    pend_eid[sl] = pend_eid[sr]

            return rem

        lax.fori_loop(0, KCH, chunk, jnp.int32(0))

        pltpu.sync_copy(table.at[pl.ds(0, RSZ)],
                        out_hbm.at[pl.ds(gbase + lo, RSZ)])

        @pl.when(r == 15)
        def _():
            pltpu.sync_copy(table.at[pl.ds(RSZ, 16)],
                            out_hbm.at[pl.ds(gbase + lo + RSZ, 16)])

        return 0

    lax.fori_loop(0, 2, combo, 0)


def _gat1_agg(h1t, coef16, src, dst):
    B = 1600
    G = 128
    f = pl.kernel(
        _gat1_agg_kernel,
        mesh=_sc_mesh(),
        out_type=jax.ShapeDtypeStruct((4 * _N, 128), jnp.float32),
        scratch_types=[
            pltpu.VMEM((640, 128), jnp.float32),
            pltpu.VMEM((B + 2 * G,), jnp.int32),
            pltpu.VMEM((B + 2 * G,), jnp.int32),
            pltpu.VMEM((B + 2 * G,), jnp.int32),
            pltpu.VMEM((B,), jnp.int32),
            pltpu.VMEM((B,), jnp.int32),
            pltpu.VMEM((G, 128), jnp.float32),
            pltpu.VMEM((G, 16), jnp.float32),
            pltpu.SemaphoreType.DMA,
        ],
        **_SC_PARAMS,
    )
    return f(h1t, coef16, src, dst)


def _att_den2_kernel(asrc_hbm, adst_hbm, src_hbm, dst_hbm, den_hbm, ex_hbm,
                     bs, bd, exb, ex2v, srcv, dstv, spden, sem):
    c = lax.axis_index("c")
    s = lax.axis_index("s")
    B = 1000
    ESH = _E // (_NCORE * _NSUB)
    ebase = (c * _NSUB + s) * ESH

    _zero_rows(bs, B)

    @pl.when(s < 10)
    def _():
        pltpu.sync_copy(bs.at[pl.ds(0, B)], spden.at[pl.ds(s * B, B)])

    plsc.subcore_barrier()

    def chunk(k, _):
        base = ebase + k * B
        pltpu.sync_copy(src_hbm.at[pl.ds(base, B)], srcv)
        pltpu.sync_copy(dst_hbm.at[pl.ds(base, B)], dstv)
        d1 = pltpu.async_copy(asrc_hbm.at[srcv], bs, sem)
        d1.wait()
        pltpu.async_copy(adst_hbm.at[dstv], bd, sem).wait()

        def edge(e, _):
            exb[e] = jnp.exp(_vlrelu(bs[e] + bd[e]))
            return 0

        lax.fori_loop(0, B, edge, 0)
        pltpu.sync_copy(exb, spden.at[dstv], add=True)
        for jj in range(63):
            off = jj * _L if jj < 62 else B - _L
            rowv = lax.iota(jnp.int32, _L) + off
            ex2v[pl.ds(off, _L)] = plsc.load_gather(
                exb, [rowv, jnp.zeros((_L,), jnp.int32)])
        pltpu.sync_copy(ex2v, ex_hbm.at[pl.ds(base, B)])
        return 0

    lax.fori_loop(0, ESH // B, chunk, 0)
    plsc.subcore_barrier()

    @pl.when(s < 10)
    def _():
        pltpu.sync_copy(spden.at[pl.ds(s * 1000, 1000)],
                        den_hbm.at[c, pl.ds(s * 1000, 1000)])


def _att_den2(asrc16, adst16, src, dst):
    B = 1000
    f = pl.kernel(
        _att_den2_kernel,
        mesh=_sc_mesh(),
        out_type=[
            jax.ShapeDtypeStruct((_NCORE, _N, 16), jnp.float32),
            jax.ShapeDtypeStruct((_E,), jnp.float32),
        ],
        scratch_types=[
            pltpu.VMEM((B, 16), jnp.float32),
            pltpu.VMEM((B, 16), jnp.float32),
            pltpu.VMEM((B, 16), jnp.float32),
            pltpu.VMEM((B,), jnp.float32),
            pltpu.VMEM((B,), jnp.int32),
            pltpu.VMEM((B,), jnp.int32),
            pltpu.VMEM_SHARED((_N, 16), jnp.float32),
            pltpu.SemaphoreType.DMA,
        ],
        **_SC_PARAMS,
    )
    return f(asrc16, adst16, src, dst)


def _gat2_agg_kernel(h2_hbm, ex_hbm, dent_hbm, src_hbm, dst_hbm, out_hbm,
                     table, pend_src, pend_ld, pend_cf, srcv, dstv, exv,
                     dn, rows, idxv, spacc, sem):
    c = lax.axis_index("c")
    s = lax.axis_index("s")
    r = s // 4
    j = s % 4
    B = 800
    G = 128
    RNG = 2500
    lo = r * RNG
    ESH = _E // 8
    ebase = c * (_E // 2) + j * ESH

    zv = jnp.zeros((_L,), jnp.float32)

    def zrow(i, _):
        table[i, pl.ds(0, _L)] = zv
        table[i, pl.ds(_L, _L)] = zv
        return 0

    lax.fori_loop(0, RNG, zrow, 0)
    _zero_i32(pend_src, B + 2 * G)

    @pl.when(s < 10)
    def _():
        pltpu.sync_copy(table.at[pl.ds(0, 1000)],
                        spacc.at[pl.ds(s * 1000, 1000)])

    plsc.subcore_barrier()

    KCH = ESH // B

    def chunk(k, np_):
        base = ebase + k * B
        pltpu.sync_copy(src_hbm.at[pl.ds(base, B)], srcv)
        pltpu.sync_copy(dst_hbm.at[pl.ds(base, B)], dstv)
        pltpu.sync_copy(ex_hbm.at[pl.ds(base, B)], exv)
        pltpu.async_copy(dent_hbm.at[dstv], dn, sem).wait()

        def scan(v, cnt):
            d = dstv[pl.ds(v * _L, _L)]
            m = (d >= lo) & (d < lo + RNG)
            rowv = lax.iota(jnp.int32, _L) + v * _L
            dnv = plsc.load_gather(dn, [rowv, jnp.zeros((_L,), jnp.int32)])
            cf = exv[pl.ds(v * _L, _L)] / (dnv + 1e-16)
            csum = plsc.cumsum(m.astype(jnp.int32))
            pos = cnt + csum - 1
            plsc.store_scatter(pend_ld, [pos], d - lo, mask=m)
            plsc.store_scatter(pend_src, [pos], srcv[pl.ds(v * _L, _L)],
                               mask=m)
            plsc.store_scatter(pend_cf, [pos], cf, mask=m)
            return cnt + jnp.max(csum)

        cnt = lax.fori_loop(0, B // _L, scan, np_)
        nf = jnp.where(k == KCH - 1, (cnt + G - 1) // G, cnt // G)

        def flush(b, _):
            off = b * G
            pltpu.async_copy(h2_hbm.at[pend_src.at[pl.ds(off, G)]], rows,
                             sem).wait()
            nin = jnp.minimum(cnt - off, G)

            def acc(i, _):
                ld = pend_ld[pl.ds(off + i, _L)][0]
                cf = jnp.full((_L,), pend_cf[pl.ds(off + i, _L)][0])
                plsc.addupdate(table.at[ld, pl.ds(0, _L)],
                               cf * rows[i, pl.ds(0, _L)])
                plsc.addupdate(table.at[ld, pl.ds(_L, _L)],
                               cf * rows[i, pl.ds(_L, _L)])
                return 0

            lax.fori_loop(0, nin, acc, 0)
            return 0

        lax.fori_loop(0, nf, flush, 0)
        rem = jnp.maximum(cnt - nf * G, 0)

        @pl.when(nf > 0)
        def _():
            fb = nf * G
            for jj in range(8):
                sl = pl.ds(jj * _L, _L)
                sr = pl.ds(fb + jj * _L, _L)
                pend_ld[sl] = pend_ld[sr]
                pend_src[sl] = pend_src[sr]
                pend_cf[sl] = pend_cf[sr]

        return rem

    lax.fori_loop(0, ESH // B, chunk, jnp.int32(0))

    _fill_iota(idxv, RNG, lo)
    pltpu.sync_copy(table, spacc.at[idxv], add=True)
    plsc.subcore_barrier()

    @pl.when(s < 10)
    def _():
        pltpu.sync_copy(spacc.at[pl.ds(s * 1000, 1000)],
                        out_hbm.at[c, pl.ds(s * 1000, 1000)])


def _gat2_agg(h2, ex2, dent2, src, dst):
    B = 800
    G = 128
    RNG = 2500
    f = pl.kernel(
        _gat2_agg_kernel,
        mesh=_sc_mesh(),
        out_type=jax.ShapeDtypeStruct((_NCORE, _N, 32), jnp.float32),
        scratch_types=[
            pltpu.VMEM((RNG, 32), jnp.float32),
            pltpu.VMEM((B + 2 * G,), jnp.int32),
            pltpu.VMEM((B + 2 * G,), jnp.int32),
            pltpu.VMEM((B + 2 * G,), jnp.float32),
            pltpu.VMEM((B,), jnp.int32),
            pltpu.VMEM((B,), jnp.int32),
            pltpu.VMEM((B,), jnp.float32),
            pltpu.VMEM((B, 16), jnp.float32),
            pltpu.VMEM((G, 32), jnp.float32),
            pltpu.VMEM((RNG,), jnp.int32),
            pltpu.VMEM_SHARED((_N, 32), jnp.float32),
            pltpu.SemaphoreType.DMA,
        ],
        **_SC_PARAMS,
    )
    return f(h2, ex2, dent2, src, dst)


def _gcn_gather_kernel(g_hbm, src_hbm, dst_hbm, out_hbm,
                       table, pend_src, pend_ld, srcv, dstv, rows, idxv,
                       spmem, sem):
    c = lax.axis_index("c")
    s = lax.axis_index("s")
    r = s // 4
    j = s % 4
    RNG = 2500
    ESH = _E // 8
    B = 1600
    G = 128
    KCH = ESH // B
    lo = r * RNG
    ebase = c * (_E // 2) + j * ESH

    _zero_rows(table, RNG)
    _zero_i32(pend_src, B + 2 * G)
    zrows = 1000

    @pl.when(s < 10)
    def _():
        pltpu.sync_copy(table.at[pl.ds(0, zrows)],
                        spmem.at[pl.ds(s * zrows, zrows)])

    plsc.subcore_barrier()

    def chunk_body(k, np_):
        pltpu.sync_copy(src_hbm.at[pl.ds(ebase + k * B, B)], srcv)
        pltpu.sync_copy(dst_hbm.at[pl.ds(ebase + k * B, B)], dstv)

        def scan_body(v, cnt):
            d = dstv[pl.ds(v * _L, _L)]
            m = (d >= lo) & (d < lo + RNG)
            csum = plsc.cumsum(m.astype(jnp.int32))
            pos = cnt + csum - 1
            plsc.store_scatter(pend_ld, [pos], d - lo, mask=m)
            plsc.store_scatter(pend_src, [pos], srcv[pl.ds(v * _L, _L)],
                               mask=m)
            return cnt + jnp.max(csum)

        cnt = lax.fori_loop(0, B // _L, scan_body, np_)
        nf = jnp.where(k == KCH - 1, (cnt + G - 1) // G, cnt // G)

        def flush_body(b, _):
            off = b * G
            pltpu.async_copy(g_hbm.at[pend_src.at[pl.ds(off, G)]], rows,
                             sem).wait()
            nin = jnp.minimum(cnt - off, G)

            def acc_body(i, _):
                ld = pend_ld[pl.ds(off + i, _L)][0]
                table[ld] = table[ld] + rows[i]
                return 0

            lax.fori_loop(0, nin, acc_body, 0)
            return 0

        lax.fori_loop(0, nf, flush_body, 0)
        rem = jnp.maximum(cnt - nf * G, 0)

        @pl.when(nf > 0)
        def _():
            fb = nf * G
            for jj in range(8):
                sl = pl.ds(jj * _L, _L)
                sr = pl.ds(fb + jj * _L, _L)
                pend_ld[sl] = pend_ld[sr]
                pend_src[sl] = pend_src[sr]

        return rem

    lax.fori_loop(0, ESH // B, chunk_body, jnp.int32(0))

    _fill_iota(idxv, RNG, lo)
    pltpu.sync_copy(table, spmem.at[idxv], add=True)
    plsc.subcore_barrier()

    @pl.when(s < 10)
    def _():
        pltpu.sync_copy(spmem.at[pl.ds(s * zrows, zrows)],
                        out_hbm.at[c, pl.ds(s * zrows, zrows)])


def _gcn_gather(g, src, dst):
    B = 1600
    G = 128
    RNG = 2500
    f = pl.kernel(
        _gcn_gather_kernel,
        mesh=_sc_mesh(),
        out_type=jax.ShapeDtypeStruct((_NCORE, _N, 16), jnp.float32),
        scratch_types=[
            pltpu.VMEM((RNG, 16), jnp.float32),
            pltpu.VMEM((B + 2 * G,), jnp.int32),
            pltpu.VMEM((B + 2 * G,), jnp.int32),
            pltpu.VMEM((B,), jnp.int32),
            pltpu.VMEM((B,), jnp.int32),
            pltpu.VMEM((G, 16), jnp.float32),
            pltpu.VMEM((RNG,), jnp.int32),
            pltpu.VMEM_SHARED((_N, 16), jnp.float32),
            pltpu.SemaphoreType.DMA,
        ],
        **_SC_PARAMS,
    )
    return f(g, src, dst)


def _dense1_body(x_ref, w_ref, aws_ref, awd_ref, h_ref, as_ref, ad_ref):
    g = pl.program_id(1)
    h = jnp.dot(x_ref[...], w_ref[...], preferred_element_type=jnp.float32)
    h_ref[...] = h
    das = jnp.dot(h, aws_ref[...], preferred_element_type=jnp.float32)
    dad = jnp.dot(h, awd_ref[...], preferred_element_type=jnp.float32)

    @pl.when(g == 0)
    def _():
        as_ref[...] = das
        ad_ref[...] = dad

    @pl.when(g > 0)
    def _():
        as_ref[...] += das
        ad_ref[...] += dad


def _dense1(x, W1, aws, awd):
    bn = 1000
    return pl.pallas_call(
        _dense1_body,
        grid=(_N // bn, 4),
        in_specs=[
            pl.BlockSpec((bn, 128), lambda i, g: (i, 0)),
            pl.BlockSpec((128, 128), lambda i, g: (0, g)),
            pl.BlockSpec((128, 16), lambda i, g: (g, 0)),
            pl.BlockSpec((128, 16), lambda i, g: (g, 0)),
        ],
        out_specs=[
            pl.BlockSpec((bn, 128), lambda i, g: (g * (_N // bn) + i, 0)),
            pl.BlockSpec((bn, 16), lambda i, g: (i, 0)),
            pl.BlockSpec((bn, 16), lambda i, g: (i, 0)),
        ],
        out_shape=[
            jax.ShapeDtypeStruct((4 * _N, 128), jnp.float32),
            jax.ShapeDtypeStruct((_N, 16), jnp.float32),
            jax.ShapeDtypeStruct((_N, 16), jnp.float32),
        ],
    )(x, W1, aws, awd)


def _combine1_body(dp_ref, as_ref, ad_ref, dent_ref):
    ex_self = jnp.exp(_vlrelu(as_ref[...] + ad_ref[...]))
    dent_ref[...] = dp_ref[0] + dp_ref[1] + ex_self


def _combine1(denp, asrc16, adst16):
    bn = 1000
    return pl.pallas_call(
        _combine1_body,
        grid=(_N // bn,),
        in_specs=[
            pl.BlockSpec((2, bn, 16), lambda i: (0, i, 0)),
            pl.BlockSpec((bn, 16), lambda i: (i, 0)),
            pl.BlockSpec((bn, 16), lambda i: (i, 0)),
        ],
        out_specs=pl.BlockSpec((bn, 16), lambda i: (i, 0)),
        out_shape=jax.ShapeDtypeStruct((_N, 16), jnp.float32),
    )(denp, asrc16, adst16)


def _dense2_body(o1_ref, h1_ref, as_ref, ad_ref, dent_ref, w2_ref, b1_ref,
                 aws2_ref, awd2_ref, h2_ref, as2_ref, ad2_ref):
    g = pl.program_id(1)
    selfc = jnp.exp(_vlrelu(as_ref[...] + ad_ref[...])) / (dent_ref[...] + 1e-16)
    col = lax.broadcasted_iota(jnp.int32, selfc.shape, 1)
    s0 = jnp.sum(jnp.where(col == 2 * g, selfc, 0.0), axis=1, keepdims=True)
    s1 = jnp.sum(jnp.where(col == 2 * g + 1, selfc, 0.0), axis=1,
                 keepdims=True)
    h1b = h1_ref[...]
    b1full = b1_ref[...]
    row = lax.broadcasted_iota(jnp.int32, b1full.shape, 0)
    b1g = jnp.sum(jnp.where(row == g, b1full, 0.0), axis=0, keepdims=True)
    slab = o1_ref[...] + jnp.concatenate(
        [s0 * h1b[:, :64], s1 * h1b[:, 64:]], axis=1) + b1g
    g1 = _elu(slab)
    dh2 = jnp.dot(g1, w2_ref[...], preferred_element_type=jnp.float32)

    @pl.when(g == 0)
    def _():
        h2_ref[...] = dh2

    @pl.when(g > 0)
    def _():
        h2_ref[...] += dh2

    @pl.when(g == 3)
    def _():
        h2f = h2_ref[...]
        as2_ref[...] = jnp.dot(h2f, aws2_ref[...],
                               preferred_element_type=jnp.float32)
        ad2_ref[...] = jnp.dot(h2f, awd2_ref[...],
                               preferred_element_type=jnp.float32)


def _dense2(out1t, h1t, asrc16, adst16, dent16, W2, b1, aws2, awd2):
    bn = 1000
    nb = _N // bn
    b1r = b1.reshape(4, 128)
    return pl.pallas_call(
        _dense2_body,
        grid=(nb, 4),
        in_specs=[
            pl.BlockSpec((bn, 128), lambda i, g: (g * nb + i, 0)),
            pl.BlockSpec((bn, 128), lambda i, g: (g * nb + i, 0)),
            pl.BlockSpec((bn, 16), lambda i, g: (i, 0)),
            pl.BlockSpec((bn, 16), lambda i, g: (i, 0)),
            pl.BlockSpec((bn, 16), lambda i, g: (i, 0)),
            pl.BlockSpec((128, 32), lambda i, g: (g, 0)),
            pl.BlockSpec((4, 128), lambda i, g: (0, 0)),
            pl.BlockSpec((32, 16), lambda i, g: (0, 0)),
            pl.BlockSpec((32, 16), lambda i, g: (0, 0)),
        ],
        out_specs=[
            pl.BlockSpec((bn, 32), lambda i, g: (i, 0)),
            pl.BlockSpec((bn, 16), lambda i, g: (i, 0)),
            pl.BlockSpec((bn, 16), lambda i, g: (i, 0)),
        ],
        out_shape=[
            jax.ShapeDtypeStruct((_N, 32), jnp.float32),
            jax.ShapeDtypeStruct((_N, 16), jnp.float32),
            jax.ShapeDtypeStruct((_N, 16), jnp.float32),
        ],
    )(out1t, h1t, asrc16, adst16, dent16, W2, b1r, aws2, awd2)


def _combine2_body(dp_ref, as_ref, ad_ref, dent_ref):
    ex_self = jnp.exp(_vlrelu(as_ref[...] + ad_ref[...]))
    dent_ref[...] = dp_ref[0] + dp_ref[1] + ex_self


def _combine2(denp2, as2_16, ad2_16):
    bn = 1000
    return pl.pallas_call(
        _combine2_body,
        grid=(_N // bn,),
        in_specs=[
            pl.BlockSpec((2, bn, 16), lambda i: (0, i, 0)),
            pl.BlockSpec((bn, 16), lambda i: (i, 0)),
            pl.BlockSpec((bn, 16), lambda i: (i, 0)),
        ],
        out_specs=pl.BlockSpec((bn, 16), lambda i: (i, 0)),
        out_shape=jax.ShapeDtypeStruct((_N, 16), jnp.float32),
    )(denp2, as2_16, ad2_16)


def _dense3_body(op_ref, h2_ref, as2_ref, ad2_ref, dent2_ref,
                 dent1_ref, w3_ref, b2_ref, g_ref, h3_ref, dinv_ref):
    selfc = jnp.exp(_vlrelu(as2_ref[...] + ad2_ref[...])) / (dent2_ref[...]
                                                             + 1e-16)
    out2 = (op_ref[0] + op_ref[1] + selfc[:, 0:1] * h2_ref[...]
            + b2_ref[...])
    g2 = _elu(out2)
    h3 = jnp.dot(g2, w3_ref[...], preferred_element_type=jnp.float32)
    deg = dent1_ref[:, 8:9]
    dinv = lax.rsqrt(deg)
    h3_ref[...] = h3
    g_ref[...] = dinv * h3
    dinv_ref[...] = jnp.broadcast_to(dinv, h3.shape)


def _dense3(out2p, h2, as2_16, ad2_16, dent2, dent1, W3, b2):
    bn = 1000
    b2c = b2.reshape(1, 32)
    return pl.pallas_call(
        _dense3_body,
        grid=(_N // bn,),
        in_specs=[
            pl.BlockSpec((2, bn, 32), lambda i: (0, i, 0)),
            pl.BlockSpec((bn, 32), lambda i: (i, 0)),
            pl.BlockSpec((bn, 16), lambda i: (i, 0)),
            pl.BlockSpec((bn, 16), lambda i: (i, 0)),
            pl.BlockSpec((bn, 16), lambda i: (i, 0)),
            pl.BlockSpec((bn, 16), lambda i: (i, 0)),
            pl.BlockSpec((32, 16), lambda i: (0, 0)),
            pl.BlockSpec((1, 32), lambda i: (0, 0)),
        ],
        out_specs=[
            pl.BlockSpec((bn, 16), lambda i: (i, 0)),
            pl.BlockSpec((bn, 16), lambda i: (i, 0)),
            pl.BlockSpec((bn, 16), lambda i: (i, 0)),
        ],
        out_shape=[
            jax.ShapeDtypeStruct((_N, 16), jnp.float32),
            jax.ShapeDtypeStruct((_N, 16), jnp.float32),
            jax.ShapeDtypeStruct((_N, 16), jnp.float32),
        ],
    )(out2p, h2, as2_16, ad2_16, dent2, dent1, W3, b2c)


def _dense4_body(ap_ref, h3_ref, dinv_ref, b3_ref, out_ref):
    dinv = dinv_ref[...]
    out_ref[...] = (dinv * (ap_ref[0] + ap_ref[1])
                    + dinv * dinv * h3_ref[...] + b3_ref[...])


def _dense4(accp, h3, dinv, b3):
    bn = 1000
    b3c = b3.reshape(1, 16)
    return pl.pallas_call(
        _dense4_body,
        grid=(_N // bn,),
        in_specs=[
            pl.BlockSpec((2, bn, 16), lambda i: (0, i, 0)),
            pl.BlockSpec((bn, 16), lambda i: (i, 0)),
            pl.BlockSpec((bn, 16), lambda i: (i, 0)),
            pl.BlockSpec((1, 16), lambda i: (0, 0)),
        ],
        out_specs=pl.BlockSpec((bn, 16), lambda i: (i, 0)),
        out_shape=jax.ShapeDtypeStruct((_N, 16), jnp.float32),
    )(accp, h3, dinv, b3c)


def _logit_weights16(a):
    heads, ch = a.shape
    eye = jnp.eye(16, dtype=a.dtype)[:heads]
    return (a[:, :, None] * eye[:, None, :]).reshape(heads * ch, 16)


def kernel(x, edge_index, W1, a_src1, a_dst1, b1, W2, a_src2, a_dst2, b2, W3, b3):
    src = edge_index[0]
    dst = edge_index[1]

    aws1 = _logit_weights16(a_src1)
    awd1 = _logit_weights16(a_dst1)
    h1t, asrc16, adst16 = _dense1(x, W1, aws1, awd1)
    den1p = _att_den(asrc16, adst16, src, dst)
    dent1 = _combine1(den1p, asrc16, adst16)
    coef16 = _coef(asrc16, adst16, dent1, src, dst)
    out1t = _gat1_agg(h1t, coef16, src, dst)

    aws2 = _logit_weights16(a_src2)
    awd2 = _logit_weights16(a_dst2)
    h2, as2_16, ad2_16 = _dense2(out1t, h1t, asrc16, adst16, dent1,
                                 W2, b1, aws2, awd2)
    den2p, ex2 = _att_den2(as2_16, ad2_16, src, dst)
    dent2 = _combine2(den2p, as2_16, ad2_16)
    out2p = _gat2_agg(h2, ex2, dent2, src, dst)

    g, h3, dinv = _dense3(out2p, h2, as2_16, ad2_16, dent2, dent1, W3, b2)
    accp = _gcn_gather(g, src, dst)
    out = _dense4(accp, h3, dinv, b3)
    return out


# speedup vs baseline: 12.1529x; 1.0071x over previous
import functools

import jax
import jax.numpy as jnp
from jax import lax
from jax.experimental import pallas as pl
from jax.experimental.pallas import tpu as pltpu
from jax.experimental.pallas import tpu_sc as plsc

_N = 10000
_E = 320000
_H1 = 8
_HID = 64
_OUT2 = 32
_NC = 16

_L = 16
_NCORE = 2
_NSUB = 16

_SC_PARAMS = dict(
    compiler_params=pltpu.CompilerParams(
        use_tc_tiling_on_sc=False, needs_layout_passes=False),
)


def _sc_mesh():
    return plsc.VectorSubcoreMesh(core_axis_name="c", subcore_axis_name="s")


def _zero_rows(ref, nrows):
    z = jnp.zeros((_L,), jnp.float32)

    def body(i, _):
        ref[i] = z
        return 0

    lax.fori_loop(0, nrows, body, 0)


def _zero_i32(ref, n):
    z = jnp.zeros((_L,), jnp.int32)

    def body(i, _):
        ref[pl.ds(i * _L, _L)] = z
        return 0

    lax.fori_loop(0, n // _L, body, 0)


def _fill_iota(ref, n, base):
    nfull = n // _L

    def body(i, _):
        ref[pl.ds(i * _L, _L)] = lax.iota(jnp.int32, _L) + (base + i * _L)
        return 0

    lax.fori_loop(0, nfull, body, 0)
    if n % _L:
        off = n - _L
        ref[pl.ds(off, _L)] = lax.iota(jnp.int32, _L) + (base + off)


def _elu(v):
    return jnp.where(v > 0, v, jnp.exp(v) - 1.0)


def _vlrelu(v):
    return jnp.where(v > 0, v, 0.2 * v)


def _att_den_kernel(asrc_hbm, adst_hbm, src_hbm, dst_hbm, den_hbm,
                    bs, bd, exb, srcv, dstv, spden, sem):
    c = lax.axis_index("c")
    s = lax.axis_index("s")
    B = 1000
    ESH = _E // (_NCORE * _NSUB)
    ebase = (c * _NSUB + s) * ESH

    _zero_rows(bs, B)

    @pl.when(s < 10)
    def _():
        pltpu.sync_copy(bs.at[pl.ds(0, B)], spden.at[pl.ds(s * B, B)])

    plsc.subcore_barrier()

    def chunk(k, _):
        base = ebase + k * B
        pltpu.sync_copy(src_hbm.at[pl.ds(base, B)], srcv)
        pltpu.sync_copy(dst_hbm.at[pl.ds(base, B)], dstv)
        d1 = pltpu.async_copy(asrc_hbm.at[srcv], bs, sem)
        d1.wait()
        pltpu.async_copy(adst_hbm.at[dstv], bd, sem).wait()

        def edge(e, _):
            exb[e] = jnp.exp(_vlrelu(bs[e] + bd[e]))
            return 0

        lax.fori_loop(0, B, edge, 0)
        pltpu.sync_copy(exb, spden.at[dstv], add=True)
        return 0

    lax.fori_loop(0, ESH // B, chunk, 0)
    plsc.subcore_barrier()

    @pl.when(s < 10)
    def _():
        pltpu.sync_copy(spden.at[pl.ds(s * 1000, 1000)],
                        den_hbm.at[c, pl.ds(s * 1000, 1000)])


def _att_den(asrc16, adst16, src, dst):
    B = 1000
    f = pl.kernel(
        _att_den_kernel,
        mesh=_sc_mesh(),
        out_type=jax.ShapeDtypeStruct((_NCORE, _N, 16), jnp.float32),
        scratch_types=[
            pltpu.VMEM((B, 16), jnp.float32),
            pltpu.VMEM((B, 16), jnp.float32),
            pltpu.VMEM((B, 16), jnp.float32),
            pltpu.VMEM((B,), jnp.int32),
            pltpu.VMEM((B,), jnp.int32),
            pltpu.VMEM_SHARED((_N, 16), jnp.float32),
            pltpu.SemaphoreType.DMA,
        ],
        **_SC_PARAMS,
    )
    return f(asrc16, adst16, src, dst)


def _coef_kernel(asrc_hbm, adst_hbm, dent_hbm, src_hbm, dst_hbm, coef_hbm,
                 bs, bd, dn, cfb, srcv, dstv, sem):
    c = lax.axis_index("c")
    s = lax.axis_index("s")
    B = 1000
    ESH = _E // (_NCORE * _NSUB)
    ebase = (c * _NSUB + s) * ESH

    def chunk(k, _):
        base = ebase + k * B
        pltpu.sync_copy(src_hbm.at[pl.ds(base, B)], srcv)
        pltpu.sync_copy(dst_hbm.at[pl.ds(base, B)], dstv)
        d1 = pltpu.async_copy(asrc_hbm.at[srcv], bs, sem)
        d2 = pltpu.async_copy(adst_hbm.at[dstv], bd, sem)
        d1.wait()
        d2.wait()
        pltpu.async_copy(dent_hbm.at[dstv], dn, sem).wait()

        def edge(e, _):
            ex = jnp.exp(_vlrelu(bs[e] + bd[e]))
            cfb[e] = ex / (dn[e] + 1e-16)
            return 0

        lax.fori_loop(0, B, edge, 0)
        pltpu.sync_copy(cfb, coef_hbm.at[pl.ds(base, B)])
        return 0

    lax.fori_loop(0, ESH // B, chunk, 0)


def _coef(asrc16, adst16, dent16, src, dst):
    B = 1000
    f = pl.kernel(
        _coef_kernel,
        mesh=_sc_mesh(),
        out_type=jax.ShapeDtypeStruct((_E, 16), jnp.float32),
        scratch_types=[
            pltpu.VMEM((B, 16), jnp.float32),
            pltpu.VMEM((B, 16), jnp.float32),
            pltpu.VMEM((B, 16), jnp.float32),
            pltpu.VMEM((B, 16), jnp.float32),
            pltpu.VMEM((B,), jnp.int32),
            pltpu.VMEM((B,), jnp.int32),
            pltpu.SemaphoreType.DMA,
        ],
        **_SC_PARAMS,
    )
    return f(asrc16, adst16, dent16, src, dst)


def _gat1_agg_kernel(h1t_hbm, coef_hbm, src_hbm, dst_hbm, out_hbm,
                     table, pend_src, pend_ld, pend_eid, srcv, dstv,
                     rows, cfb, sem):
    c = lax.axis_index("c")
    s = lax.axis_index("s")
    w = c * _NSUB + s
    B = 1600
    G = 128
    NP = B + 2 * G
    KCH = _E // B
    RSZ = 624

    _zero_i32(pend_src, NP)
    _zero_i32(pend_eid, NP)
    _zero_i32(pend_ld, NP)

    def combo(q, _):
        cid = w * 2 + q
        r = cid % 16
        g = cid // 16
        lo = r * RSZ
        hi = jnp.where(r == 15, _N, lo + RSZ)
        gbase = g * _N

        zv = jnp.zeros((_L,), jnp.float32)

        def zrow(i, _):
            for t in range(8):
                table[i, pl.ds(t * _L, _L)] = zv
            return 0

        lax.fori_loop(0, 640, zrow, 0)

        col0 = jnp.full((_L,), 2 * g, jnp.int32)
        col1 = col0 + 1

        def chunk(k, np_):
            base = k * B
            dc1 = pltpu.async_copy(src_hbm.at[pl.ds(base, B)], srcv, sem)
            dc2 = pltpu.async_copy(dst_hbm.at[pl.ds(base, B)], dstv, sem)
            dc1.wait()
            dc2.wait()

            def scan(v, cnt):
                d = dstv[pl.ds(v * _L, _L)]
                m = (d >= lo) & (d < hi)
                csum = plsc.cumsum(m.astype(jnp.int32))
                pos = cnt + csum - 1
                plsc.store_scatter(pend_ld, [pos], d - lo, mask=m)
                plsc.store_scatter(pend_src, [pos],
                                   srcv[pl.ds(v * _L, _L)] + gbase, mask=m)
                plsc.store_scatter(pend_eid, [pos],
                                   lax.iota(jnp.int32, _L) + (base + v * _L),
                                   mask=m)
                return cnt + plsc.all_reduce_population_count(m)[0]

            cnt = lax.fori_loop(0, B // _L, scan, np_)
            nf = jnp.where(k == KCH - 1, (cnt + G - 1) // G, cnt // G)

            def flush(b, _):
                off = b * G
                ds_ = []
                for p in range(4):
                    ds_.append(pltpu.async_copy(
                        h1t_hbm.at[pend_src.at[pl.ds(off + p * 32, 32)]],
                        rows.at[pl.ds(p * 32, 32)], sem))
                d2 = pltpu.async_copy(coef_hbm.at[pend_eid.at[pl.ds(off, G)]],
                                      cfb, sem)
                for d in ds_:
                    d.wait()
                d2.wait()
                nin = jnp.minimum(cnt - off, G)

                def one(i):
                    ld = pend_ld[pl.ds(off + i, _L)][0]
                    iv = jnp.full((_L,), i, jnp.int32)
                    cf0 = plsc.load_gather(cfb, [iv, col0])
                    cf1 = plsc.load_gather(cfb, [iv, col1])
                    for t in range(8):
                        cf = cf0 if t < 4 else cf1
                        plsc.addupdate(table.at[ld, pl.ds(t * _L, _L)],
                                       cf * rows[i, pl.ds(t * _L, _L)])

                def acc2(i2, _):
                    one(i2 * 2)
                    one(i2 * 2 + 1)
                    return 0

                lax.fori_loop(0, nin // 2, acc2, 0)

                @pl.when(nin % 2 == 1)
                def _():
                    one(nin - 1)

                return 0

            lax.fori_loop(0, nf, flush, 0)
            rem = jnp.maximum(cnt - nf * G, 0)

            @pl.when(nf > 0)
            def _():
                fb = nf * G
                for jj in range(8):
                    sl = pl.ds(jj * _L, _L)
                    sr = pl.ds(fb + jj * _L, _L)
                    pend_ld[sl] = pend_ld[sr]
                    pend_src[sl] = pend_src[sr]
                    pend_eid[sl] = pend_eid[sr]

            return rem

        lax.fori_loop(0, KCH, chunk, jnp.int32(0))

        pltpu.sync_copy(table.at[pl.ds(0, RSZ)],
                        out_hbm.at[pl.ds(gbase + lo, RSZ)])

        @pl.when(r == 15)
        def _():
            pltpu.sync_copy(table.at[pl.ds(RSZ, 16)],
                            out_hbm.at[pl.ds(gbase + lo + RSZ, 16)])

        return 0

    lax.fori_loop(0, 2, combo, 0)


def _gat1_agg(h1t, coef16, src, dst):
    B = 1600
    G = 128
    f = pl.kernel(
        _gat1_agg_kernel,
        mesh=_sc_mesh(),
        out_type=jax.ShapeDtypeStruct((4 * _N, 128), jnp.float32),
        scratch_types=[
            pltpu.VMEM((640, 128), jnp.float32),
            pltpu.VMEM((B + 2 * G,), jnp.int32),
            pltpu.VMEM((B + 2 * G,), jnp.int32),
            pltpu.VMEM((B + 2 * G,), jnp.int32),
            pltpu.VMEM((B,), jnp.int32),
            pltpu.VMEM((B,), jnp.int32),
            pltpu.VMEM((G, 128), jnp.float32),
            pltpu.VMEM((G, 16), jnp.float32),
            pltpu.SemaphoreType.DMA,
        ],
        **_SC_PARAMS,
    )
    return f(h1t, coef16, src, dst)


def _att_den2_kernel(asrc_hbm, adst_hbm, src_hbm, dst_hbm, den_hbm, ex_hbm,
                     bs, bd, exb, ex2v, srcv, dstv, spden, sem):
    c = lax.axis_index("c")
    s = lax.axis_index("s")
    B = 1000
    ESH = _E // (_NCORE * _NSUB)
    ebase = (c * _NSUB + s) * ESH

    _zero_rows(bs, B)

    @pl.when(s < 10)
    def _():
        pltpu.sync_copy(bs.at[pl.ds(0, B)], spden.at[pl.ds(s * B, B)])

    plsc.subcore_barrier()

    def chunk(k, _):
        base = ebase + k * B
        pltpu.sync_copy(src_hbm.at[pl.ds(base, B)], srcv)
        pltpu.sync_copy(dst_hbm.at[pl.ds(base, B)], dstv)
        d1 = pltpu.async_copy(asrc_hbm.at[srcv], bs, sem)
        d1.wait()
        pltpu.async_copy(adst_hbm.at[dstv], bd, sem).wait()

        def edge(e, _):
            exb[e] = jnp.exp(_vlrelu(bs[e] + bd[e]))
            return 0

        lax.fori_loop(0, B, edge, 0)
        pltpu.sync_copy(exb, spden.at[dstv], add=True)
        for jj in range(63):
            off = jj * _L if jj < 62 else B - _L
            rowv = lax.iota(jnp.int32, _L) + off
            ex2v[pl.ds(off, _L)] = plsc.load_gather(
                exb, [rowv, jnp.zeros((_L,), jnp.int32)])
        pltpu.sync_copy(ex2v, ex_hbm.at[pl.ds(base, B)])
        return 0

    lax.fori_loop(0, ESH // B, chunk, 0)
    plsc.subcore_barrier()

    @pl.when(s < 10)
    def _():
        pltpu.sync_copy(spden.at[pl.ds(s * 1000, 1000)],
                        den_hbm.at[c, pl.ds(s * 1000, 1000)])


def _att_den2(asrc16, adst16, src, dst):
    B = 1000
    f = pl.kernel(
        _att_den2_kernel,
        mesh=_sc_mesh(),
        out_type=[
            jax.ShapeDtypeStruct((_NCORE, _N, 16), jnp.float32),
            jax.ShapeDtypeStruct((_E,), jnp.float32),
        ],
        scratch_types=[
            pltpu.VMEM((B, 16), jnp.float32),
            pltpu.VMEM((B, 16), jnp.float32),
            pltpu.VMEM((B, 16), jnp.float32),
            pltpu.VMEM((B,), jnp.float32),
            pltpu.VMEM((B,), jnp.int32),
            pltpu.VMEM((B,), jnp.int32),
            pltpu.VMEM_SHARED((_N, 16), jnp.float32),
            pltpu.SemaphoreType.DMA,
        ],
        **_SC_PARAMS,
    )
    return f(asrc16, adst16, src, dst)


def _gat2_agg_kernel(h2_hbm, ex_hbm, dent_hbm, src_hbm, dst_hbm, out_hbm,
                     table, pend_src, pend_ld, pend_cf, srcv, dstv, exv,
                     dn, rows, idxv, spacc, sem):
    c = lax.axis_index("c")
    s = lax.axis_index("s")
    r = s // 4
    j = s % 4
    B = 800
    G = 128
    RNG = 2500
    lo = r * RNG
    ESH = _E // 8
    ebase = c * (_E // 2) + j * ESH

    zv = jnp.zeros((_L,), jnp.float32)

    def zrow(i, _):
        table[i, pl.ds(0, _L)] = zv
        table[i, pl.ds(_L, _L)] = zv
        return 0

    lax.fori_loop(0, RNG, zrow, 0)
    _zero_i32(pend_src, B + 2 * G)

    @pl.when(s < 10)
    def _():
        pltpu.sync_copy(table.at[pl.ds(0, 1000)],
                        spacc.at[pl.ds(s * 1000, 1000)])

    plsc.subcore_barrier()

    KCH = ESH // B

    def chunk(k, np_):
        base = ebase + k * B
        pltpu.sync_copy(src_hbm.at[pl.ds(base, B)], srcv)
        pltpu.sync_copy(dst_hbm.at[pl.ds(base, B)], dstv)
        pltpu.sync_copy(ex_hbm.at[pl.ds(base, B)], exv)
        pltpu.async_copy(dent_hbm.at[dstv], dn, sem).wait()

        def scan(v, cnt):
            d = dstv[pl.ds(v * _L, _L)]
            m = (d >= lo) & (d < lo + RNG)
            rowv = lax.iota(jnp.int32, _L) + v * _L
            dnv = plsc.load_gather(dn, [rowv, jnp.zeros((_L,), jnp.int32)])
            cf = exv[pl.ds(v * _L, _L)] / (dnv + 1e-16)
            csum = plsc.cumsum(m.astype(jnp.int32))
            pos = cnt + csum - 1
            plsc.store_scatter(pend_ld, [pos], d - lo, mask=m)
            plsc.store_scatter(pend_src, [pos], srcv[pl.ds(v * _L, _L)],
                               mask=m)
            plsc.store_scatter(pend_cf, [pos], cf, mask=m)
            return cnt + plsc.all_reduce_population_count(m)[0]

        cnt = lax.fori_loop(0, B // _L, scan, np_)
        nf = jnp.where(k == KCH - 1, (cnt + G - 1) // G, cnt // G)

        def flush(b, _):
            off = b * G
            pltpu.async_copy(h2_hbm.at[pend_src.at[pl.ds(off, G)]], rows,
                             sem).wait()
            nin = jnp.minimum(cnt - off, G)

            def acc(i, _):
                ld = pend_ld[pl.ds(off + i, _L)][0]
                cf = jnp.full((_L,), pend_cf[pl.ds(off + i, _L)][0])
                plsc.addupdate(table.at[ld, pl.ds(0, _L)],
                               cf * rows[i, pl.ds(0, _L)])
                plsc.addupdate(table.at[ld, pl.ds(_L, _L)],
                               cf * rows[i, pl.ds(_L, _L)])
                return 0

            lax.fori_loop(0, nin, acc, 0)
            return 0

        lax.fori_loop(0, nf, flush, 0)
        rem = jnp.maximum(cnt - nf * G, 0)

        @pl.when(nf > 0)
        def _():
            fb = nf * G
            for jj in range(8):
                sl = pl.ds(jj * _L, _L)
                sr = pl.ds(fb + jj * _L, _L)
                pend_ld[sl] = pend_ld[sr]
                pend_src[sl] = pend_src[sr]
                pend_cf[sl] = pend_cf[sr]

        return rem

    lax.fori_loop(0, ESH // B, chunk, jnp.int32(0))

    _fill_iota(idxv, RNG, lo)
    pltpu.sync_copy(table, spacc.at[idxv], add=True)
    plsc.subcore_barrier()

    @pl.when(s < 10)
    def _():
        pltpu.sync_copy(spacc.at[pl.ds(s * 1000, 1000)],
                        out_hbm.at[c, pl.ds(s * 1000, 1000)])


def _gat2_agg(h2, ex2, dent2, src, dst):
    B = 800
    G = 128
    RNG = 2500
    f = pl.kernel(
        _gat2_agg_kernel,
        mesh=_sc_mesh(),
        out_type=jax.ShapeDtypeStruct((_NCORE, _N, 32), jnp.float32),
        scratch_types=[
            pltpu.VMEM((RNG, 32), jnp.float32),
            pltpu.VMEM((B + 2 * G,), jnp.int32),
            pltpu.VMEM((B + 2 * G,), jnp.int32),
            pltpu.VMEM((B + 2 * G,), jnp.float32),
            pltpu.VMEM((B,), jnp.int32),
            pltpu.VMEM((B,), jnp.int32),
            pltpu.VMEM((B,), jnp.float32),
            pltpu.VMEM((B, 16), jnp.float32),
            pltpu.VMEM((G, 32), jnp.float32),
            pltpu.VMEM((RNG,), jnp.int32),
            pltpu.VMEM_SHARED((_N, 32), jnp.float32),
            pltpu.SemaphoreType.DMA,
        ],
        **_SC_PARAMS,
    )
    return f(h2, ex2, dent2, src, dst)


def _gcn_gather_kernel(g_hbm, src_hbm, dst_hbm, out_hbm,
                       table, pend_src, pend_ld, srcv, dstv, rows, idxv,
                       spmem, sem):
    c = lax.axis_index("c")
    s = lax.axis_index("s")
    r = s // 4
    j = s % 4
    RNG = 2500
    ESH = _E // 8
    B = 1600
    G = 128
    KCH = ESH // B
    lo = r * RNG
    ebase = c * (_E // 2) + j * ESH

    _zero_rows(table, RNG)
    _zero_i32(pend_src, B + 2 * G)
    zrows = 1000

    @pl.when(s < 10)
    def _():
        pltpu.sync_copy(table.at[pl.ds(0, zrows)],
                        spmem.at[pl.ds(s * zrows, zrows)])

    plsc.subcore_barrier()

    def chunk_body(k, np_):
        pltpu.sync_copy(src_hbm.at[pl.ds(ebase + k * B, B)], srcv)
        pltpu.sync_copy(dst_hbm.at[pl.ds(ebase + k * B, B)], dstv)

        def scan_body(v, cnt):
            d = dstv[pl.ds(v * _L, _L)]
            m = (d >= lo) & (d < lo + RNG)
            csum = plsc.cumsum(m.astype(jnp.int32))
            pos = cnt + csum - 1
            plsc.store_scatter(pend_ld, [pos], d - lo, mask=m)
            plsc.store_scatter(pend_src, [pos], srcv[pl.ds(v * _L, _L)],
                               mask=m)
            return cnt + plsc.all_reduce_population_count(m)[0]

        cnt = lax.fori_loop(0, B // _L, scan_body, np_)
        nf = jnp.where(k == KCH - 1, (cnt + G - 1) // G, cnt // G)

        def flush_body(b, _):
            off = b * G
            pltpu.async_copy(g_hbm.at[pend_src.at[pl.ds(off, G)]], rows,
                             sem).wait()
            nin = jnp.minimum(cnt - off, G)

            def acc_body(i, _):
                ld = pend_ld[pl.ds(off + i, _L)][0]
                table[ld] = table[ld] + rows[i]
                return 0

            lax.fori_loop(0, nin, acc_body, 0)
            return 0

        lax.fori_loop(0, nf, flush_body, 0)
        rem = jnp.maximum(cnt - nf * G, 0)

        @pl.when(nf > 0)
        def _():
            fb = nf * G
            for jj in range(8):
                sl = pl.ds(jj * _L, _L)
                sr = pl.ds(fb + jj * _L, _L)
                pend_ld[sl] = pend_ld[sr]
                pend_src[sl] = pend_src[sr]

        return rem

    lax.fori_loop(0, ESH // B, chunk_body, jnp.int32(0))

    _fill_iota(idxv, RNG, lo)
    pltpu.sync_copy(table, spmem.at[idxv], add=True)
    plsc.subcore_barrier()

    @pl.when(s < 10)
    def _():
        pltpu.sync_copy(spmem.at[pl.ds(s * zrows, zrows)],
                        out_hbm.at[c, pl.ds(s * zrows, zrows)])


def _gcn_gather(g, src, dst):
    B = 1600
    G = 128
    RNG = 2500
    f = pl.kernel(
        _gcn_gather_kernel,
        mesh=_sc_mesh(),
        out_type=jax.ShapeDtypeStruct((_NCORE, _N, 16), jnp.float32),
        scratch_types=[
            pltpu.VMEM((RNG, 16), jnp.float32),
            pltpu.VMEM((B + 2 * G,), jnp.int32),
            pltpu.VMEM((B + 2 * G,), jnp.int32),
            pltpu.VMEM((B,), jnp.int32),
            pltpu.VMEM((B,), jnp.int32),
            pltpu.VMEM((G, 16), jnp.float32),
            pltpu.VMEM((RNG,), jnp.int32),
            pltpu.VMEM_SHARED((_N, 16), jnp.float32),
            pltpu.SemaphoreType.DMA,
        ],
        **_SC_PARAMS,
    )
    return f(g, src, dst)


def _dense1_body(x_ref, w_ref, aws_ref, awd_ref, h_ref, as_ref, ad_ref):
    g = pl.program_id(1)
    h = jnp.dot(x_ref[...], w_ref[...], preferred_element_type=jnp.float32)
    h_ref[...] = h
    das = jnp.dot(h, aws_ref[...], preferred_element_type=jnp.float32)
    dad = jnp.dot(h, awd_ref[...], preferred_element_type=jnp.float32)

    @pl.when(g == 0)
    def _():
        as_ref[...] = das
        ad_ref[...] = dad

    @pl.when(g > 0)
    def _():
        as_ref[...] += das
        ad_ref[...] += dad


def _dense1(x, W1, aws, awd):
    bn = 1000
    return pl.pallas_call(
        _dense1_body,
        grid=(_N // bn, 4),
        in_specs=[
            pl.BlockSpec((bn, 128), lambda i, g: (i, 0)),
            pl.BlockSpec((128, 128), lambda i, g: (0, g)),
            pl.BlockSpec((128, 16), lambda i, g: (g, 0)),
            pl.BlockSpec((128, 16), lambda i, g: (g, 0)),
        ],
        out_specs=[
            pl.BlockSpec((bn, 128), lambda i, g: (g * (_N // bn) + i, 0)),
            pl.BlockSpec((bn, 16), lambda i, g: (i, 0)),
            pl.BlockSpec((bn, 16), lambda i, g: (i, 0)),
        ],
        out_shape=[
            jax.ShapeDtypeStruct((4 * _N, 128), jnp.float32),
            jax.ShapeDtypeStruct((_N, 16), jnp.float32),
            jax.ShapeDtypeStruct((_N, 16), jnp.float32),
        ],
    )(x, W1, aws, awd)


def _combine1_body(dp_ref, as_ref, ad_ref, dent_ref):
    ex_self = jnp.exp(_vlrelu(as_ref[...] + ad_ref[...]))
    dent_ref[...] = dp_ref[0] + dp_ref[1] + ex_self


def _combine1(denp, asrc16, adst16):
    bn = 1000
    return pl.pallas_call(
        _combine1_body,
        grid=(_N // bn,),
        in_specs=[
            pl.BlockSpec((2, bn, 16), lambda i: (0, i, 0)),
            pl.BlockSpec((bn, 16), lambda i: (i, 0)),
            pl.BlockSpec((bn, 16), lambda i: (i, 0)),
        ],
        out_specs=pl.BlockSpec((bn, 16), lambda i: (i, 0)),
        out_shape=jax.ShapeDtypeStruct((_N, 16), jnp.float32),
    )(denp, asrc16, adst16)


def _dense2_body(o1_ref, h1_ref, as_ref, ad_ref, dent_ref, w2_ref, b1_ref,
                 aws2_ref, awd2_ref, h2_ref, as2_ref, ad2_ref):
    g = pl.program_id(1)
    selfc = jnp.exp(_vlrelu(as_ref[...] + ad_ref[...])) / (dent_ref[...] + 1e-16)
    col = lax.broadcasted_iota(jnp.int32, selfc.shape, 1)
    s0 = jnp.sum(jnp.where(col == 2 * g, selfc, 0.0), axis=1, keepdims=True)
    s1 = jnp.sum(jnp.where(col == 2 * g + 1, selfc, 0.0), axis=1,
                 keepdims=True)
    h1b = h1_ref[...]
    b1full = b1_ref[...]
    row = lax.broadcasted_iota(jnp.int32, b1full.shape, 0)
    b1g = jnp.sum(jnp.where(row == g, b1full, 0.0), axis=0, keepdims=True)
    slab = o1_ref[...] + jnp.concatenate(
        [s0 * h1b[:, :64], s1 * h1b[:, 64:]], axis=1) + b1g
    g1 = _elu(slab)
    dh2 = jnp.dot(g1, w2_ref[...], preferred_element_type=jnp.float32)

    @pl.when(g == 0)
    def _():
        h2_ref[...] = dh2

    @pl.when(g > 0)
    def _():
        h2_ref[...] += dh2

    @pl.when(g == 3)
    def _():
        h2f = h2_ref[...]
        as2_ref[...] = jnp.dot(h2f, aws2_ref[...],
                               preferred_element_type=jnp.float32)
        ad2_ref[...] = jnp.dot(h2f, awd2_ref[...],
                               preferred_element_type=jnp.float32)


def _dense2(out1t, h1t, asrc16, adst16, dent16, W2, b1, aws2, awd2):
    bn = 1000
    nb = _N // bn
    b1r = b1.reshape(4, 128)
    return pl.pallas_call(
        _dense2_body,
        grid=(nb, 4),
        in_specs=[
            pl.BlockSpec((bn, 128), lambda i, g: (g * nb + i, 0)),
            pl.BlockSpec((bn, 128), lambda i, g: (g * nb + i, 0)),
            pl.BlockSpec((bn, 16), lambda i, g: (i, 0)),
            pl.BlockSpec((bn, 16), lambda i, g: (i, 0)),
            pl.BlockSpec((bn, 16), lambda i, g: (i, 0)),
            pl.BlockSpec((128, 32), lambda i, g: (g, 0)),
            pl.BlockSpec((4, 128), lambda i, g: (0, 0)),
            pl.BlockSpec((32, 16), lambda i, g: (0, 0)),
            pl.BlockSpec((32, 16), lambda i, g: (0, 0)),
        ],
        out_specs=[
            pl.BlockSpec((bn, 32), lambda i, g: (i, 0)),
            pl.BlockSpec((bn, 16), lambda i, g: (i, 0)),
            pl.BlockSpec((bn, 16), lambda i, g: (i, 0)),
        ],
        out_shape=[
            jax.ShapeDtypeStruct((_N, 32), jnp.float32),
            jax.ShapeDtypeStruct((_N, 16), jnp.float32),
            jax.ShapeDtypeStruct((_N, 16), jnp.float32),
        ],
    )(out1t, h1t, asrc16, adst16, dent16, W2, b1r, aws2, awd2)


def _combine2_body(dp_ref, as_ref, ad_ref, dent_ref):
    ex_self = jnp.exp(_vlrelu(as_ref[...] + ad_ref[...]))
    dent_ref[...] = dp_ref[0] + dp_ref[1] + ex_self


def _combine2(denp2, as2_16, ad2_16):
    bn = 1000
    return pl.pallas_call(
        _combine2_body,
        grid=(_N // bn,),
        in_specs=[
            pl.BlockSpec((2, bn, 16), lambda i: (0, i, 0)),
            pl.BlockSpec((bn, 16), lambda i: (i, 0)),
            pl.BlockSpec((bn, 16), lambda i: (i, 0)),
        ],
        out_specs=pl.BlockSpec((bn, 16), lambda i: (i, 0)),
        out_shape=jax.ShapeDtypeStruct((_N, 16), jnp.float32),
    )(denp2, as2_16, ad2_16)


def _dense3_body(op_ref, h2_ref, as2_ref, ad2_ref, dent2_ref,
                 dent1_ref, w3_ref, b2_ref, g_ref, h3_ref, dinv_ref):
    selfc = jnp.exp(_vlrelu(as2_ref[...] + ad2_ref[...])) / (dent2_ref[...]
                                                             + 1e-16)
    out2 = (op_ref[0] + op_ref[1] + selfc[:, 0:1] * h2_ref[...]
            + b2_ref[...])
    g2 = _elu(out2)
    h3 = jnp.dot(g2, w3_ref[...], preferred_element_type=jnp.float32)
    deg = dent1_ref[:, 8:9]
    dinv = lax.rsqrt(deg)
    h3_ref[...] = h3
    g_ref[...] = dinv * h3
    dinv_ref[...] = jnp.broadcast_to(dinv, h3.shape)


def _dense3(out2p, h2, as2_16, ad2_16, dent2, dent1, W3, b2):
    bn = 1000
    b2c = b2.reshape(1, 32)
    return pl.pallas_call(
        _dense3_body,
        grid=(_N // bn,),
        in_specs=[
            pl.BlockSpec((2, bn, 32), lambda i: (0, i, 0)),
            pl.BlockSpec((bn, 32), lambda i: (i, 0)),
            pl.BlockSpec((bn, 16), lambda i: (i, 0)),
            pl.BlockSpec((bn, 16), lambda i: (i, 0)),
            pl.BlockSpec((bn, 16), lambda i: (i, 0)),
            pl.BlockSpec((bn, 16), lambda i: (i, 0)),
            pl.BlockSpec((32, 16), lambda i: (0, 0)),
            pl.BlockSpec((1, 32), lambda i: (0, 0)),
        ],
        out_specs=[
            pl.BlockSpec((bn, 16), lambda i: (i, 0)),
            pl.BlockSpec((bn, 16), lambda i: (i, 0)),
            pl.BlockSpec((bn, 16), lambda i: (i, 0)),
        ],
        out_shape=[
            jax.ShapeDtypeStruct((_N, 16), jnp.float32),
            jax.ShapeDtypeStruct((_N, 16), jnp.float32),
            jax.ShapeDtypeStruct((_N, 16), jnp.float32),
        ],
    )(out2p, h2, as2_16, ad2_16, dent2, dent1, W3, b2c)


def _dense4_body(ap_ref, h3_ref, dinv_ref, b3_ref, out_ref):
    dinv = dinv_ref[...]
    out_ref[...] = (dinv * (ap_ref[0] + ap_ref[1])
                    + dinv * dinv * h3_ref[...] + b3_ref[...])


def _dense4(accp, h3, dinv, b3):
    bn = 1000
    b3c = b3.reshape(1, 16)
    return pl.pallas_call(
        _dense4_body,
        grid=(_N // bn,),
        in_specs=[
            pl.BlockSpec((2, bn, 16), lambda i: (0, i, 0)),
            pl.BlockSpec((bn, 16), lambda i: (i, 0)),
            pl.BlockSpec((bn, 16), lambda i: (i, 0)),
            pl.BlockSpec((1, 16), lambda i: (0, 0)),
        ],
        out_specs=pl.BlockSpec((bn, 16), lambda i: (i, 0)),
        out_shape=jax.ShapeDtypeStruct((_N, 16), jnp.float32),
    )(accp, h3, dinv, b3c)


def _logit_weights16(a):
    heads, ch = a.shape
    eye = jnp.eye(16, dtype=a.dtype)[:heads]
    return (a[:, :, None] * eye[:, None, :]).reshape(heads * ch, 16)


def kernel(x, edge_index, W1, a_src1, a_dst1, b1, W2, a_src2, a_dst2, b2, W3, b3):
    src = edge_index[0]
    dst = edge_index[1]

    aws1 = _logit_weights16(a_src1)
    awd1 = _logit_weights16(a_dst1)
    h1t, asrc16, adst16 = _dense1(x, W1, aws1, awd1)
    den1p = _att_den(asrc16, adst16, src, dst)
    dent1 = _combine1(den1p, asrc16, adst16)
    coef16 = _coef(asrc16, adst16, dent1, src, dst)
    out1t = _gat1_agg(h1t, coef16, src, dst)

    aws2 = _logit_weights16(a_src2)
    awd2 = _logit_weights16(a_dst2)
    h2, as2_16, ad2_16 = _dense2(out1t, h1t, asrc16, adst16, dent1,
                                 W2, b1, aws2, awd2)
    den2p, ex2 = _att_den2(as2_16, ad2_16, src, dst)
    dent2 = _combine2(den2p, as2_16, ad2_16)
    out2p = _gat2_agg(h2, ex2, dent2, src, dst)

    g, h3, dinv = _dense3(out2p, h2, as2_16, ad2_16, dent2, dent1, W3, b2)
    accp = _gcn_gather(g, src, dst)
    out = _dense4(accp, h3, dinv, b3)
    return out


# speedup vs baseline: 13.0889x; 1.0770x over previous
import functools

import jax
import jax.numpy as jnp
from jax import lax
from jax.experimental import pallas as pl
from jax.experimental.pallas import tpu as pltpu
from jax.experimental.pallas import tpu_sc as plsc

_N = 10000
_E = 320000
_H1 = 8
_HID = 64
_OUT2 = 32
_NC = 16

_L = 16
_NCORE = 2
_NSUB = 16

_SC_PARAMS = dict(
    compiler_params=pltpu.CompilerParams(
        use_tc_tiling_on_sc=False, needs_layout_passes=False),
)


def _sc_mesh():
    return plsc.VectorSubcoreMesh(core_axis_name="c", subcore_axis_name="s")


def _zero_rows(ref, nrows):
    z = jnp.zeros((_L,), jnp.float32)

    def body(i, _):
        ref[i] = z
        return 0

    lax.fori_loop(0, nrows, body, 0)


def _zero_i32(ref, n):
    z = jnp.zeros((_L,), jnp.int32)

    def body(i, _):
        ref[pl.ds(i * _L, _L)] = z
        return 0

    lax.fori_loop(0, n // _L, body, 0)


def _fill_iota(ref, n, base):
    nfull = n // _L

    def body(i, _):
        ref[pl.ds(i * _L, _L)] = lax.iota(jnp.int32, _L) + (base + i * _L)
        return 0

    lax.fori_loop(0, nfull, body, 0)
    if n % _L:
        off = n - _L
        ref[pl.ds(off, _L)] = lax.iota(jnp.int32, _L) + (base + off)


def _elu(v):
    return jnp.where(v > 0, v, jnp.exp(v) - 1.0)


def _vlrelu(v):
    return jnp.where(v > 0, v, 0.2 * v)


def _att_den_kernel(asrc_hbm, adst_hbm, src_hbm, dst_hbm, den_hbm,
                    bs, bd, exb, srcv, dstv, spden, sem):
    c = lax.axis_index("c")
    s = lax.axis_index("s")
    B = 1000
    ESH = _E // (_NCORE * _NSUB)
    ebase = (c * _NSUB + s) * ESH

    _zero_rows(bs, B)

    @pl.when(s < 10)
    def _():
        pltpu.sync_copy(bs.at[pl.ds(0, B)], spden.at[pl.ds(s * B, B)])

    plsc.subcore_barrier()

    def chunk(k, _):
        base = ebase + k * B
        pltpu.sync_copy(src_hbm.at[pl.ds(base, B)], srcv)
        pltpu.sync_copy(dst_hbm.at[pl.ds(base, B)], dstv)
        d1 = pltpu.async_copy(asrc_hbm.at[srcv], bs, sem)
        d1.wait()
        pltpu.async_copy(adst_hbm.at[dstv], bd, sem).wait()

        def edge(e, _):
            exb[e] = jnp.exp(_vlrelu(bs[e] + bd[e]))
            return 0

        lax.fori_loop(0, B, edge, 0)
        pltpu.sync_copy(exb, spden.at[dstv], add=True)
        return 0

    lax.fori_loop(0, ESH // B, chunk, 0)
    plsc.subcore_barrier()

    @pl.when(s < 10)
    def _():
        pltpu.sync_copy(spden.at[pl.ds(s * 1000, 1000)],
                        den_hbm.at[c, pl.ds(s * 1000, 1000)])


def _att_den(asrc16, adst16, src, dst):
    B = 1000
    f = pl.kernel(
        _att_den_kernel,
        mesh=_sc_mesh(),
        out_type=jax.ShapeDtypeStruct((_NCORE, _N, 16), jnp.float32),
        scratch_types=[
            pltpu.VMEM((B, 16), jnp.float32),
            pltpu.VMEM((B, 16), jnp.float32),
            pltpu.VMEM((B, 16), jnp.float32),
            pltpu.VMEM((B,), jnp.int32),
            pltpu.VMEM((B,), jnp.int32),
            pltpu.VMEM_SHARED((_N, 16), jnp.float32),
            pltpu.SemaphoreType.DMA,
        ],
        **_SC_PARAMS,
    )
    return f(asrc16, adst16, src, dst)


def _coef_kernel(asrc_hbm, adst_hbm, dent_hbm, src_hbm, dst_hbm, coef_hbm,
                 bs, bd, dn, cfb, srcv, dstv, sem):
    c = lax.axis_index("c")
    s = lax.axis_index("s")
    B = 1000
    ESH = _E // (_NCORE * _NSUB)
    ebase = (c * _NSUB + s) * ESH

    def chunk(k, _):
        base = ebase + k * B
        pltpu.sync_copy(src_hbm.at[pl.ds(base, B)], srcv)
        pltpu.sync_copy(dst_hbm.at[pl.ds(base, B)], dstv)
        d1 = pltpu.async_copy(asrc_hbm.at[srcv], bs, sem)
        d2 = pltpu.async_copy(adst_hbm.at[dstv], bd, sem)
        d1.wait()
        d2.wait()
        pltpu.async_copy(dent_hbm.at[dstv], dn, sem).wait()

        def edge(e, _):
            ex = jnp.exp(_vlrelu(bs[e] + bd[e]))
            cfb[e] = ex / (dn[e] + 1e-16)
            return 0

        lax.fori_loop(0, B, edge, 0)
        pltpu.sync_copy(cfb, coef_hbm.at[pl.ds(base, B)])
        return 0

    lax.fori_loop(0, ESH // B, chunk, 0)


def _coef(asrc16, adst16, dent16, src, dst):
    B = 1000
    f = pl.kernel(
        _coef_kernel,
        mesh=_sc_mesh(),
        out_type=jax.ShapeDtypeStruct((_E, 16), jnp.float32),
        scratch_types=[
            pltpu.VMEM((B, 16), jnp.float32),
            pltpu.VMEM((B, 16), jnp.float32),
            pltpu.VMEM((B, 16), jnp.float32),
            pltpu.VMEM((B, 16), jnp.float32),
            pltpu.VMEM((B,), jnp.int32),
            pltpu.VMEM((B,), jnp.int32),
            pltpu.SemaphoreType.DMA,
        ],
        **_SC_PARAMS,
    )
    return f(asrc16, adst16, dent16, src, dst)


def _gat1_agg_kernel(h1t_hbm, coef_hbm, src_hbm, dst_hbm, out_hbm,
                     table, pend_src, pend_ld, pend_eid, srcv, dstv,
                     srcv2, dstv2, rows, cfb, sem):
    c = lax.axis_index("c")
    s = lax.axis_index("s")
    w = c * _NSUB + s
    B = 1600
    G = 128
    NP = B + 2 * G
    KCH = _E // B
    RSZ = 624

    _zero_i32(pend_src, NP)
    _zero_i32(pend_eid, NP)
    _zero_i32(pend_ld, NP)

    def combo(q, _):
        cid = w * 2 + q
        r = cid % 16
        g = cid // 16
        lo = r * RSZ
        hi = jnp.where(r == 15, _N, lo + RSZ)
        gbase = g * _N

        zv = jnp.zeros((_L,), jnp.float32)

        def zrow(i, _):
            for t in range(8):
                table[i, pl.ds(t * _L, _L)] = zv
            return 0

        lax.fori_loop(0, 640, zrow, 0)

        col0 = jnp.full((_L,), 2 * g, jnp.int32)
        col1 = col0 + 1

        def process(srcb, dstb, base, np_):
            def scan(v, cnt):
                d = dstb[pl.ds(v * _L, _L)]
                m = (d >= lo) & (d < hi)
                csum = plsc.cumsum(m.astype(jnp.int32))
                pos = cnt + csum - 1
                plsc.store_scatter(pend_ld, [pos], d - lo, mask=m)
                plsc.store_scatter(pend_src, [pos],
                                   srcb[pl.ds(v * _L, _L)] + gbase, mask=m)
                plsc.store_scatter(pend_eid, [pos],
                                   lax.iota(jnp.int32, _L) + base + v * _L,
                                   mask=m)
                return cnt + plsc.all_reduce_population_count(m)[0]

            cnt = lax.fori_loop(0, B // _L, scan, np_)
            nf = jnp.where(base == (KCH - 1) * B, (cnt + G - 1) // G,
                           cnt // G)

            def flush(b, _):
                off = b * G
                ds_ = []
                for p in range(4):
                    ds_.append(pltpu.async_copy(
                        h1t_hbm.at[pend_src.at[pl.ds(off + p * 32, 32)]],
                        rows.at[pl.ds(p * 32, 32)], sem))
                d2 = pltpu.async_copy(coef_hbm.at[pend_eid.at[pl.ds(off, G)]],
                                      cfb, sem)
                for d in ds_:
                    d.wait()
                d2.wait()
                nin = jnp.minimum(cnt - off, G)

                def one(i):
                    ld = pend_ld[pl.ds(off + i, _L)][0]
                    iv = jnp.full((_L,), i, jnp.int32)
                    cf0 = plsc.load_gather(cfb, [iv, col0])
                    cf1 = plsc.load_gather(cfb, [iv, col1])
                    for t in range(8):
                        cf = cf0 if t < 4 else cf1
                        plsc.addupdate(table.at[ld, pl.ds(t * _L, _L)],
                                       cf * rows[i, pl.ds(t * _L, _L)])

                def acc2(i2, _):
                    one(i2 * 2)
                    one(i2 * 2 + 1)
                    return 0

                lax.fori_loop(0, nin // 2, acc2, 0)

                @pl.when(nin % 2 == 1)
                def _():
                    one(nin - 1)

                return 0

            lax.fori_loop(0, nf, flush, 0)
            rem = jnp.maximum(cnt - nf * G, 0)

            @pl.when(nf > 0)
            def _():
                fb = nf * G
                for jj in range(8):
                    sl = pl.ds(jj * _L, _L)
                    sr = pl.ds(fb + jj * _L, _L)
                    pend_ld[sl] = pend_ld[sr]
                    pend_src[sl] = pend_src[sr]
                    pend_eid[sl] = pend_eid[sr]

            return rem

        pltpu.async_copy(src_hbm.at[pl.ds(0, B)], srcv, sem)
        pltpu.async_copy(dst_hbm.at[pl.ds(0, B)], dstv, sem)

        def pair(i, np_):
            baseA = 2 * i * B
            baseB = baseA + B
            baseN = jnp.minimum(baseA + 2 * B, (KCH - 1) * B)
            pltpu.make_async_copy(src_hbm.at[pl.ds(baseA, B)], srcv,
                                  sem).wait()
            pltpu.make_async_copy(dst_hbm.at[pl.ds(baseA, B)], dstv,
                                  sem).wait()
            pltpu.async_copy(src_hbm.at[pl.ds(baseB, B)], srcv2, sem)
            pltpu.async_copy(dst_hbm.at[pl.ds(baseB, B)], dstv2, sem)
            np1 = process(srcv, dstv, baseA, np_)
            pltpu.make_async_copy(src_hbm.at[pl.ds(baseB, B)], srcv2,
                                  sem).wait()
            pltpu.make_async_copy(dst_hbm.at[pl.ds(baseB, B)], dstv2,
                                  sem).wait()
            pltpu.async_copy(src_hbm.at[pl.ds(baseN, B)], srcv, sem)
            pltpu.async_copy(dst_hbm.at[pl.ds(baseN, B)], dstv, sem)
            return process(srcv2, dstv2, baseB, np1)

        lax.fori_loop(0, KCH // 2, pair, jnp.int32(0))
        pltpu.make_async_copy(src_hbm.at[pl.ds(0, B)], srcv, sem).wait()
        pltpu.make_async_copy(dst_hbm.at[pl.ds(0, B)], dstv, sem).wait()

        pltpu.sync_copy(table.at[pl.ds(0, RSZ)],
                        out_hbm.at[pl.ds(gbase + lo, RSZ)])

        @pl.when(r == 15)
        def _():
            pltpu.sync_copy(table.at[pl.ds(RSZ, 16)],
                            out_hbm.at[pl.ds(gbase + lo + RSZ, 16)])

        return 0

    lax.fori_loop(0, 2, combo, 0)


def _gat1_agg(h1t, coef16, src, dst):
    B = 1600
    G = 128
    f = pl.kernel(
        _gat1_agg_kernel,
        mesh=_sc_mesh(),
        out_type=jax.ShapeDtypeStruct((4 * _N, 128), jnp.float32),
        scratch_types=[
            pltpu.VMEM((640, 128), jnp.float32),
            pltpu.VMEM((B + 2 * G,), jnp.int32),
            pltpu.VMEM((B + 2 * G,), jnp.int32),
            pltpu.VMEM((B + 2 * G,), jnp.int32),
            pltpu.VMEM((B,), jnp.int32),
            pltpu.VMEM((B,), jnp.int32),
            pltpu.VMEM((B,), jnp.int32),
            pltpu.VMEM((B,), jnp.int32),
            pltpu.VMEM((G, 128), jnp.float32),
            pltpu.VMEM((G, 16), jnp.float32),
            pltpu.SemaphoreType.DMA,
        ],
        **_SC_PARAMS,
    )
    return f(h1t, coef16, src, dst)


def _att_den2_kernel(asrc_hbm, adst_hbm, src_hbm, dst_hbm, den_hbm, ex_hbm,
                     bs, bd, exb, ex2v, srcv, dstv, spden, sem):
    c = lax.axis_index("c")
    s = lax.axis_index("s")
    B = 1000
    ESH = _E // (_NCORE * _NSUB)
    ebase = (c * _NSUB + s) * ESH

    _zero_rows(bs, B)

    @pl.when(s < 10)
    def _():
        pltpu.sync_copy(bs.at[pl.ds(0, B)], spden.at[pl.ds(s * B, B)])

    plsc.subcore_barrier()

    def chunk(k, _):
        base = ebase + k * B
        pltpu.sync_copy(src_hbm.at[pl.ds(base, B)], srcv)
        pltpu.sync_copy(dst_hbm.at[pl.ds(base, B)], dstv)
        d1 = pltpu.async_copy(asrc_hbm.at[srcv], bs, sem)
        d1.wait()
        pltpu.async_copy(adst_hbm.at[dstv], bd, sem).wait()

        def edge(e, _):
            exb[e] = jnp.exp(_vlrelu(bs[e] + bd[e]))
            return 0

        lax.fori_loop(0, B, edge, 0)
        pltpu.sync_copy(exb, spden.at[dstv], add=True)
        for jj in range(63):
            off = jj * _L if jj < 62 else B - _L
            rowv = lax.iota(jnp.int32, _L) + off
            ex2v[pl.ds(off, _L)] = plsc.load_gather(
                exb, [rowv, jnp.zeros((_L,), jnp.int32)])
        pltpu.sync_copy(ex2v, ex_hbm.at[pl.ds(base, B)])
        return 0

    lax.fori_loop(0, ESH // B, chunk, 0)
    plsc.subcore_barrier()

    @pl.when(s < 10)
    def _():
        pltpu.sync_copy(spden.at[pl.ds(s * 1000, 1000)],
                        den_hbm.at[c, pl.ds(s * 1000, 1000)])


def _att_den2(asrc16, adst16, src, dst):
    B = 1000
    f = pl.kernel(
        _att_den2_kernel,
        mesh=_sc_mesh(),
        out_type=[
            jax.ShapeDtypeStruct((_NCORE, _N, 16), jnp.float32),
            jax.ShapeDtypeStruct((_E,), jnp.float32),
        ],
        scratch_types=[
            pltpu.VMEM((B, 16), jnp.float32),
            pltpu.VMEM((B, 16), jnp.float32),
            pltpu.VMEM((B, 16), jnp.float32),
            pltpu.VMEM((B,), jnp.float32),
            pltpu.VMEM((B,), jnp.int32),
            pltpu.VMEM((B,), jnp.int32),
            pltpu.VMEM_SHARED((_N, 16), jnp.float32),
            pltpu.SemaphoreType.DMA,
        ],
        **_SC_PARAMS,
    )
    return f(asrc16, adst16, src, dst)


def _gat2_agg_kernel(h2_hbm, ex_hbm, dent_hbm, src_hbm, dst_hbm, out_hbm,
                     table, pend_src, pend_ld, pend_cf, srcv, dstv, exv,
                     dn, rows, idxv, spacc, sem):
    c = lax.axis_index("c")
    s = lax.axis_index("s")
    r = s // 4
    j = s % 4
    B = 800
    G = 128
    RNG = 2500
    lo = r * RNG
    ESH = _E // 8
    ebase = c * (_E // 2) + j * ESH

    zv = jnp.zeros((_L,), jnp.float32)

    def zrow(i, _):
        table[i, pl.ds(0, _L)] = zv
        table[i, pl.ds(_L, _L)] = zv
        return 0

    lax.fori_loop(0, RNG, zrow, 0)
    _zero_i32(pend_src, B + 2 * G)

    @pl.when(s < 10)
    def _():
        pltpu.sync_copy(table.at[pl.ds(0, 1000)],
                        spacc.at[pl.ds(s * 1000, 1000)])

    plsc.subcore_barrier()

    KCH = ESH // B

    def chunk(k, np_):
        base = ebase + k * B
        pltpu.sync_copy(src_hbm.at[pl.ds(base, B)], srcv)
        pltpu.sync_copy(dst_hbm.at[pl.ds(base, B)], dstv)
        pltpu.sync_copy(ex_hbm.at[pl.ds(base, B)], exv)
        pltpu.async_copy(dent_hbm.at[dstv], dn, sem).wait()

        def scan(v, cnt):
            d = dstv[pl.ds(v * _L, _L)]
            m = (d >= lo) & (d < lo + RNG)
            rowv = lax.iota(jnp.int32, _L) + v * _L
            dnv = plsc.load_gather(dn, [rowv, jnp.zeros((_L,), jnp.int32)])
            cf = exv[pl.ds(v * _L, _L)] / (dnv + 1e-16)
            csum = plsc.cumsum(m.astype(jnp.int32))
            pos = cnt + csum - 1
            plsc.store_scatter(pend_ld, [pos], d - lo, mask=m)
            plsc.store_scatter(pend_src, [pos], srcv[pl.ds(v * _L, _L)],
                               mask=m)
            plsc.store_scatter(pend_cf, [pos], cf, mask=m)
            return cnt + plsc.all_reduce_population_count(m)[0]

        cnt = lax.fori_loop(0, B // _L, scan, np_)
        nf = jnp.where(k == KCH - 1, (cnt + G - 1) // G, cnt // G)

        def flush(b, _):
            off = b * G
            pltpu.async_copy(h2_hbm.at[pend_src.at[pl.ds(off, G)]], rows,
                             sem).wait()
            nin = jnp.minimum(cnt - off, G)

            def acc(i, _):
                ld = pend_ld[pl.ds(off + i, _L)][0]
                cf = jnp.full((_L,), pend_cf[pl.ds(off + i, _L)][0])
                plsc.addupdate(table.at[ld, pl.ds(0, _L)],
                               cf * rows[i, pl.ds(0, _L)])
                plsc.addupdate(table.at[ld, pl.ds(_L, _L)],
                               cf * rows[i, pl.ds(_L, _L)])
                return 0

            lax.fori_loop(0, nin, acc, 0)
            return 0

        lax.fori_loop(0, nf, flush, 0)
        rem = jnp.maximum(cnt - nf * G, 0)

        @pl.when(nf > 0)
        def _():
            fb = nf * G
            for jj in range(8):
                sl = pl.ds(jj * _L, _L)
                sr = pl.ds(fb + jj * _L, _L)
                pend_ld[sl] = pend_ld[sr]
                pend_src[sl] = pend_src[sr]
                pend_cf[sl] = pend_cf[sr]

        return rem

    lax.fori_loop(0, ESH // B, chunk, jnp.int32(0))

    _fill_iota(idxv, RNG, lo)
    pltpu.sync_copy(table, spacc.at[idxv], add=True)
    plsc.subcore_barrier()

    @pl.when(s < 10)
    def _():
        pltpu.sync_copy(spacc.at[pl.ds(s * 1000, 1000)],
                        out_hbm.at[c, pl.ds(s * 1000, 1000)])


def _gat2_agg(h2, ex2, dent2, src, dst):
    B = 800
    G = 128
    RNG = 2500
    f = pl.kernel(
        _gat2_agg_kernel,
        mesh=_sc_mesh(),
        out_type=jax.ShapeDtypeStruct((_NCORE, _N, 32), jnp.float32),
        scratch_types=[
            pltpu.VMEM((RNG, 32), jnp.float32),
            pltpu.VMEM((B + 2 * G,), jnp.int32),
            pltpu.VMEM((B + 2 * G,), jnp.int32),
            pltpu.VMEM((B + 2 * G,), jnp.float32),
            pltpu.VMEM((B,), jnp.int32),
            pltpu.VMEM((B,), jnp.int32),
            pltpu.VMEM((B,), jnp.float32),
            pltpu.VMEM((B, 16), jnp.float32),
            pltpu.VMEM((G, 32), jnp.float32),
            pltpu.VMEM((RNG,), jnp.int32),
            pltpu.VMEM_SHARED((_N, 32), jnp.float32),
            pltpu.SemaphoreType.DMA,
        ],
        **_SC_PARAMS,
    )
    return f(h2, ex2, dent2, src, dst)


def _gcn_gather_kernel(g_hbm, src_hbm, dst_hbm, out_hbm,
                       table, pend_src, pend_ld, srcv, dstv, rows, idxv,
                       spmem, sem):
    c = lax.axis_index("c")
    s = lax.axis_index("s")
    r = s // 4
    j = s % 4
    RNG = 2500
    ESH = _E // 8
    B = 1600
    G = 128
    KCH = ESH // B
    lo = r * RNG
    ebase = c * (_E // 2) + j * ESH

    _zero_rows(table, RNG)
    _zero_i32(pend_src, B + 2 * G)
    zrows = 1000

    @pl.when(s < 10)
    def _():
        pltpu.sync_copy(table.at[pl.ds(0, zrows)],
                        spmem.at[pl.ds(s * zrows, zrows)])

    plsc.subcore_barrier()

    def chunk_body(k, np_):
        pltpu.sync_copy(src_hbm.at[pl.ds(ebase + k * B, B)], srcv)
        pltpu.sync_copy(dst_hbm.at[pl.ds(ebase + k * B, B)], dstv)

        def scan_body(v, cnt):
            d = dstv[pl.ds(v * _L, _L)]
            m = (d >= lo) & (d < lo + RNG)
            csum = plsc.cumsum(m.astype(jnp.int32))
            pos = cnt + csum - 1
            plsc.store_scatter(pend_ld, [pos], d - lo, mask=m)
            plsc.store_scatter(pend_src, [pos], srcv[pl.ds(v * _L, _L)],
                               mask=m)
            return cnt + plsc.all_reduce_population_count(m)[0]

        cnt = lax.fori_loop(0, B // _L, scan_body, np_)
        nf = jnp.where(k == KCH - 1, (cnt + G - 1) // G, cnt // G)

        def flush_body(b, _):
            off = b * G
            pltpu.async_copy(g_hbm.at[pend_src.at[pl.ds(off, G)]], rows,
                             sem).wait()
            nin = jnp.minimum(cnt - off, G)

            def acc_body(i, _):
                ld = pend_ld[pl.ds(off + i, _L)][0]
                table[ld] = table[ld] + rows[i]
                return 0

            lax.fori_loop(0, nin, acc_body, 0)
            return 0

        lax.fori_loop(0, nf, flush_body, 0)
        rem = jnp.maximum(cnt - nf * G, 0)

        @pl.when(nf > 0)
        def _():
            fb = nf * G
            for jj in range(8):
                sl = pl.ds(jj * _L, _L)
                sr = pl.ds(fb + jj * _L, _L)
                pend_ld[sl] = pend_ld[sr]
                pend_src[sl] = pend_src[sr]

        return rem

    lax.fori_loop(0, ESH // B, chunk_body, jnp.int32(0))

    _fill_iota(idxv, RNG, lo)
    pltpu.sync_copy(table, spmem.at[idxv], add=True)
    plsc.subcore_barrier()

    @pl.when(s < 10)
    def _():
        pltpu.sync_copy(spmem.at[pl.ds(s * zrows, zrows)],
                        out_hbm.at[c, pl.ds(s * zrows, zrows)])


def _gcn_gather(g, src, dst):
    B = 1600
    G = 128
    RNG = 2500
    f = pl.kernel(
        _gcn_gather_kernel,
        mesh=_sc_mesh(),
        out_type=jax.ShapeDtypeStruct((_NCORE, _N, 16), jnp.float32),
        scratch_types=[
            pltpu.VMEM((RNG, 16), jnp.float32),
            pltpu.VMEM((B + 2 * G,), jnp.int32),
            pltpu.VMEM((B + 2 * G,), jnp.int32),
            pltpu.VMEM((B,), jnp.int32),
            pltpu.VMEM((B,), jnp.int32),
            pltpu.VMEM((G, 16), jnp.float32),
            pltpu.VMEM((RNG,), jnp.int32),
            pltpu.VMEM_SHARED((_N, 16), jnp.float32),
            pltpu.SemaphoreType.DMA,
        ],
        **_SC_PARAMS,
    )
    return f(g, src, dst)


def _dense1_body(x_ref, w_ref, aws_ref, awd_ref, h_ref, as_ref, ad_ref):
    g = pl.program_id(1)
    h = jnp.dot(x_ref[...], w_ref[...], preferred_element_type=jnp.float32)
    h_ref[...] = h
    das = jnp.dot(h, aws_ref[...], preferred_element_type=jnp.float32)
    dad = jnp.dot(h, awd_ref[...], preferred_element_type=jnp.float32)

    @pl.when(g == 0)
    def _():
        as_ref[...] = das
        ad_ref[...] = dad

    @pl.when(g > 0)
    def _():
        as_ref[...] += das
        ad_ref[...] += dad


def _dense1(x, W1, aws, awd):
    bn = 1000
    return pl.pallas_call(
        _dense1_body,
        grid=(_N // bn, 4),
        in_specs=[
            pl.BlockSpec((bn, 128), lambda i, g: (i, 0)),
            pl.BlockSpec((128, 128), lambda i, g: (0, g)),
            pl.BlockSpec((128, 16), lambda i, g: (g, 0)),
            pl.BlockSpec((128, 16), lambda i, g: (g, 0)),
        ],
        out_specs=[
            pl.BlockSpec((bn, 128), lambda i, g: (g * (_N // bn) + i, 0)),
            pl.BlockSpec((bn, 16), lambda i, g: (i, 0)),
            pl.BlockSpec((bn, 16), lambda i, g: (i, 0)),
        ],
        out_shape=[
            jax.ShapeDtypeStruct((4 * _N, 128), jnp.float32),
            jax.ShapeDtypeStruct((_N, 16), jnp.float32),
            jax.ShapeDtypeStruct((_N, 16), jnp.float32),
        ],
    )(x, W1, aws, awd)


def _combine1_body(dp_ref, as_ref, ad_ref, dent_ref):
    ex_self = jnp.exp(_vlrelu(as_ref[...] + ad_ref[...]))
    dent_ref[...] = dp_ref[0] + dp_ref[1] + ex_self


def _combine1(denp, asrc16, adst16):
    bn = 1000
    return pl.pallas_call(
        _combine1_body,
        grid=(_N // bn,),
        in_specs=[
            pl.BlockSpec((2, bn, 16), lambda i: (0, i, 0)),
            pl.BlockSpec((bn, 16), lambda i: (i, 0)),
            pl.BlockSpec((bn, 16), lambda i: (i, 0)),
        ],
        out_specs=pl.BlockSpec((bn, 16), lambda i: (i, 0)),
        out_shape=jax.ShapeDtypeStruct((_N, 16), jnp.float32),
    )(denp, asrc16, adst16)


def _dense2_body(o1_ref, h1_ref, as_ref, ad_ref, dent_ref, w2_ref, b1_ref,
                 aws2_ref, awd2_ref, h2_ref, as2_ref, ad2_ref):
    g = pl.program_id(1)
    selfc = jnp.exp(_vlrelu(as_ref[...] + ad_ref[...])) / (dent_ref[...] + 1e-16)
    col = lax.broadcasted_iota(jnp.int32, selfc.shape, 1)
    s0 = jnp.sum(jnp.where(col == 2 * g, selfc, 0.0), axis=1, keepdims=True)
    s1 = jnp.sum(jnp.where(col == 2 * g + 1, selfc, 0.0), axis=1,
                 keepdims=True)
    h1b = h1_ref[...]
    b1full = b1_ref[...]
    row = lax.broadcasted_iota(jnp.int32, b1full.shape, 0)
    b1g = jnp.sum(jnp.where(row == g, b1full, 0.0), axis=0, keepdims=True)
    slab = o1_ref[...] + jnp.concatenate(
        [s0 * h1b[:, :64], s1 * h1b[:, 64:]], axis=1) + b1g
    g1 = _elu(slab)
    dh2 = jnp.dot(g1, w2_ref[...], preferred_element_type=jnp.float32)

    @pl.when(g == 0)
    def _():
        h2_ref[...] = dh2

    @pl.when(g > 0)
    def _():
        h2_ref[...] += dh2

    @pl.when(g == 3)
    def _():
        h2f = h2_ref[...]
        as2_ref[...] = jnp.dot(h2f, aws2_ref[...],
                               preferred_element_type=jnp.float32)
        ad2_ref[...] = jnp.dot(h2f, awd2_ref[...],
                               preferred_element_type=jnp.float32)


def _dense2(out1t, h1t, asrc16, adst16, dent16, W2, b1, aws2, awd2):
    bn = 1000
    nb = _N // bn
    b1r = b1.reshape(4, 128)
    return pl.pallas_call(
        _dense2_body,
        grid=(nb, 4),
        in_specs=[
            pl.BlockSpec((bn, 128), lambda i, g: (g * nb + i, 0)),
            pl.BlockSpec((bn, 128), lambda i, g: (g * nb + i, 0)),
            pl.BlockSpec((bn, 16), lambda i, g: (i, 0)),
            pl.BlockSpec((bn, 16), lambda i, g: (i, 0)),
            pl.BlockSpec((bn, 16), lambda i, g: (i, 0)),
            pl.BlockSpec((128, 32), lambda i, g: (g, 0)),
            pl.BlockSpec((4, 128), lambda i, g: (0, 0)),
            pl.BlockSpec((32, 16), lambda i, g: (0, 0)),
            pl.BlockSpec((32, 16), lambda i, g: (0, 0)),
        ],
        out_specs=[
            pl.BlockSpec((bn, 32), lambda i, g: (i, 0)),
            pl.BlockSpec((bn, 16), lambda i, g: (i, 0)),
            pl.BlockSpec((bn, 16), lambda i, g: (i, 0)),
        ],
        out_shape=[
            jax.ShapeDtypeStruct((_N, 32), jnp.float32),
            jax.ShapeDtypeStruct((_N, 16), jnp.float32),
            jax.ShapeDtypeStruct((_N, 16), jnp.float32),
        ],
    )(out1t, h1t, asrc16, adst16, dent16, W2, b1r, aws2, awd2)


def _combine2_body(dp_ref, as_ref, ad_ref, dent_ref):
    ex_self = jnp.exp(_vlrelu(as_ref[...] + ad_ref[...]))
    dent_ref[...] = dp_ref[0] + dp_ref[1] + ex_self


def _combine2(denp2, as2_16, ad2_16):
    bn = 1000
    return pl.pallas_call(
        _combine2_body,
        grid=(_N // bn,),
        in_specs=[
            pl.BlockSpec((2, bn, 16), lambda i: (0, i, 0)),
            pl.BlockSpec((bn, 16), lambda i: (i, 0)),
            pl.BlockSpec((bn, 16), lambda i: (i, 0)),
        ],
        out_specs=pl.BlockSpec((bn, 16), lambda i: (i, 0)),
        out_shape=jax.ShapeDtypeStruct((_N, 16), jnp.float32),
    )(denp2, as2_16, ad2_16)


def _dense3_body(op_ref, h2_ref, as2_ref, ad2_ref, dent2_ref,
                 dent1_ref, w3_ref, b2_ref, g_ref, h3_ref, dinv_ref):
    selfc = jnp.exp(_vlrelu(as2_ref[...] + ad2_ref[...])) / (dent2_ref[...]
                                                             + 1e-16)
    out2 = (op_ref[0] + op_ref[1] + selfc[:, 0:1] * h2_ref[...]
            + b2_ref[...])
    g2 = _elu(out2)
    h3 = jnp.dot(g2, w3_ref[...], preferred_element_type=jnp.float32)
    deg = dent1_ref[:, 8:9]
    dinv = lax.rsqrt(deg)
    h3_ref[...] = h3
    g_ref[...] = dinv * h3
    dinv_ref[...] = jnp.broadcast_to(dinv, h3.shape)


def _dense3(out2p, h2, as2_16, ad2_16, dent2, dent1, W3, b2):
    bn = 1000
    b2c = b2.reshape(1, 32)
    return pl.pallas_call(
        _dense3_body,
        grid=(_N // bn,),
        in_specs=[
            pl.BlockSpec((2, bn, 32), lambda i: (0, i, 0)),
            pl.BlockSpec((bn, 32), lambda i: (i, 0)),
            pl.BlockSpec((bn, 16), lambda i: (i, 0)),
            pl.BlockSpec((bn, 16), lambda i: (i, 0)),
            pl.BlockSpec((bn, 16), lambda i: (i, 0)),
            pl.BlockSpec((bn, 16), lambda i: (i, 0)),
            pl.BlockSpec((32, 16), lambda i: (0, 0)),
            pl.BlockSpec((1, 32), lambda i: (0, 0)),
        ],
        out_specs=[
            pl.BlockSpec((bn, 16), lambda i: (i, 0)),
            pl.BlockSpec((bn, 16), lambda i: (i, 0)),
            pl.BlockSpec((bn, 16), lambda i: (i, 0)),
        ],
        out_shape=[
            jax.ShapeDtypeStruct((_N, 16), jnp.float32),
            jax.ShapeDtypeStruct((_N, 16), jnp.float32),
            jax.ShapeDtypeStruct((_N, 16), jnp.float32),
        ],
    )(out2p, h2, as2_16, ad2_16, dent2, dent1, W3, b2c)


def _dense4_body(ap_ref, h3_ref, dinv_ref, b3_ref, out_ref):
    dinv = dinv_ref[...]
    out_ref[...] = (dinv * (ap_ref[0] + ap_ref[1])
                    + dinv * dinv * h3_ref[...] + b3_ref[...])


def _dense4(accp, h3, dinv, b3):
    bn = 1000
    b3c = b3.reshape(1, 16)
    return pl.pallas_call(
        _dense4_body,
        grid=(_N // bn,),
        in_specs=[
            pl.BlockSpec((2, bn, 16), lambda i: (0, i, 0)),
            pl.BlockSpec((bn, 16), lambda i: (i, 0)),
            pl.BlockSpec((bn, 16), lambda i: (i, 0)),
            pl.BlockSpec((1, 16), lambda i: (0, 0)),
        ],
        out_specs=pl.BlockSpec((bn, 16), lambda i: (i, 0)),
        out_shape=jax.ShapeDtypeStruct((_N, 16), jnp.float32),
    )(accp, h3, dinv, b3c)


def _logit_weights16(a):
    heads, ch = a.shape
    eye = jnp.eye(16, dtype=a.dtype)[:heads]
    return (a[:, :, None] * eye[:, None, :]).reshape(heads * ch, 16)


def kernel(x, edge_index, W1, a_src1, a_dst1, b1, W2, a_src2, a_dst2, b2, W3, b3):
    src = edge_index[0]
    dst = edge_index[1]

    aws1 = _logit_weights16(a_src1)
    awd1 = _logit_weights16(a_dst1)
    h1t, asrc16, adst16 = _dense1(x, W1, aws1, awd1)
    den1p = _att_den(asrc16, adst16, src, dst)
    dent1 = _combine1(den1p, asrc16, adst16)
    coef16 = _coef(asrc16, adst16, dent1, src, dst)
    out1t = _gat1_agg(h1t, coef16, src, dst)

    aws2 = _logit_weights16(a_src2)
    awd2 = _logit_weights16(a_dst2)
    h2, as2_16, ad2_16 = _dense2(out1t, h1t, asrc16, adst16, dent1,
                                 W2, b1, aws2, awd2)
    den2p, ex2 = _att_den2(as2_16, ad2_16, src, dst)
    dent2 = _combine2(den2p, as2_16, ad2_16)
    out2p = _gat2_agg(h2, ex2, dent2, src, dst)

    g, h3, dinv = _dense3(out2p, h2, as2_16, ad2_16, dent2, dent1, W3, b2)
    accp = _gcn_gather(g, src, dst)
    out = _dense4(accp, h3, dinv, b3)
    return out
